# Initial kernel scaffold; baseline (speedup 1.0000x reference)
import jax
import jax.numpy as jnp
from jax.experimental import pallas as pl


def kernel(x, edge_index, distance_matrix, nodes_to_community, params):
    raise NotImplementedError("write your pallas kernel here")


# speedup vs baseline: 1.2530x; 1.2530x over previous
import functools

import jax
import jax.numpy as jnp
from jax import lax
from jax.experimental import pallas as pl
from jax.experimental.pallas import tpu as pltpu

N = 10000
E = 320000
D_IN = 128
HID = 256
OUT = 128
C = 512
HEADS = 4
DHEAD = HID // HEADS
SCALE = 1.0 / (DHEAD ** 0.5)
BN = 1000
GRID_N = N // BN
AUG = 16


def _const2(shape):
    return pl.BlockSpec(shape, lambda i: (0, 0))


def _fcin_body(x_r, w1_r, b1_r, w2_r, b2_r, comm_r,
               g_r, praw_r, cnt_r):
    i = pl.program_id(0)
    g = jnp.maximum(jnp.dot(x_r[...], w1_r[...],
                            preferred_element_type=jnp.float32) + b1_r[...], 0.0)
    g = jnp.dot(g, w2_r[...], preferred_element_type=jnp.float32) + b2_r[...]
    g_r[...] = g
    comm = comm_r[0, 0, :]
    m2 = (comm[:, None] == lax.broadcasted_iota(jnp.int32, (BN, C), 1)
          ).astype(jnp.float32)
    g_aug = jnp.concatenate([g, jnp.ones((BN, AUG), jnp.float32)], axis=1)
    pb = lax.dot_general(m2, g_aug, (((0,), (0,)), ((), ())),
                         preferred_element_type=jnp.float32)
    cb = jnp.sum(m2, axis=0, keepdims=True)

    @pl.when(i == 0)
    def _():
        praw_r[...] = pb
        cnt_r[...] = cb

    @pl.when(i > 0)
    def _():
        praw_r[...] += pb
        cnt_r[...] += cb


def _fcin(x, comm2d, fp):
    return pl.pallas_call(
        _fcin_body,
        grid=(GRID_N,),
        in_specs=[
            pl.BlockSpec((BN, D_IN), lambda i: (i, 0)),
            _const2((D_IN, HID)), _const2((1, HID)),
            _const2((HID, HID)), _const2((1, HID)),
            pl.BlockSpec((1, 1, BN), lambda i: (i, 0, 0)),
        ],
        out_specs=[
            pl.BlockSpec((BN, HID), lambda i: (i, 0)),
            pl.BlockSpec((C, HID + AUG), lambda i: (0, 0)),
            pl.BlockSpec((1, C), lambda i: (0, 0)),
        ],
        out_shape=[
            jax.ShapeDtypeStruct((N, HID), jnp.float32),
            jax.ShapeDtypeStruct((C, HID + AUG), jnp.float32),
            jax.ShapeDtypeStruct((1, C), jnp.float32),
        ],
    )(x, fp['W1'], fp['b1'].reshape(1, HID), fp['W2'], fp['b2'].reshape(1, HID),
      comm2d)


def _cprep_body(pa_r, wk_r, bk_r, wv_r, bv_r, k_r, v_r):
    praw = pa_r[:, :HID]
    cnt = pa_r[:, HID:HID + 1]
    inv = 1.0 / jnp.maximum(cnt, 1.0)
    cavg = praw * inv
    k_r[...] = jnp.dot(cavg, wk_r[...],
                       preferred_element_type=jnp.float32) + bk_r[...]
    v_r[...] = jnp.dot(cavg, wv_r[...],
                       preferred_element_type=jnp.float32) + bv_r[...]


def _cprep(praw_aug, cp):
    return pl.pallas_call(
        _cprep_body,
        grid=(1,),
        in_specs=[
            _const2((C, HID + AUG)),
            _const2((HID, HID)), _const2((1, HID)),
            _const2((HID, HID)), _const2((1, HID)),
        ],
        out_specs=[
            pl.BlockSpec((C, HID), lambda i: (0, 0)),
            pl.BlockSpec((C, HID), lambda i: (0, 0)),
        ],
        out_shape=[
            jax.ShapeDtypeStruct((C, HID), jnp.float32),
            jax.ShapeDtypeStruct((C, HID), jnp.float32),
        ],
    )(praw_aug, cp['Wk'], cp['bk'].reshape(1, HID),
      cp['Wv'], cp['bv'].reshape(1, HID))


def _attn_core(g, dmat, cnt_r, k_r, v_r, wp_r, bp_r, wq_r, bq_r,
               w1_r, b1_r, w2_r, b2_r, wdis, bdis):
    q = jnp.dot(g, wp_r[...], preferred_element_type=jnp.float32) + bp_r[...]
    q = jnp.dot(q, wq_r[...], preferred_element_type=jnp.float32) + bq_r[...]
    base = dmat * wdis + bdis + jnp.log(cnt_r[...])
    outs = []
    for h in range(HEADS):
        sl = slice(h * DHEAD, (h + 1) * DHEAD)
        qh = q[:, sl]
        kh = k_r[:, sl]
        vh = v_r[:, sl]
        dots = lax.dot_general(qh, kh, (((1,), (1,)), ((), ())),
                               preferred_element_type=jnp.float32) * SCALE + base
        mx = jnp.max(dots, axis=-1, keepdims=True)
        e = jnp.exp(dots - mx)
        s = jnp.sum(e, axis=-1, keepdims=True)
        outs.append(jnp.dot(e, vh, preferred_element_type=jnp.float32) / s)
    o = jnp.concatenate(outs, axis=1)
    o = jnp.maximum(jnp.dot(o, w1_r[...],
                            preferred_element_type=jnp.float32) + b1_r[...], 0.0)
    return jnp.maximum(jnp.dot(o, w2_r[...],
                               preferred_element_type=jnp.float32) + b2_r[...], 0.0)


def _attn_mid_body(g_r, dm_r, cnt_r, k_r, v_r, wp_r, bp_r, wq_r, bq_r,
                   w1_r, b1_r, w2_r, b2_r, sc_r, comm_r,
                   gn_r, praw_r):
    i = pl.program_id(0)
    gn = _attn_core(g_r[...], dm_r[...], cnt_r, k_r, v_r, wp_r, bp_r,
                    wq_r, bq_r, w1_r, b1_r, w2_r, b2_r,
                    sc_r[0, 0], sc_r[0, 1])
    gn_r[...] = gn
    comm = comm_r[0, 0, :]
    m2 = (comm[:, None] == lax.broadcasted_iota(jnp.int32, (BN, C), 1)
          ).astype(jnp.float32)
    g_aug = jnp.concatenate([gn, jnp.ones((BN, AUG), jnp.float32)], axis=1)
    pb = lax.dot_general(m2, g_aug, (((0,), (0,)), ((), ())),
                         preferred_element_type=jnp.float32)

    @pl.when(i == 0)
    def _():
        praw_r[...] = pb

    @pl.when(i > 0)
    def _():
        praw_r[...] += pb


def _attn_mid(g, dmat, cnt_row, k, v, cp, fp, comm2d):
    sc = jnp.stack([cp['w_dis'], cp['b_dis']]).reshape(1, 2)
    return pl.pallas_call(
        _attn_mid_body,
        grid=(GRID_N,),
        in_specs=[
            pl.BlockSpec((BN, HID), lambda i: (i, 0)),
            pl.BlockSpec((BN, C), lambda i: (i, 0)),
            _const2((1, C)),
            _const2((C, HID)), _const2((C, HID)),
            _const2((HID, HID)), _const2((1, HID)),
            _const2((HID, HID)), _const2((1, HID)),
            _const2((HID, HID)), _const2((1, HID)),
            _const2((HID, HID)), _const2((1, HID)),
            _const2((1, 2)),
            pl.BlockSpec((1, 1, BN), lambda i: (i, 0, 0)),
        ],
        out_specs=[
            pl.BlockSpec((BN, HID), lambda i: (i, 0)),
            pl.BlockSpec((C, HID + AUG), lambda i: (0, 0)),
        ],
        out_shape=[
            jax.ShapeDtypeStruct((N, HID), jnp.float32),
            jax.ShapeDtypeStruct((C, HID + AUG), jnp.float32),
        ],
    )(g, dmat, cnt_row, k, v,
      cp['Wp'], cp['bp'].reshape(1, HID), cp['Wq'], cp['bq'].reshape(1, HID),
      fp['W1'], fp['b1'].reshape(1, HID), fp['W2'], fp['b2'].reshape(1, HID),
      sc, comm2d)


def _attn_last_body(g_r, dm_r, cnt_r, k_r, v_r, wp_r, bp_r, wq_r, bq_r,
                    w1_r, b1_r, w2_r, b2_r, sc_r, wo_r, bo_r, xloc_r,
                    out_r):
    gn = _attn_core(g_r[...], dm_r[...], cnt_r, k_r, v_r, wp_r, bp_r,
                    wq_r, bq_r, w1_r, b1_r, w2_r, b2_r,
                    sc_r[0, 0], sc_r[0, 1])
    out_r[...] = (jnp.dot(gn, wo_r[...], preferred_element_type=jnp.float32)
                  + bo_r[...] + xloc_r[...])


def _attn_last(g, dmat, cnt_row, k, v, cp, fp, op, xloc):
    sc = jnp.stack([cp['w_dis'], cp['b_dis']]).reshape(1, 2)
    return pl.pallas_call(
        _attn_last_body,
        grid=(GRID_N,),
        in_specs=[
            pl.BlockSpec((BN, HID), lambda i: (i, 0)),
            pl.BlockSpec((BN, C), lambda i: (i, 0)),
            _const2((1, C)),
            _const2((C, HID)), _const2((C, HID)),
            _const2((HID, HID)), _const2((1, HID)),
            _const2((HID, HID)), _const2((1, HID)),
            _const2((HID, HID)), _const2((1, HID)),
            _const2((HID, HID)), _const2((1, HID)),
            _const2((1, 2)),
            _const2((HID, OUT)), _const2((1, OUT)),
            pl.BlockSpec((BN, OUT), lambda i: (i, 0)),
        ],
        out_specs=pl.BlockSpec((BN, OUT), lambda i: (i, 0)),
        out_shape=jax.ShapeDtypeStruct((N, OUT), jnp.float32),
    )(g, dmat, cnt_row, k, v,
      cp['Wp'], cp['bp'].reshape(1, HID), cp['Wq'], cp['bq'].reshape(1, HID),
      fp['W1'], fp['b1'].reshape(1, HID), fp['W2'], fp['b2'].reshape(1, HID),
      sc, op['W'], op['b'].reshape(1, OUT), xloc)


def _sage1_body(ap_r, x_r, wl_r, bl_r, wr_r, h1a_r, h1b_r, di_r):
    a = ap_r[0] + ap_r[1]
    sums = a[:, :D_IN]
    deg = a[:, D_IN:D_IN + 1]
    deginv = 1.0 / jnp.maximum(deg, 1.0)
    mean = sums * deginv
    h = jnp.dot(mean, wl_r[...], preferred_element_type=jnp.float32) + bl_r[...]
    h = h + jnp.dot(x_r[...], wr_r[...], preferred_element_type=jnp.float32)
    h = jnp.maximum(h, 0.0)
    h1a_r[...] = h[:, :D_IN]
    h1b_r[...] = h[:, D_IN:]
    di_r[...] = jnp.broadcast_to(deginv, (BN, D_IN))


def _sage1(agg1p, x, gp):
    return pl.pallas_call(
        _sage1_body,
        grid=(GRID_N,),
        in_specs=[
            pl.BlockSpec((2, BN, D_IN + AUG), lambda i: (0, i, 0)),
            pl.BlockSpec((BN, D_IN), lambda i: (i, 0)),
            _const2((D_IN, HID)), _const2((1, HID)),
            _const2((D_IN, HID)),
        ],
        out_specs=[
            pl.BlockSpec((BN, D_IN), lambda i: (i, 0)),
            pl.BlockSpec((BN, D_IN), lambda i: (i, 0)),
            pl.BlockSpec((BN, D_IN), lambda i: (i, 0)),
        ],
        out_shape=[
            jax.ShapeDtypeStruct((N, D_IN), jnp.float32),
            jax.ShapeDtypeStruct((N, D_IN), jnp.float32),
            jax.ShapeDtypeStruct((N, D_IN), jnp.float32),
        ],
    )(agg1p, x, gp['Wl'], gp['bl'].reshape(1, HID), gp['Wr'])


def _sage2_body(ag_r, h1a_r, h1b_r, di_r, wl_r, bl_r, wr_r, wl3_r,
                h2_r, z_r):
    di = di_r[...]
    mean = jnp.concatenate([ag_r[0] * di, ag_r[1] * di], axis=1)
    h1 = jnp.concatenate([h1a_r[...], h1b_r[...]], axis=1)
    h = jnp.dot(mean, wl_r[...], preferred_element_type=jnp.float32) + bl_r[...]
    h = h + jnp.dot(h1, wr_r[...], preferred_element_type=jnp.float32)
    h = jnp.maximum(h, 0.0)
    h2_r[...] = h
    z_r[...] = jnp.dot(h, wl3_r[...], preferred_element_type=jnp.float32)


def _sage2(agg2, h1a, h1b, deginv, gp, gp3):
    return pl.pallas_call(
        _sage2_body,
        grid=(GRID_N,),
        in_specs=[
            pl.BlockSpec((2, BN, D_IN), lambda i: (0, i, 0)),
            pl.BlockSpec((BN, D_IN), lambda i: (i, 0)),
            pl.BlockSpec((BN, D_IN), lambda i: (i, 0)),
            pl.BlockSpec((BN, D_IN), lambda i: (i, 0)),
            _const2((HID, HID)), _const2((1, HID)),
            _const2((HID, HID)),
            _const2((HID, OUT)),
        ],
        out_specs=[
            pl.BlockSpec((BN, HID), lambda i: (i, 0)),
            pl.BlockSpec((BN, OUT), lambda i: (i, 0)),
        ],
        out_shape=[
            jax.ShapeDtypeStruct((N, HID), jnp.float32),
            jax.ShapeDtypeStruct((N, OUT), jnp.float32),
        ],
    )(agg2, h1a, h1b, deginv, gp['Wl'], gp['bl'].reshape(1, HID), gp['Wr'],
      gp3['Wl'])


def _sage3_body(ap_r, h2_r, di_r, bl_r, wr_r, xloc_r):
    agg = (ap_r[0] + ap_r[1]) * di_r[...]
    xloc_r[...] = (agg + bl_r[...]
                   + jnp.dot(h2_r[...], wr_r[...],
                             preferred_element_type=jnp.float32))


def _sage3(agg3p, h2, deginv, gp3):
    return pl.pallas_call(
        _sage3_body,
        grid=(GRID_N,),
        in_specs=[
            pl.BlockSpec((2, BN, OUT), lambda i: (0, i, 0)),
            pl.BlockSpec((BN, HID), lambda i: (i, 0)),
            pl.BlockSpec((BN, OUT), lambda i: (i, 0)),
            _const2((1, OUT)),
            _const2((HID, OUT)),
        ],
        out_specs=pl.BlockSpec((BN, OUT), lambda i: (i, 0)),
        out_shape=jax.ShapeDtypeStruct((N, OUT), jnp.float32),
    )(agg3p, h2, deginv, gp3['bl'].reshape(1, OUT), gp3['Wr'])


def _agg_edge_split(table, src, dst):
    half = E // 2
    p0 = jax.ops.segment_sum(table[src[:half]], dst[:half], num_segments=N)
    p1 = jax.ops.segment_sum(table[src[half:]], dst[half:], num_segments=N)
    return jnp.stack([p0, p1])


def _agg_feat_split(ta, tb, src, dst):
    pa = jax.ops.segment_sum(ta[src], dst, num_segments=N)
    pb = jax.ops.segment_sum(tb[src], dst, num_segments=N)
    return jnp.stack([pa, pb])


def kernel(x, edge_index, distance_matrix, nodes_to_community, params):
    src = edge_index[0]
    dst = edge_index[1]
    comm2d = nodes_to_community.reshape(GRID_N, 1, BN)

    x_aug = jnp.concatenate([x, jnp.ones((N, AUG), jnp.float32)], axis=1)
    agg1p = _agg_edge_split(x_aug, src, dst)
    h1a, h1b, deginv = _sage1(agg1p, x, params['gnn'][0])
    agg2 = _agg_feat_split(h1a, h1b, src, dst)
    h2, z = _sage2(agg2, h1a, h1b, deginv, params['gnn'][1], params['gnn'][2])
    agg3p = _agg_edge_split(z, src, dst)
    xloc = _sage3(agg3p, h2, deginv, params['gnn'][2])

    g, praw, cnt_row = _fcin(x, comm2d, params['fc_in'])
    k0, v0 = _cprep(praw, params['convs'][0])
    g, praw1 = _attn_mid(g, distance_matrix, cnt_row, k0, v0,
                         params['convs'][0], params['ffs'][0], comm2d)
    k1, v1 = _cprep(praw1, params['convs'][1])
    return _attn_last(g, distance_matrix, cnt_row, k1, v1,
                      params['convs'][1], params['ffs'][1], params['fc_out'],
                      xloc)


# speedup vs baseline: 4.2934x; 3.4266x over previous
import functools

import jax
import jax.numpy as jnp
from jax import lax
from jax.experimental import pallas as pl
from jax.experimental.pallas import tpu as pltpu
from jax.experimental.pallas import tpu_sc as plsc

N = 10000
E = 320000
D_IN = 128
HID = 256
OUT = 128
C = 512
HEADS = 4
DHEAD = HID // HEADS
SCALE = 1.0 / (DHEAD ** 0.5)
BN = 1000
GRID_N = N // BN
AUG = 16


def _const2(shape):
    return pl.BlockSpec(shape, lambda i: (0, 0))


def _fcin_body(x_r, w1_r, b1_r, w2_r, b2_r, comm_r,
               g_r, praw_r, cnt_r):
    i = pl.program_id(0)
    g = jnp.maximum(jnp.dot(x_r[...], w1_r[...],
                            preferred_element_type=jnp.float32) + b1_r[...], 0.0)
    g = jnp.dot(g, w2_r[...], preferred_element_type=jnp.float32) + b2_r[...]
    g_r[...] = g
    comm = comm_r[0, 0, :]
    m2 = (comm[:, None] == lax.broadcasted_iota(jnp.int32, (BN, C), 1)
          ).astype(jnp.float32)
    g_aug = jnp.concatenate([g, jnp.ones((BN, AUG), jnp.float32)], axis=1)
    pb = lax.dot_general(m2, g_aug, (((0,), (0,)), ((), ())),
                         preferred_element_type=jnp.float32)
    cb = jnp.sum(m2, axis=0, keepdims=True)

    @pl.when(i == 0)
    def _():
        praw_r[...] = pb
        cnt_r[...] = cb

    @pl.when(i > 0)
    def _():
        praw_r[...] += pb
        cnt_r[...] += cb


def _fcin(x, comm2d, fp):
    return pl.pallas_call(
        _fcin_body,
        grid=(GRID_N,),
        in_specs=[
            pl.BlockSpec((BN, D_IN), lambda i: (i, 0)),
            _const2((D_IN, HID)), _const2((1, HID)),
            _const2((HID, HID)), _const2((1, HID)),
            pl.BlockSpec((1, 1, BN), lambda i: (i, 0, 0)),
        ],
        out_specs=[
            pl.BlockSpec((BN, HID), lambda i: (i, 0)),
            pl.BlockSpec((C, HID + AUG), lambda i: (0, 0)),
            pl.BlockSpec((1, C), lambda i: (0, 0)),
        ],
        out_shape=[
            jax.ShapeDtypeStruct((N, HID), jnp.float32),
            jax.ShapeDtypeStruct((C, HID + AUG), jnp.float32),
            jax.ShapeDtypeStruct((1, C), jnp.float32),
        ],
    )(x, fp['W1'], fp['b1'].reshape(1, HID), fp['W2'], fp['b2'].reshape(1, HID),
      comm2d)


def _cprep_body(pa_r, wk_r, bk_r, wv_r, bv_r, k_r, v_r):
    praw = pa_r[:, :HID]
    cnt = pa_r[:, HID:HID + 1]
    inv = 1.0 / jnp.maximum(cnt, 1.0)
    cavg = praw * inv
    k_r[...] = jnp.dot(cavg, wk_r[...],
                       preferred_element_type=jnp.float32) + bk_r[...]
    v_r[...] = jnp.dot(cavg, wv_r[...],
                       preferred_element_type=jnp.float32) + bv_r[...]


def _cprep(praw_aug, cp):
    return pl.pallas_call(
        _cprep_body,
        grid=(1,),
        in_specs=[
            _const2((C, HID + AUG)),
            _const2((HID, HID)), _const2((1, HID)),
            _const2((HID, HID)), _const2((1, HID)),
        ],
        out_specs=[
            pl.BlockSpec((C, HID), lambda i: (0, 0)),
            pl.BlockSpec((C, HID), lambda i: (0, 0)),
        ],
        out_shape=[
            jax.ShapeDtypeStruct((C, HID), jnp.float32),
            jax.ShapeDtypeStruct((C, HID), jnp.float32),
        ],
    )(praw_aug, cp['Wk'], cp['bk'].reshape(1, HID),
      cp['Wv'], cp['bv'].reshape(1, HID))


def _attn_core(g, dmat, cnt_r, k_r, v_r, wp_r, bp_r, wq_r, bq_r,
               w1_r, b1_r, w2_r, b2_r, wdis, bdis):
    q = jnp.dot(g, wp_r[...], preferred_element_type=jnp.float32) + bp_r[...]
    q = jnp.dot(q, wq_r[...], preferred_element_type=jnp.float32) + bq_r[...]
    base = dmat * wdis + bdis + jnp.log(cnt_r[...])
    outs = []
    for h in range(HEADS):
        sl = slice(h * DHEAD, (h + 1) * DHEAD)
        qh = q[:, sl]
        kh = k_r[:, sl]
        vh = v_r[:, sl]
        dots = lax.dot_general(qh, kh, (((1,), (1,)), ((), ())),
                               preferred_element_type=jnp.float32) * SCALE + base
        mx = jnp.max(dots, axis=-1, keepdims=True)
        e = jnp.exp(dots - mx)
        s = jnp.sum(e, axis=-1, keepdims=True)
        outs.append(jnp.dot(e, vh, preferred_element_type=jnp.float32) / s)
    o = jnp.concatenate(outs, axis=1)
    o = jnp.maximum(jnp.dot(o, w1_r[...],
                            preferred_element_type=jnp.float32) + b1_r[...], 0.0)
    return jnp.maximum(jnp.dot(o, w2_r[...],
                               preferred_element_type=jnp.float32) + b2_r[...], 0.0)


def _attn_mid_body(g_r, dm_r, cnt_r, k_r, v_r, wp_r, bp_r, wq_r, bq_r,
                   w1_r, b1_r, w2_r, b2_r, sc_r, comm_r,
                   gn_r, praw_r):
    i = pl.program_id(0)
    gn = _attn_core(g_r[...], dm_r[...], cnt_r, k_r, v_r, wp_r, bp_r,
                    wq_r, bq_r, w1_r, b1_r, w2_r, b2_r,
                    sc_r[0, 0], sc_r[0, 1])
    gn_r[...] = gn
    comm = comm_r[0, 0, :]
    m2 = (comm[:, None] == lax.broadcasted_iota(jnp.int32, (BN, C), 1)
          ).astype(jnp.float32)
    g_aug = jnp.concatenate([gn, jnp.ones((BN, AUG), jnp.float32)], axis=1)
    pb = lax.dot_general(m2, g_aug, (((0,), (0,)), ((), ())),
                         preferred_element_type=jnp.float32)

    @pl.when(i == 0)
    def _():
        praw_r[...] = pb

    @pl.when(i > 0)
    def _():
        praw_r[...] += pb


def _attn_mid(g, dmat, cnt_row, k, v, cp, fp, comm2d):
    sc = jnp.stack([cp['w_dis'], cp['b_dis']]).reshape(1, 2)
    return pl.pallas_call(
        _attn_mid_body,
        grid=(GRID_N,),
        in_specs=[
            pl.BlockSpec((BN, HID), lambda i: (i, 0)),
            pl.BlockSpec((BN, C), lambda i: (i, 0)),
            _const2((1, C)),
            _const2((C, HID)), _const2((C, HID)),
            _const2((HID, HID)), _const2((1, HID)),
            _const2((HID, HID)), _const2((1, HID)),
            _const2((HID, HID)), _const2((1, HID)),
            _const2((HID, HID)), _const2((1, HID)),
            _const2((1, 2)),
            pl.BlockSpec((1, 1, BN), lambda i: (i, 0, 0)),
        ],
        out_specs=[
            pl.BlockSpec((BN, HID), lambda i: (i, 0)),
            pl.BlockSpec((C, HID + AUG), lambda i: (0, 0)),
        ],
        out_shape=[
            jax.ShapeDtypeStruct((N, HID), jnp.float32),
            jax.ShapeDtypeStruct((C, HID + AUG), jnp.float32),
        ],
    )(g, dmat, cnt_row, k, v,
      cp['Wp'], cp['bp'].reshape(1, HID), cp['Wq'], cp['bq'].reshape(1, HID),
      fp['W1'], fp['b1'].reshape(1, HID), fp['W2'], fp['b2'].reshape(1, HID),
      sc, comm2d)


def _attn_last_body(g_r, dm_r, cnt_r, k_r, v_r, wp_r, bp_r, wq_r, bq_r,
                    w1_r, b1_r, w2_r, b2_r, sc_r, wo_r, bo_r, xloc_r,
                    out_r):
    gn = _attn_core(g_r[...], dm_r[...], cnt_r, k_r, v_r, wp_r, bp_r,
                    wq_r, bq_r, w1_r, b1_r, w2_r, b2_r,
                    sc_r[0, 0], sc_r[0, 1])
    out_r[...] = (jnp.dot(gn, wo_r[...], preferred_element_type=jnp.float32)
                  + bo_r[...] + xloc_r[...])


def _attn_last(g, dmat, cnt_row, k, v, cp, fp, op, xloc):
    sc = jnp.stack([cp['w_dis'], cp['b_dis']]).reshape(1, 2)
    return pl.pallas_call(
        _attn_last_body,
        grid=(GRID_N,),
        in_specs=[
            pl.BlockSpec((BN, HID), lambda i: (i, 0)),
            pl.BlockSpec((BN, C), lambda i: (i, 0)),
            _const2((1, C)),
            _const2((C, HID)), _const2((C, HID)),
            _const2((HID, HID)), _const2((1, HID)),
            _const2((HID, HID)), _const2((1, HID)),
            _const2((HID, HID)), _const2((1, HID)),
            _const2((HID, HID)), _const2((1, HID)),
            _const2((1, 2)),
            _const2((HID, OUT)), _const2((1, OUT)),
            pl.BlockSpec((BN, OUT), lambda i: (i, 0)),
        ],
        out_specs=pl.BlockSpec((BN, OUT), lambda i: (i, 0)),
        out_shape=jax.ShapeDtypeStruct((N, OUT), jnp.float32),
    )(g, dmat, cnt_row, k, v,
      cp['Wp'], cp['bp'].reshape(1, HID), cp['Wq'], cp['bq'].reshape(1, HID),
      fp['W1'], fp['b1'].reshape(1, HID), fp['W2'], fp['b2'].reshape(1, HID),
      sc, op['W'], op['b'].reshape(1, OUT), xloc)


def _sage1_body(ap_r, x_r, wl_r, bl_r, wr_r, h1a_r, h1b_r, di_r):
    a = ap_r[0] + ap_r[1]
    sums = a[:, :D_IN]
    deg = a[:, D_IN:D_IN + 1]
    deginv = 1.0 / jnp.maximum(deg, 1.0)
    mean = sums * deginv
    h = jnp.dot(mean, wl_r[...], preferred_element_type=jnp.float32) + bl_r[...]
    h = h + jnp.dot(x_r[...], wr_r[...], preferred_element_type=jnp.float32)
    h = jnp.maximum(h, 0.0)
    h1a_r[...] = h[:, :D_IN]
    h1b_r[...] = h[:, D_IN:]
    di_r[...] = jnp.broadcast_to(deginv, (BN, D_IN))


def _sage1(agg1p, x, gp):
    return pl.pallas_call(
        _sage1_body,
        grid=(GRID_N,),
        in_specs=[
            pl.BlockSpec((2, BN, D_IN + AUG), lambda i: (0, i, 0)),
            pl.BlockSpec((BN, D_IN), lambda i: (i, 0)),
            _const2((D_IN, HID)), _const2((1, HID)),
            _const2((D_IN, HID)),
        ],
        out_specs=[
            pl.BlockSpec((BN, D_IN), lambda i: (i, 0)),
            pl.BlockSpec((BN, D_IN), lambda i: (i, 0)),
            pl.BlockSpec((BN, D_IN), lambda i: (i, 0)),
        ],
        out_shape=[
            jax.ShapeDtypeStruct((N, D_IN), jnp.float32),
            jax.ShapeDtypeStruct((N, D_IN), jnp.float32),
            jax.ShapeDtypeStruct((N, D_IN), jnp.float32),
        ],
    )(agg1p, x, gp['Wl'], gp['bl'].reshape(1, HID), gp['Wr'])


def _sage2_body(ag_r, h1a_r, h1b_r, di_r, wl_r, bl_r, wr_r, wl3_r,
                h2_r, z_r):
    di = di_r[...]
    mean = jnp.concatenate([ag_r[0] * di, ag_r[1] * di], axis=1)
    h1 = jnp.concatenate([h1a_r[...], h1b_r[...]], axis=1)
    h = jnp.dot(mean, wl_r[...], preferred_element_type=jnp.float32) + bl_r[...]
    h = h + jnp.dot(h1, wr_r[...], preferred_element_type=jnp.float32)
    h = jnp.maximum(h, 0.0)
    h2_r[...] = h
    z_r[...] = jnp.dot(h, wl3_r[...], preferred_element_type=jnp.float32)


def _sage2(agg2, h1a, h1b, deginv, gp, gp3):
    return pl.pallas_call(
        _sage2_body,
        grid=(GRID_N,),
        in_specs=[
            pl.BlockSpec((2, BN, D_IN), lambda i: (0, i, 0)),
            pl.BlockSpec((BN, D_IN), lambda i: (i, 0)),
            pl.BlockSpec((BN, D_IN), lambda i: (i, 0)),
            pl.BlockSpec((BN, D_IN), lambda i: (i, 0)),
            _const2((HID, HID)), _const2((1, HID)),
            _const2((HID, HID)),
            _const2((HID, OUT)),
        ],
        out_specs=[
            pl.BlockSpec((BN, HID), lambda i: (i, 0)),
            pl.BlockSpec((BN, OUT), lambda i: (i, 0)),
        ],
        out_shape=[
            jax.ShapeDtypeStruct((N, HID), jnp.float32),
            jax.ShapeDtypeStruct((N, OUT), jnp.float32),
        ],
    )(agg2, h1a, h1b, deginv, gp['Wl'], gp['bl'].reshape(1, HID), gp['Wr'],
      gp3['Wl'])


def _sage3_body(ap_r, h2_r, di_r, bl_r, wr_r, xloc_r):
    agg = (ap_r[0] + ap_r[1]) * di_r[...]
    xloc_r[...] = (agg + bl_r[...]
                   + jnp.dot(h2_r[...], wr_r[...],
                             preferred_element_type=jnp.float32))


def _sage3(agg3p, h2, deginv, gp3):
    return pl.pallas_call(
        _sage3_body,
        grid=(GRID_N,),
        in_specs=[
            pl.BlockSpec((2, BN, OUT), lambda i: (0, i, 0)),
            pl.BlockSpec((BN, HID), lambda i: (i, 0)),
            pl.BlockSpec((BN, OUT), lambda i: (i, 0)),
            _const2((1, OUT)),
            _const2((HID, OUT)),
        ],
        out_specs=pl.BlockSpec((BN, OUT), lambda i: (i, 0)),
        out_shape=jax.ShapeDtypeStruct((N, OUT), jnp.float32),
    )(agg3p, h2, deginv, gp3['bl'].reshape(1, OUT), gp3['Wr'])


NTILES = 16
KCH = 80
ZR = 624
ZR_LAST = N - (NTILES - 1) * ZR


def _make_edge_agg(ncols, edge_split):
    mesh = plsc.VectorSubcoreMesh(core_axis_name="c", subcore_axis_name="s")
    epw = E // 32 if edge_split else E // NTILES
    nch = epw // KCH

    @functools.partial(
        pl.kernel,
        out_type=jax.ShapeDtypeStruct((2, N, ncols), jnp.float32),
        mesh=mesh,
        scratch_types=[
            pltpu.VMEM_SHARED((N, ncols), jnp.float32),
            pltpu.VMEM((KCH,), jnp.int32),
            pltpu.VMEM((KCH,), jnp.int32),
            pltpu.VMEM((KCH, ncols), jnp.float32),
            pltpu.SemaphoreType.DMA,
        ],
        compiler_params=pltpu.CompilerParams(use_tc_tiling_on_sc=False),
    )
    def k(tab_a, tab_b, src_h, dst_h, zeros_h, out_h,
          acc, sidx, didx, rows, sem):
        c = lax.axis_index("c")
        s = lax.axis_index("s")

        @pl.when(s < NTILES - 1)
        def _():
            pltpu.sync_copy(zeros_h.at[pl.ds(0, ZR)],
                            acc.at[pl.ds(s * ZR, ZR)])

        @pl.when(s == NTILES - 1)
        def _():
            pltpu.sync_copy(zeros_h, acc.at[pl.ds((NTILES - 1) * ZR, ZR_LAST)])

        plsc.subcore_barrier()

        def run(tab, base):
            def chunk(i, carry):
                off = base + i * KCH
                pltpu.sync_copy(src_h.at[pl.ds(off, KCH)], sidx)
                pltpu.sync_copy(dst_h.at[pl.ds(off, KCH)], didx)
                pltpu.async_copy(tab.at[sidx], rows, sem).wait()
                pltpu.sync_copy(rows, acc.at[didx], add=True)
                return carry
            lax.fori_loop(0, nch, chunk, 0)

        if edge_split:
            run(tab_a, (c * NTILES + s) * epw)
        else:
            @pl.when(c == 0)
            def _():
                run(tab_a, s * epw)

            @pl.when(c == 1)
            def _():
                run(tab_b, s * epw)

        plsc.subcore_barrier()

        @pl.when(s < NTILES - 1)
        def _():
            pltpu.sync_copy(acc.at[pl.ds(s * ZR, ZR)],
                            out_h.at[c, pl.ds(s * ZR, ZR)])

        @pl.when(s == NTILES - 1)
        def _():
            pltpu.sync_copy(acc.at[pl.ds((NTILES - 1) * ZR, ZR_LAST)],
                            out_h.at[c, pl.ds((NTILES - 1) * ZR, ZR_LAST)])

    return k


_edge_agg_aug = _make_edge_agg(D_IN + AUG, True)
_edge_agg_feat = _make_edge_agg(D_IN, False)
_edge_agg_128 = _make_edge_agg(D_IN, True)


def _agg_edge_split_aug(table, src, dst):
    zeros = jnp.zeros((ZR_LAST, D_IN + AUG), jnp.float32)
    return _edge_agg_aug(table, table, src, dst, zeros)


def _agg_edge_split_128(table, src, dst):
    zeros = jnp.zeros((ZR_LAST, D_IN), jnp.float32)
    return _edge_agg_128(table, table, src, dst, zeros)


def _agg_feat_split(ta, tb, src, dst):
    zeros = jnp.zeros((ZR_LAST, D_IN), jnp.float32)
    return _edge_agg_feat(ta, tb, src, dst, zeros)


def kernel(x, edge_index, distance_matrix, nodes_to_community, params):
    src = edge_index[0]
    dst = edge_index[1]
    comm2d = nodes_to_community.reshape(GRID_N, 1, BN)

    x_aug = jnp.concatenate([x, jnp.ones((N, AUG), jnp.float32)], axis=1)
    agg1p = _agg_edge_split_aug(x_aug, src, dst)
    h1a, h1b, deginv = _sage1(agg1p, x, params['gnn'][0])
    agg2 = _agg_feat_split(h1a, h1b, src, dst)
    h2, z = _sage2(agg2, h1a, h1b, deginv, params['gnn'][1], params['gnn'][2])
    agg3p = _agg_edge_split_128(z, src, dst)
    xloc = _sage3(agg3p, h2, deginv, params['gnn'][2])

    g, praw, cnt_row = _fcin(x, comm2d, params['fc_in'])
    k0, v0 = _cprep(praw, params['convs'][0])
    g, praw1 = _attn_mid(g, distance_matrix, cnt_row, k0, v0,
                         params['convs'][0], params['ffs'][0], comm2d)
    k1, v1 = _cprep(praw1, params['convs'][1])
    return _attn_last(g, distance_matrix, cnt_row, k1, v1,
                      params['convs'][1], params['ffs'][1], params['fc_out'],
                      xloc)


# speedup vs baseline: 6.9503x; 1.6188x over previous
import functools

import jax
import jax.numpy as jnp
from jax import lax
from jax.experimental import pallas as pl
from jax.experimental.pallas import tpu as pltpu
from jax.experimental.pallas import tpu_sc as plsc

N = 10000
E = 320000
D_IN = 128
HID = 256
OUT = 128
C = 512
HEADS = 4
DHEAD = HID // HEADS
SCALE = 1.0 / (DHEAD ** 0.5)
BN = 1000
GRID_N = N // BN
AUG = 16


def _const2(shape):
    return pl.BlockSpec(shape, lambda i: (0, 0))


def _fcin_body(x_r, w1_r, b1_r, w2_r, b2_r, comm_r,
               g_r, praw_r, cnt_r):
    i = pl.program_id(0)
    g = jnp.maximum(jnp.dot(x_r[...], w1_r[...],
                            preferred_element_type=jnp.float32) + b1_r[...], 0.0)
    g = jnp.dot(g, w2_r[...], preferred_element_type=jnp.float32) + b2_r[...]
    g_r[...] = g
    comm = comm_r[0, 0, :]
    m2 = (comm[:, None] == lax.broadcasted_iota(jnp.int32, (BN, C), 1)
          ).astype(jnp.float32)
    g_aug = jnp.concatenate([g, jnp.ones((BN, AUG), jnp.float32)], axis=1)
    pb = lax.dot_general(m2, g_aug, (((0,), (0,)), ((), ())),
                         preferred_element_type=jnp.float32)
    cb = jnp.sum(m2, axis=0, keepdims=True)

    @pl.when(i == 0)
    def _():
        praw_r[...] = pb
        cnt_r[...] = cb

    @pl.when(i > 0)
    def _():
        praw_r[...] += pb
        cnt_r[...] += cb


def _fcin(x, comm2d, fp):
    return pl.pallas_call(
        _fcin_body,
        grid=(GRID_N,),
        in_specs=[
            pl.BlockSpec((BN, D_IN), lambda i: (i, 0)),
            _const2((D_IN, HID)), _const2((1, HID)),
            _const2((HID, HID)), _const2((1, HID)),
            pl.BlockSpec((1, 1, BN), lambda i: (i, 0, 0)),
        ],
        out_specs=[
            pl.BlockSpec((BN, HID), lambda i: (i, 0)),
            pl.BlockSpec((C, HID + AUG), lambda i: (0, 0)),
            pl.BlockSpec((1, C), lambda i: (0, 0)),
        ],
        out_shape=[
            jax.ShapeDtypeStruct((N, HID), jnp.float32),
            jax.ShapeDtypeStruct((C, HID + AUG), jnp.float32),
            jax.ShapeDtypeStruct((1, C), jnp.float32),
        ],
    )(x, fp['W1'], fp['b1'].reshape(1, HID), fp['W2'], fp['b2'].reshape(1, HID),
      comm2d)


def _cprep_body(pa_r, wk_r, bk_r, wv_r, bv_r, k_r, v_r):
    praw = pa_r[:, :HID]
    cnt = pa_r[:, HID:HID + 1]
    inv = 1.0 / jnp.maximum(cnt, 1.0)
    cavg = praw * inv
    k_r[...] = jnp.dot(cavg, wk_r[...],
                       preferred_element_type=jnp.float32) + bk_r[...]
    v_r[...] = jnp.dot(cavg, wv_r[...],
                       preferred_element_type=jnp.float32) + bv_r[...]


def _cprep(praw_aug, cp):
    return pl.pallas_call(
        _cprep_body,
        grid=(1,),
        in_specs=[
            _const2((C, HID + AUG)),
            _const2((HID, HID)), _const2((1, HID)),
            _const2((HID, HID)), _const2((1, HID)),
        ],
        out_specs=[
            pl.BlockSpec((C, HID), lambda i: (0, 0)),
            pl.BlockSpec((C, HID), lambda i: (0, 0)),
        ],
        out_shape=[
            jax.ShapeDtypeStruct((C, HID), jnp.float32),
            jax.ShapeDtypeStruct((C, HID), jnp.float32),
        ],
    )(praw_aug, cp['Wk'], cp['bk'].reshape(1, HID),
      cp['Wv'], cp['bv'].reshape(1, HID))


def _attn_core(g, dmat, cnt_r, k_r, v_r, wp_r, bp_r, wq_r, bq_r,
               w1_r, b1_r, w2_r, b2_r, wdis, bdis):
    q = jnp.dot(g, wp_r[...], preferred_element_type=jnp.float32) + bp_r[...]
    q = jnp.dot(q, wq_r[...], preferred_element_type=jnp.float32) + bq_r[...]
    base = dmat * wdis + bdis + jnp.log(cnt_r[...])
    outs = []
    for h in range(HEADS):
        sl = slice(h * DHEAD, (h + 1) * DHEAD)
        qh = q[:, sl]
        kh = k_r[:, sl]
        vh = v_r[:, sl]
        dots = lax.dot_general(qh, kh, (((1,), (1,)), ((), ())),
                               preferred_element_type=jnp.float32) * SCALE + base
        mx = jnp.max(dots, axis=-1, keepdims=True)
        e = jnp.exp(dots - mx)
        s = jnp.sum(e, axis=-1, keepdims=True)
        outs.append(jnp.dot(e, vh, preferred_element_type=jnp.float32) / s)
    o = jnp.concatenate(outs, axis=1)
    o = jnp.maximum(jnp.dot(o, w1_r[...],
                            preferred_element_type=jnp.float32) + b1_r[...], 0.0)
    return jnp.maximum(jnp.dot(o, w2_r[...],
                               preferred_element_type=jnp.float32) + b2_r[...], 0.0)


def _attn_mid_body(g_r, dm_r, cnt_r, k_r, v_r, wp_r, bp_r, wq_r, bq_r,
                   w1_r, b1_r, w2_r, b2_r, sc_r, comm_r,
                   gn_r, praw_r):
    i = pl.program_id(0)
    gn = _attn_core(g_r[...], dm_r[...], cnt_r, k_r, v_r, wp_r, bp_r,
                    wq_r, bq_r, w1_r, b1_r, w2_r, b2_r,
                    sc_r[0, 0], sc_r[0, 1])
    gn_r[...] = gn
    comm = comm_r[0, 0, :]
    m2 = (comm[:, None] == lax.broadcasted_iota(jnp.int32, (BN, C), 1)
          ).astype(jnp.float32)
    g_aug = jnp.concatenate([gn, jnp.ones((BN, AUG), jnp.float32)], axis=1)
    pb = lax.dot_general(m2, g_aug, (((0,), (0,)), ((), ())),
                         preferred_element_type=jnp.float32)

    @pl.when(i == 0)
    def _():
        praw_r[...] = pb

    @pl.when(i > 0)
    def _():
        praw_r[...] += pb


def _attn_mid(g, dmat, cnt_row, k, v, cp, fp, comm2d):
    sc = jnp.stack([cp['w_dis'], cp['b_dis']]).reshape(1, 2)
    return pl.pallas_call(
        _attn_mid_body,
        grid=(GRID_N,),
        in_specs=[
            pl.BlockSpec((BN, HID), lambda i: (i, 0)),
            pl.BlockSpec((BN, C), lambda i: (i, 0)),
            _const2((1, C)),
            _const2((C, HID)), _const2((C, HID)),
            _const2((HID, HID)), _const2((1, HID)),
            _const2((HID, HID)), _const2((1, HID)),
            _const2((HID, HID)), _const2((1, HID)),
            _const2((HID, HID)), _const2((1, HID)),
            _const2((1, 2)),
            pl.BlockSpec((1, 1, BN), lambda i: (i, 0, 0)),
        ],
        out_specs=[
            pl.BlockSpec((BN, HID), lambda i: (i, 0)),
            pl.BlockSpec((C, HID + AUG), lambda i: (0, 0)),
        ],
        out_shape=[
            jax.ShapeDtypeStruct((N, HID), jnp.float32),
            jax.ShapeDtypeStruct((C, HID + AUG), jnp.float32),
        ],
    )(g, dmat, cnt_row, k, v,
      cp['Wp'], cp['bp'].reshape(1, HID), cp['Wq'], cp['bq'].reshape(1, HID),
      fp['W1'], fp['b1'].reshape(1, HID), fp['W2'], fp['b2'].reshape(1, HID),
      sc, comm2d)


def _attn_last_body(g_r, dm_r, cnt_r, k_r, v_r, wp_r, bp_r, wq_r, bq_r,
                    w1_r, b1_r, w2_r, b2_r, sc_r, wo_r, bo_r, xloc_r,
                    out_r):
    gn = _attn_core(g_r[...], dm_r[...], cnt_r, k_r, v_r, wp_r, bp_r,
                    wq_r, bq_r, w1_r, b1_r, w2_r, b2_r,
                    sc_r[0, 0], sc_r[0, 1])
    out_r[...] = (jnp.dot(gn, wo_r[...], preferred_element_type=jnp.float32)
                  + bo_r[...] + xloc_r[...])


def _attn_last(g, dmat, cnt_row, k, v, cp, fp, op, xloc):
    sc = jnp.stack([cp['w_dis'], cp['b_dis']]).reshape(1, 2)
    return pl.pallas_call(
        _attn_last_body,
        grid=(GRID_N,),
        in_specs=[
            pl.BlockSpec((BN, HID), lambda i: (i, 0)),
            pl.BlockSpec((BN, C), lambda i: (i, 0)),
            _const2((1, C)),
            _const2((C, HID)), _const2((C, HID)),
            _const2((HID, HID)), _const2((1, HID)),
            _const2((HID, HID)), _const2((1, HID)),
            _const2((HID, HID)), _const2((1, HID)),
            _const2((HID, HID)), _const2((1, HID)),
            _const2((1, 2)),
            _const2((HID, OUT)), _const2((1, OUT)),
            pl.BlockSpec((BN, OUT), lambda i: (i, 0)),
        ],
        out_specs=pl.BlockSpec((BN, OUT), lambda i: (i, 0)),
        out_shape=jax.ShapeDtypeStruct((N, OUT), jnp.float32),
    )(g, dmat, cnt_row, k, v,
      cp['Wp'], cp['bp'].reshape(1, HID), cp['Wq'], cp['bq'].reshape(1, HID),
      fp['W1'], fp['b1'].reshape(1, HID), fp['W2'], fp['b2'].reshape(1, HID),
      sc, op['W'], op['b'].reshape(1, OUT), xloc)


def _sage1_body(ap_r, x_r, wl_r, bl_r, wr_r, h1a_r, h1b_r, di_r):
    a = ap_r[0] + ap_r[1]
    sums = a[:, :D_IN]
    deg = a[:, D_IN:D_IN + 1]
    deginv = 1.0 / jnp.maximum(deg, 1.0)
    mean = sums * deginv
    h = jnp.dot(mean, wl_r[...], preferred_element_type=jnp.float32) + bl_r[...]
    h = h + jnp.dot(x_r[...], wr_r[...], preferred_element_type=jnp.float32)
    h = jnp.maximum(h, 0.0)
    h1a_r[...] = h[:, :D_IN]
    h1b_r[...] = h[:, D_IN:]
    di_r[...] = jnp.broadcast_to(deginv, (BN, D_IN))


def _sage1(agg1p, x, gp):
    return pl.pallas_call(
        _sage1_body,
        grid=(GRID_N,),
        in_specs=[
            pl.BlockSpec((2, BN, D_IN + AUG), lambda i: (0, i, 0)),
            pl.BlockSpec((BN, D_IN), lambda i: (i, 0)),
            _const2((D_IN, HID)), _const2((1, HID)),
            _const2((D_IN, HID)),
        ],
        out_specs=[
            pl.BlockSpec((BN, D_IN), lambda i: (i, 0)),
            pl.BlockSpec((BN, D_IN), lambda i: (i, 0)),
            pl.BlockSpec((BN, D_IN), lambda i: (i, 0)),
        ],
        out_shape=[
            jax.ShapeDtypeStruct((N, D_IN), jnp.float32),
            jax.ShapeDtypeStruct((N, D_IN), jnp.float32),
            jax.ShapeDtypeStruct((N, D_IN), jnp.float32),
        ],
    )(agg1p, x, gp['Wl'], gp['bl'].reshape(1, HID), gp['Wr'])


def _sage2_body(ag_r, h1a_r, h1b_r, di_r, wl_r, bl_r, wr_r, wl3_r,
                h2_r, z_r):
    di = di_r[...]
    mean = jnp.concatenate([ag_r[0] * di, ag_r[1] * di], axis=1)
    h1 = jnp.concatenate([h1a_r[...], h1b_r[...]], axis=1)
    h = jnp.dot(mean, wl_r[...], preferred_element_type=jnp.float32) + bl_r[...]
    h = h + jnp.dot(h1, wr_r[...], preferred_element_type=jnp.float32)
    h = jnp.maximum(h, 0.0)
    h2_r[...] = h
    z_r[...] = jnp.dot(h, wl3_r[...], preferred_element_type=jnp.float32)


def _sage2(agg2, h1a, h1b, deginv, gp, gp3):
    return pl.pallas_call(
        _sage2_body,
        grid=(GRID_N,),
        in_specs=[
            pl.BlockSpec((2, BN, D_IN), lambda i: (0, i, 0)),
            pl.BlockSpec((BN, D_IN), lambda i: (i, 0)),
            pl.BlockSpec((BN, D_IN), lambda i: (i, 0)),
            pl.BlockSpec((BN, D_IN), lambda i: (i, 0)),
            _const2((HID, HID)), _const2((1, HID)),
            _const2((HID, HID)),
            _const2((HID, OUT)),
        ],
        out_specs=[
            pl.BlockSpec((BN, HID), lambda i: (i, 0)),
            pl.BlockSpec((BN, OUT), lambda i: (i, 0)),
        ],
        out_shape=[
            jax.ShapeDtypeStruct((N, HID), jnp.float32),
            jax.ShapeDtypeStruct((N, OUT), jnp.float32),
        ],
    )(agg2, h1a, h1b, deginv, gp['Wl'], gp['bl'].reshape(1, HID), gp['Wr'],
      gp3['Wl'])


def _sage3_body(ap_r, h2_r, di_r, bl_r, wr_r, xloc_r):
    agg = (ap_r[0] + ap_r[1]) * di_r[...]
    xloc_r[...] = (agg + bl_r[...]
                   + jnp.dot(h2_r[...], wr_r[...],
                             preferred_element_type=jnp.float32))


def _sage3(agg3p, h2, deginv, gp3):
    return pl.pallas_call(
        _sage3_body,
        grid=(GRID_N,),
        in_specs=[
            pl.BlockSpec((2, BN, OUT), lambda i: (0, i, 0)),
            pl.BlockSpec((BN, HID), lambda i: (i, 0)),
            pl.BlockSpec((BN, OUT), lambda i: (i, 0)),
            _const2((1, OUT)),
            _const2((HID, OUT)),
        ],
        out_specs=pl.BlockSpec((BN, OUT), lambda i: (i, 0)),
        out_shape=jax.ShapeDtypeStruct((N, OUT), jnp.float32),
    )(agg3p, h2, deginv, gp3['bl'].reshape(1, OUT), gp3['Wr'])


NTILES = 16
NCH = 250
GRP = 50
NGRP = NCH // GRP
GPAIRS = GRP // 2
ZR = 624
ZR_LAST = N - (NTILES - 1) * ZR


def _make_edge_agg(ncols, edge_split):
    mesh = plsc.VectorSubcoreMesh(core_axis_name="c", subcore_axis_name="s")
    kch = (E // 32 if edge_split else E // NTILES) // NCH

    @functools.partial(
        pl.kernel,
        out_type=jax.ShapeDtypeStruct((2, N, ncols), jnp.float32),
        mesh=mesh,
        scratch_types=[
            pltpu.VMEM_SHARED((N, ncols), jnp.float32),
            pltpu.VMEM((GRP, kch), jnp.int32),
            pltpu.VMEM((GRP, kch), jnp.int32),
            pltpu.VMEM((kch, ncols), jnp.float32),
            pltpu.VMEM((kch, ncols), jnp.float32),
            pltpu.SemaphoreType.DMA,
            pltpu.SemaphoreType.DMA,
            pltpu.SemaphoreType.DMA,
            pltpu.SemaphoreType.DMA,
        ],
        compiler_params=pltpu.CompilerParams(use_tc_tiling_on_sc=False),
    )
    def k(tab_a, tab_b, src2_h, dst2_h, zeros_h, out_h,
          acc, sidxs, didxs, rows0, rows1, g0, g1, s0, s1):
        c = lax.axis_index("c")
        s = lax.axis_index("s")

        @pl.when(s < NTILES - 1)
        def _():
            pltpu.sync_copy(zeros_h.at[pl.ds(0, ZR)],
                            acc.at[pl.ds(s * ZR, ZR)])

        @pl.when(s == NTILES - 1)
        def _():
            pltpu.sync_copy(zeros_h, acc.at[pl.ds((NTILES - 1) * ZR, ZR_LAST)])

        plsc.subcore_barrier()

        def run(tab, cb):
            def group(gi, carry):
                gb = cb + gi * GRP
                pltpu.sync_copy(src2_h.at[pl.ds(gb, GRP)], sidxs)
                pltpu.sync_copy(dst2_h.at[pl.ds(gb, GRP)], didxs)
                pltpu.async_copy(tab.at[sidxs.at[0]], rows0, g0)
                pltpu.async_copy(tab.at[sidxs.at[1]], rows1, g1)

                def pair(j, carry2):
                    i0 = 2 * j
                    pltpu.make_async_copy(tab.at[sidxs.at[i0]], rows0,
                                          g0).wait()
                    pltpu.async_copy(rows0, acc.at[didxs.at[i0]], s0,
                                     add=True)
                    pltpu.make_async_copy(tab.at[sidxs.at[i0 + 1]], rows1,
                                          g1).wait()
                    pltpu.async_copy(rows1, acc.at[didxs.at[i0 + 1]], s1,
                                     add=True)

                    @pl.when(j < GPAIRS - 1)
                    def _():
                        pltpu.make_async_copy(rows0, acc.at[didxs.at[i0]],
                                              s0).wait()
                        pltpu.async_copy(tab.at[sidxs.at[i0 + 2]], rows0, g0)
                        pltpu.make_async_copy(rows1,
                                              acc.at[didxs.at[i0 + 1]],
                                              s1).wait()
                        pltpu.async_copy(tab.at[sidxs.at[i0 + 3]], rows1, g1)

                    return carry2

                lax.fori_loop(0, GPAIRS, pair, 0)
                pltpu.make_async_copy(rows0, acc.at[didxs.at[GRP - 2]],
                                      s0).wait()
                pltpu.make_async_copy(rows1, acc.at[didxs.at[GRP - 1]],
                                      s1).wait()
                return carry

            lax.fori_loop(0, NGRP, group, 0)

        if edge_split:
            run(tab_a, (c * NTILES + s) * NCH)
        else:
            @pl.when(c == 0)
            def _():
                run(tab_a, s * NCH)

            @pl.when(c == 1)
            def _():
                run(tab_b, s * NCH)

        plsc.subcore_barrier()

        @pl.when(s < NTILES - 1)
        def _():
            pltpu.sync_copy(acc.at[pl.ds(s * ZR, ZR)],
                            out_h.at[c, pl.ds(s * ZR, ZR)])

        @pl.when(s == NTILES - 1)
        def _():
            pltpu.sync_copy(acc.at[pl.ds((NTILES - 1) * ZR, ZR_LAST)],
                            out_h.at[c, pl.ds((NTILES - 1) * ZR, ZR_LAST)])

    return k


_edge_agg_aug = _make_edge_agg(D_IN + AUG, True)
_edge_agg_feat = _make_edge_agg(D_IN, False)
_edge_agg_128 = _make_edge_agg(D_IN, True)


def _agg_edge_split_aug(table, src, dst):
    zeros = jnp.zeros((ZR_LAST, D_IN + AUG), jnp.float32)
    return _edge_agg_aug(table, table,
                         src.reshape(-1, 40), dst.reshape(-1, 40), zeros)


def _agg_edge_split_128(table, src, dst):
    zeros = jnp.zeros((ZR_LAST, D_IN), jnp.float32)
    return _edge_agg_128(table, table,
                         src.reshape(-1, 40), dst.reshape(-1, 40), zeros)


def _agg_feat_split(ta, tb, src, dst):
    zeros = jnp.zeros((ZR_LAST, D_IN), jnp.float32)
    return _edge_agg_feat(ta, tb,
                          src.reshape(-1, 80), dst.reshape(-1, 80), zeros)


def kernel(x, edge_index, distance_matrix, nodes_to_community, params):
    src = edge_index[0]
    dst = edge_index[1]
    comm2d = nodes_to_community.reshape(GRID_N, 1, BN)

    x_aug = jnp.concatenate([x, jnp.ones((N, AUG), jnp.float32)], axis=1)
    agg1p = _agg_edge_split_aug(x_aug, src, dst)
    h1a, h1b, deginv = _sage1(agg1p, x, params['gnn'][0])
    agg2 = _agg_feat_split(h1a, h1b, src, dst)
    h2, z = _sage2(agg2, h1a, h1b, deginv, params['gnn'][1], params['gnn'][2])
    agg3p = _agg_edge_split_128(z, src, dst)
    xloc = _sage3(agg3p, h2, deginv, params['gnn'][2])

    g, praw, cnt_row = _fcin(x, comm2d, params['fc_in'])
    k0, v0 = _cprep(praw, params['convs'][0])
    g, praw1 = _attn_mid(g, distance_matrix, cnt_row, k0, v0,
                         params['convs'][0], params['ffs'][0], comm2d)
    k1, v1 = _cprep(praw1, params['convs'][1])
    return _attn_last(g, distance_matrix, cnt_row, k1, v1,
                      params['convs'][1], params['ffs'][1], params['fc_out'],
                      xloc)


# speedup vs baseline: 7.6624x; 1.1025x over previous
import functools

import jax
import jax.numpy as jnp
from jax import lax
from jax.experimental import pallas as pl
from jax.experimental.pallas import tpu as pltpu
from jax.experimental.pallas import tpu_sc as plsc

N = 10000
E = 320000
D_IN = 128
HID = 256
OUT = 128
C = 512
HEADS = 4
DHEAD = HID // HEADS
SCALE = 1.0 / (DHEAD ** 0.5)
BN = 1000
GRID_N = N // BN
AUG = 16


def _const2(shape):
    return pl.BlockSpec(shape, lambda i: (0, 0))


def _fcin_body(x_r, w1_r, b1_r, w2_r, b2_r, comm_r,
               g_r, praw_r, cnt_r):
    i = pl.program_id(0)
    g = jnp.maximum(jnp.dot(x_r[...], w1_r[...],
                            preferred_element_type=jnp.float32) + b1_r[...], 0.0)
    g = jnp.dot(g, w2_r[...], preferred_element_type=jnp.float32) + b2_r[...]
    g_r[...] = g
    comm = comm_r[0, 0, :]
    m2 = (comm[:, None] == lax.broadcasted_iota(jnp.int32, (BN, C), 1)
          ).astype(jnp.float32)
    g_aug = jnp.concatenate([g, jnp.ones((BN, AUG), jnp.float32)], axis=1)
    pb = lax.dot_general(m2, g_aug, (((0,), (0,)), ((), ())),
                         preferred_element_type=jnp.float32)
    cb = jnp.sum(m2, axis=0, keepdims=True)

    @pl.when(i == 0)
    def _():
        praw_r[...] = pb
        cnt_r[...] = cb

    @pl.when(i > 0)
    def _():
        praw_r[...] += pb
        cnt_r[...] += cb


def _fcin(x, comm2d, fp):
    return pl.pallas_call(
        _fcin_body,
        grid=(GRID_N,),
        in_specs=[
            pl.BlockSpec((BN, D_IN), lambda i: (i, 0)),
            _const2((D_IN, HID)), _const2((1, HID)),
            _const2((HID, HID)), _const2((1, HID)),
            pl.BlockSpec((1, 1, BN), lambda i: (i, 0, 0)),
        ],
        out_specs=[
            pl.BlockSpec((BN, HID), lambda i: (i, 0)),
            pl.BlockSpec((C, HID + AUG), lambda i: (0, 0)),
            pl.BlockSpec((1, C), lambda i: (0, 0)),
        ],
        out_shape=[
            jax.ShapeDtypeStruct((N, HID), jnp.float32),
            jax.ShapeDtypeStruct((C, HID + AUG), jnp.float32),
            jax.ShapeDtypeStruct((1, C), jnp.float32),
        ],
    )(x, fp['W1'], fp['b1'].reshape(1, HID), fp['W2'], fp['b2'].reshape(1, HID),
      comm2d)


def _cprep_body(pa_r, wk_r, bk_r, wv_r, bv_r, k_r, v_r):
    praw = pa_r[:, :HID]
    cnt = pa_r[:, HID:HID + 1]
    inv = 1.0 / jnp.maximum(cnt, 1.0)
    cavg = praw * inv
    k_r[...] = jnp.dot(cavg, wk_r[...],
                       preferred_element_type=jnp.float32) + bk_r[...]
    v_r[...] = jnp.dot(cavg, wv_r[...],
                       preferred_element_type=jnp.float32) + bv_r[...]


def _cprep(praw_aug, cp):
    return pl.pallas_call(
        _cprep_body,
        grid=(1,),
        in_specs=[
            _const2((C, HID + AUG)),
            _const2((HID, HID)), _const2((1, HID)),
            _const2((HID, HID)), _const2((1, HID)),
        ],
        out_specs=[
            pl.BlockSpec((C, HID), lambda i: (0, 0)),
            pl.BlockSpec((C, HID), lambda i: (0, 0)),
        ],
        out_shape=[
            jax.ShapeDtypeStruct((C, HID), jnp.float32),
            jax.ShapeDtypeStruct((C, HID), jnp.float32),
        ],
    )(praw_aug, cp['Wk'], cp['bk'].reshape(1, HID),
      cp['Wv'], cp['bv'].reshape(1, HID))


def _attn_core(g, dmat, cnt_r, k_r, v_r, wp_r, bp_r, wq_r, bq_r,
               w1_r, b1_r, w2_r, b2_r, wdis, bdis):
    q = jnp.dot(g, wp_r[...], preferred_element_type=jnp.float32) + bp_r[...]
    q = jnp.dot(q, wq_r[...], preferred_element_type=jnp.float32) + bq_r[...]
    base = dmat * wdis + bdis + jnp.log(cnt_r[...])
    outs = []
    for h in range(HEADS):
        sl = slice(h * DHEAD, (h + 1) * DHEAD)
        qh = q[:, sl]
        kh = k_r[:, sl]
        vh = v_r[:, sl]
        dots = lax.dot_general(qh, kh, (((1,), (1,)), ((), ())),
                               preferred_element_type=jnp.float32) * SCALE + base
        mx = jnp.max(dots, axis=-1, keepdims=True)
        e = jnp.exp(dots - mx)
        s = jnp.sum(e, axis=-1, keepdims=True)
        outs.append(jnp.dot(e, vh, preferred_element_type=jnp.float32) / s)
    o = jnp.concatenate(outs, axis=1)
    o = jnp.maximum(jnp.dot(o, w1_r[...],
                            preferred_element_type=jnp.float32) + b1_r[...], 0.0)
    return jnp.maximum(jnp.dot(o, w2_r[...],
                               preferred_element_type=jnp.float32) + b2_r[...], 0.0)


def _attn_mid_body(g_r, dm_r, cnt_r, k_r, v_r, wp_r, bp_r, wq_r, bq_r,
                   w1_r, b1_r, w2_r, b2_r, sc_r, comm_r,
                   gn_r, praw_r):
    i = pl.program_id(0)
    gn = _attn_core(g_r[...], dm_r[...], cnt_r, k_r, v_r, wp_r, bp_r,
                    wq_r, bq_r, w1_r, b1_r, w2_r, b2_r,
                    sc_r[0, 0], sc_r[0, 1])
    gn_r[...] = gn
    comm = comm_r[0, 0, :]
    m2 = (comm[:, None] == lax.broadcasted_iota(jnp.int32, (BN, C), 1)
          ).astype(jnp.float32)
    g_aug = jnp.concatenate([gn, jnp.ones((BN, AUG), jnp.float32)], axis=1)
    pb = lax.dot_general(m2, g_aug, (((0,), (0,)), ((), ())),
                         preferred_element_type=jnp.float32)

    @pl.when(i == 0)
    def _():
        praw_r[...] = pb

    @pl.when(i > 0)
    def _():
        praw_r[...] += pb


def _attn_mid(g, dmat, cnt_row, k, v, cp, fp, comm2d):
    sc = jnp.stack([cp['w_dis'], cp['b_dis']]).reshape(1, 2)
    return pl.pallas_call(
        _attn_mid_body,
        grid=(GRID_N,),
        in_specs=[
            pl.BlockSpec((BN, HID), lambda i: (i, 0)),
            pl.BlockSpec((BN, C), lambda i: (i, 0)),
            _const2((1, C)),
            _const2((C, HID)), _const2((C, HID)),
            _const2((HID, HID)), _const2((1, HID)),
            _const2((HID, HID)), _const2((1, HID)),
            _const2((HID, HID)), _const2((1, HID)),
            _const2((HID, HID)), _const2((1, HID)),
            _const2((1, 2)),
            pl.BlockSpec((1, 1, BN), lambda i: (i, 0, 0)),
        ],
        out_specs=[
            pl.BlockSpec((BN, HID), lambda i: (i, 0)),
            pl.BlockSpec((C, HID + AUG), lambda i: (0, 0)),
        ],
        out_shape=[
            jax.ShapeDtypeStruct((N, HID), jnp.float32),
            jax.ShapeDtypeStruct((C, HID + AUG), jnp.float32),
        ],
    )(g, dmat, cnt_row, k, v,
      cp['Wp'], cp['bp'].reshape(1, HID), cp['Wq'], cp['bq'].reshape(1, HID),
      fp['W1'], fp['b1'].reshape(1, HID), fp['W2'], fp['b2'].reshape(1, HID),
      sc, comm2d)


def _attn_last_body(g_r, dm_r, cnt_r, k_r, v_r, wp_r, bp_r, wq_r, bq_r,
                    w1_r, b1_r, w2_r, b2_r, sc_r, wo_r, bo_r,
                    out_r):
    gn = _attn_core(g_r[...], dm_r[...], cnt_r, k_r, v_r, wp_r, bp_r,
                    wq_r, bq_r, w1_r, b1_r, w2_r, b2_r,
                    sc_r[0, 0], sc_r[0, 1])
    out_r[...] = (jnp.dot(gn, wo_r[...], preferred_element_type=jnp.float32)
                  + bo_r[...])


def _attn_last(g, dmat, cnt_row, k, v, cp, fp, op):
    sc = jnp.stack([cp['w_dis'], cp['b_dis']]).reshape(1, 2)
    return pl.pallas_call(
        _attn_last_body,
        grid=(GRID_N,),
        in_specs=[
            pl.BlockSpec((BN, HID), lambda i: (i, 0)),
            pl.BlockSpec((BN, C), lambda i: (i, 0)),
            _const2((1, C)),
            _const2((C, HID)), _const2((C, HID)),
            _const2((HID, HID)), _const2((1, HID)),
            _const2((HID, HID)), _const2((1, HID)),
            _const2((HID, HID)), _const2((1, HID)),
            _const2((HID, HID)), _const2((1, HID)),
            _const2((1, 2)),
            _const2((HID, OUT)), _const2((1, OUT)),
        ],
        out_specs=pl.BlockSpec((BN, OUT), lambda i: (i, 0)),
        out_shape=jax.ShapeDtypeStruct((N, OUT), jnp.float32),
    )(g, dmat, cnt_row, k, v,
      cp['Wp'], cp['bp'].reshape(1, HID), cp['Wq'], cp['bq'].reshape(1, HID),
      fp['W1'], fp['b1'].reshape(1, HID), fp['W2'], fp['b2'].reshape(1, HID),
      sc, op['W'], op['b'].reshape(1, OUT))


def _sage1_body(ap_r, x_r, wl_r, bl_r, wr_r, h1a_r, h1b_r, di_r):
    a = ap_r[0] + ap_r[1]
    sums = a[:, :D_IN]
    deg = a[:, D_IN:D_IN + 1]
    deginv = 1.0 / jnp.maximum(deg, 1.0)
    mean = sums * deginv
    h = jnp.dot(mean, wl_r[...], preferred_element_type=jnp.float32) + bl_r[...]
    h = h + jnp.dot(x_r[...], wr_r[...], preferred_element_type=jnp.float32)
    h = jnp.maximum(h, 0.0)
    h1a_r[...] = h[:, :D_IN]
    h1b_r[...] = h[:, D_IN:]
    di_r[...] = jnp.broadcast_to(deginv, (BN, D_IN))


def _sage1(agg1p, x, gp):
    return pl.pallas_call(
        _sage1_body,
        grid=(GRID_N,),
        in_specs=[
            pl.BlockSpec((2, BN, D_IN + AUG), lambda i: (0, i, 0)),
            pl.BlockSpec((BN, D_IN), lambda i: (i, 0)),
            _const2((D_IN, HID)), _const2((1, HID)),
            _const2((D_IN, HID)),
        ],
        out_specs=[
            pl.BlockSpec((BN, D_IN), lambda i: (i, 0)),
            pl.BlockSpec((BN, D_IN), lambda i: (i, 0)),
            pl.BlockSpec((BN, D_IN), lambda i: (i, 0)),
        ],
        out_shape=[
            jax.ShapeDtypeStruct((N, D_IN), jnp.float32),
            jax.ShapeDtypeStruct((N, D_IN), jnp.float32),
            jax.ShapeDtypeStruct((N, D_IN), jnp.float32),
        ],
    )(agg1p, x, gp['Wl'], gp['bl'].reshape(1, HID), gp['Wr'])


def _sage2_body(ag_r, h1a_r, h1b_r, di_r, wl_r, bl_r, wr_r, wl3_r,
                h2_r, z_r):
    di = di_r[...]
    mean = jnp.concatenate([ag_r[0] * di, ag_r[1] * di], axis=1)
    h1 = jnp.concatenate([h1a_r[...], h1b_r[...]], axis=1)
    h = jnp.dot(mean, wl_r[...], preferred_element_type=jnp.float32) + bl_r[...]
    h = h + jnp.dot(h1, wr_r[...], preferred_element_type=jnp.float32)
    h = jnp.maximum(h, 0.0)
    h2_r[...] = h
    z_r[...] = jnp.dot(h, wl3_r[...], preferred_element_type=jnp.float32)


def _sage2(agg2, h1a, h1b, deginv, gp, gp3):
    return pl.pallas_call(
        _sage2_body,
        grid=(GRID_N,),
        in_specs=[
            pl.BlockSpec((2, BN, D_IN), lambda i: (0, i, 0)),
            pl.BlockSpec((BN, D_IN), lambda i: (i, 0)),
            pl.BlockSpec((BN, D_IN), lambda i: (i, 0)),
            pl.BlockSpec((BN, D_IN), lambda i: (i, 0)),
            _const2((HID, HID)), _const2((1, HID)),
            _const2((HID, HID)),
            _const2((HID, OUT)),
        ],
        out_specs=[
            pl.BlockSpec((BN, HID), lambda i: (i, 0)),
            pl.BlockSpec((BN, OUT), lambda i: (i, 0)),
        ],
        out_shape=[
            jax.ShapeDtypeStruct((N, HID), jnp.float32),
            jax.ShapeDtypeStruct((N, OUT), jnp.float32),
        ],
    )(agg2, h1a, h1b, deginv, gp['Wl'], gp['bl'].reshape(1, HID), gp['Wr'],
      gp3['Wl'])


def _sage3_body(ap_r, h2_r, di_r, xg_r, bl_r, wr_r, out_r):
    agg = (ap_r[0] + ap_r[1]) * di_r[...]
    out_r[...] = (agg + bl_r[...] + xg_r[...]
                  + jnp.dot(h2_r[...], wr_r[...],
                            preferred_element_type=jnp.float32))


def _sage3(agg3p, h2, deginv, xglobal, gp3):
    return pl.pallas_call(
        _sage3_body,
        grid=(GRID_N,),
        in_specs=[
            pl.BlockSpec((2, BN, OUT), lambda i: (0, i, 0)),
            pl.BlockSpec((BN, HID), lambda i: (i, 0)),
            pl.BlockSpec((BN, OUT), lambda i: (i, 0)),
            pl.BlockSpec((BN, OUT), lambda i: (i, 0)),
            _const2((1, OUT)),
            _const2((HID, OUT)),
        ],
        out_specs=pl.BlockSpec((BN, OUT), lambda i: (i, 0)),
        out_shape=jax.ShapeDtypeStruct((N, OUT), jnp.float32),
    )(agg3p, h2, deginv, xglobal, gp3['bl'].reshape(1, OUT), gp3['Wr'])


NTILES = 16
ZR = 624
ZR_LAST = N - (NTILES - 1) * ZR


def _make_edge_agg(ncols, edge_split, kch, grp):
    mesh = plsc.VectorSubcoreMesh(core_axis_name="c", subcore_axis_name="s")
    nch = (E // 32 if edge_split else E // NTILES) // kch
    ngrp = nch // grp
    gpairs = grp // 2
    tail = grp % 2

    @functools.partial(
        pl.kernel,
        out_type=jax.ShapeDtypeStruct((2, N, ncols), jnp.float32),
        mesh=mesh,
        scratch_types=[
            pltpu.VMEM_SHARED((N, ncols), jnp.float32),
            pltpu.VMEM((grp, kch), jnp.int32),
            pltpu.VMEM((grp, kch), jnp.int32),
            pltpu.VMEM((kch, ncols), jnp.float32),
            pltpu.VMEM((kch, ncols), jnp.float32),
            pltpu.SemaphoreType.DMA,
            pltpu.SemaphoreType.DMA,
            pltpu.SemaphoreType.DMA,
            pltpu.SemaphoreType.DMA,
        ],
        compiler_params=pltpu.CompilerParams(use_tc_tiling_on_sc=False),
    )
    def k(tab_a, tab_b, src2_h, dst2_h, zeros_h, out_h,
          acc, sidxs, didxs, rows0, rows1, g0, g1, s0, s1):
        c = lax.axis_index("c")
        s = lax.axis_index("s")

        @pl.when(s < NTILES - 1)
        def _():
            pltpu.sync_copy(zeros_h.at[pl.ds(0, ZR)],
                            acc.at[pl.ds(s * ZR, ZR)])

        @pl.when(s == NTILES - 1)
        def _():
            pltpu.sync_copy(zeros_h, acc.at[pl.ds((NTILES - 1) * ZR, ZR_LAST)])

        plsc.subcore_barrier()

        def run(tab, cb):
            def group(gi, carry):
                gb = cb + gi * grp
                pltpu.sync_copy(src2_h.at[pl.ds(gb, grp)], sidxs)
                pltpu.sync_copy(dst2_h.at[pl.ds(gb, grp)], didxs)
                pltpu.async_copy(tab.at[sidxs.at[0]], rows0, g0)
                pltpu.async_copy(tab.at[sidxs.at[1]], rows1, g1)

                def pair(j, carry2):
                    i0 = 2 * j
                    pltpu.make_async_copy(tab.at[sidxs.at[i0]], rows0,
                                          g0).wait()
                    pltpu.async_copy(rows0, acc.at[didxs.at[i0]], s0,
                                     add=True)
                    pltpu.make_async_copy(tab.at[sidxs.at[i0 + 1]], rows1,
                                          g1).wait()
                    pltpu.async_copy(rows1, acc.at[didxs.at[i0 + 1]], s1,
                                     add=True)

                    @pl.when(j < gpairs - 1)
                    def _():
                        pltpu.make_async_copy(rows0, acc.at[didxs.at[i0]],
                                              s0).wait()
                        pltpu.async_copy(tab.at[sidxs.at[i0 + 2]], rows0, g0)
                        pltpu.make_async_copy(rows1,
                                              acc.at[didxs.at[i0 + 1]],
                                              s1).wait()
                        pltpu.async_copy(tab.at[sidxs.at[i0 + 3]], rows1, g1)

                    return carry2

                lax.fori_loop(0, gpairs, pair, 0)
                pltpu.make_async_copy(rows0, acc.at[didxs.at[2 * gpairs - 2]],
                                      s0).wait()
                pltpu.make_async_copy(rows1, acc.at[didxs.at[2 * gpairs - 1]],
                                      s1).wait()
                if tail:
                    pltpu.async_copy(tab.at[sidxs.at[grp - 1]], rows0,
                                     g0).wait()
                    pltpu.async_copy(rows0, acc.at[didxs.at[grp - 1]], s0,
                                     add=True).wait()
                return carry

            lax.fori_loop(0, ngrp, group, 0)

        if edge_split:
            run(tab_a, (c * NTILES + s) * nch)
        else:
            @pl.when(c == 0)
            def _():
                run(tab_a, s * nch)

            @pl.when(c == 1)
            def _():
                run(tab_b, s * nch)

        plsc.subcore_barrier()

        @pl.when(s < NTILES - 1)
        def _():
            pltpu.sync_copy(acc.at[pl.ds(s * ZR, ZR)],
                            out_h.at[c, pl.ds(s * ZR, ZR)])

        @pl.when(s == NTILES - 1)
        def _():
            pltpu.sync_copy(acc.at[pl.ds((NTILES - 1) * ZR, ZR_LAST)],
                            out_h.at[c, pl.ds((NTILES - 1) * ZR, ZR_LAST)])

    return k


_edge_agg_aug = _make_edge_agg(D_IN + AUG, True, 40, 50)
_edge_agg_feat = _make_edge_agg(D_IN, False, 80, 50)
_edge_agg_128 = _make_edge_agg(D_IN, True, 80, 25)


def _agg_edge_split_aug(table, src, dst):
    zeros = jnp.zeros((ZR_LAST, D_IN + AUG), jnp.float32)
    return _edge_agg_aug(table, table,
                         src.reshape(-1, 40), dst.reshape(-1, 40), zeros)


def _agg_edge_split_128(table, src, dst):
    zeros = jnp.zeros((ZR_LAST, D_IN), jnp.float32)
    return _edge_agg_128(table, table,
                         src.reshape(-1, 80), dst.reshape(-1, 80), zeros)


def _agg_feat_split(ta, tb, src, dst):
    zeros = jnp.zeros((ZR_LAST, D_IN), jnp.float32)
    return _edge_agg_feat(ta, tb,
                          src.reshape(-1, 80), dst.reshape(-1, 80), zeros)


def kernel(x, edge_index, distance_matrix, nodes_to_community, params):
    src = edge_index[0]
    dst = edge_index[1]
    comm2d = nodes_to_community.reshape(GRID_N, 1, BN)

    g, praw, cnt_row = _fcin(x, comm2d, params['fc_in'])
    k0, v0 = _cprep(praw, params['convs'][0])
    g, praw1 = _attn_mid(g, distance_matrix, cnt_row, k0, v0,
                         params['convs'][0], params['ffs'][0], comm2d)
    k1, v1 = _cprep(praw1, params['convs'][1])
    xglobal = _attn_last(g, distance_matrix, cnt_row, k1, v1,
                         params['convs'][1], params['ffs'][1],
                         params['fc_out'])

    x_aug = jnp.concatenate([x, jnp.ones((N, AUG), jnp.float32)], axis=1)
    agg1p = _agg_edge_split_aug(x_aug, src, dst)
    h1a, h1b, deginv = _sage1(agg1p, x, params['gnn'][0])
    agg2 = _agg_feat_split(h1a, h1b, src, dst)
    h2, z = _sage2(agg2, h1a, h1b, deginv, params['gnn'][1], params['gnn'][2])
    agg3p = _agg_edge_split_128(z, src, dst)
    return _sage3(agg3p, h2, deginv, xglobal, params['gnn'][2])


# speedup vs baseline: 7.8337x; 1.0224x over previous
import functools

import jax
import jax.numpy as jnp
from jax import lax
from jax.experimental import pallas as pl
from jax.experimental.pallas import tpu as pltpu
from jax.experimental.pallas import tpu_sc as plsc

N = 10000
E = 320000
D_IN = 128
HID = 256
OUT = 128
C = 512
HEADS = 4
DHEAD = HID // HEADS
SCALE = 1.0 / (DHEAD ** 0.5)
BN = 1000
GRID_N = N // BN
AUG = 16


def _const2(shape):
    return pl.BlockSpec(shape, lambda i: (0, 0))


def _fcin_body(x_r, w1_r, b1_r, w2_r, b2_r, comm_r,
               g_r, praw_r, cnt_r):
    i = pl.program_id(0)
    g = jnp.maximum(jnp.dot(x_r[...], w1_r[...],
                            preferred_element_type=jnp.float32) + b1_r[...], 0.0)
    g = jnp.dot(g, w2_r[...], preferred_element_type=jnp.float32) + b2_r[...]
    g_r[...] = g
    comm = comm_r[0, 0, :]
    m2 = (comm[:, None] == lax.broadcasted_iota(jnp.int32, (BN, C), 1)
          ).astype(jnp.float32)
    g_aug = jnp.concatenate([g, jnp.ones((BN, AUG), jnp.float32)], axis=1)
    pb = lax.dot_general(m2, g_aug, (((0,), (0,)), ((), ())),
                         preferred_element_type=jnp.float32)
    cb = jnp.sum(m2, axis=0, keepdims=True)

    @pl.when(i == 0)
    def _():
        praw_r[...] = pb
        cnt_r[...] = cb

    @pl.when(i > 0)
    def _():
        praw_r[...] += pb
        cnt_r[...] += cb


def _fcin(x, comm2d, fp):
    return pl.pallas_call(
        _fcin_body,
        grid=(GRID_N,),
        in_specs=[
            pl.BlockSpec((BN, D_IN), lambda i: (i, 0)),
            _const2((D_IN, HID)), _const2((1, HID)),
            _const2((HID, HID)), _const2((1, HID)),
            pl.BlockSpec((1, 1, BN), lambda i: (i, 0, 0)),
        ],
        out_specs=[
            pl.BlockSpec((BN, HID), lambda i: (i, 0)),
            pl.BlockSpec((C, HID + AUG), lambda i: (0, 0)),
            pl.BlockSpec((1, C), lambda i: (0, 0)),
        ],
        out_shape=[
            jax.ShapeDtypeStruct((N, HID), jnp.float32),
            jax.ShapeDtypeStruct((C, HID + AUG), jnp.float32),
            jax.ShapeDtypeStruct((1, C), jnp.float32),
        ],
    )(x, fp['W1'], fp['b1'].reshape(1, HID), fp['W2'], fp['b2'].reshape(1, HID),
      comm2d)


def _cprep_body(pa_r, wk_r, bk_r, wv_r, bv_r, k_r, v_r):
    praw = pa_r[:, :HID]
    cnt = pa_r[:, HID:HID + 1]
    inv = 1.0 / jnp.maximum(cnt, 1.0)
    cavg = praw * inv
    k_r[...] = jnp.dot(cavg, wk_r[...],
                       preferred_element_type=jnp.float32) + bk_r[...]
    v_r[...] = jnp.dot(cavg, wv_r[...],
                       preferred_element_type=jnp.float32) + bv_r[...]


def _cprep(praw_aug, cp):
    return pl.pallas_call(
        _cprep_body,
        grid=(1,),
        in_specs=[
            _const2((C, HID + AUG)),
            _const2((HID, HID)), _const2((1, HID)),
            _const2((HID, HID)), _const2((1, HID)),
        ],
        out_specs=[
            pl.BlockSpec((C, HID), lambda i: (0, 0)),
            pl.BlockSpec((C, HID), lambda i: (0, 0)),
        ],
        out_shape=[
            jax.ShapeDtypeStruct((C, HID), jnp.float32),
            jax.ShapeDtypeStruct((C, HID), jnp.float32),
        ],
    )(praw_aug, cp['Wk'], cp['bk'].reshape(1, HID),
      cp['Wv'], cp['bv'].reshape(1, HID))


def _attn_core(g, dmat, cnt_r, k_r, v_r, wp_r, bp_r, wq_r, bq_r,
               w1_r, b1_r, w2_r, b2_r, wdis, bdis):
    q = jnp.dot(g, wp_r[...], preferred_element_type=jnp.float32) + bp_r[...]
    q = jnp.dot(q, wq_r[...], preferred_element_type=jnp.float32) + bq_r[...]
    base = dmat * wdis + bdis + jnp.log(cnt_r[...])
    outs = []
    for h in range(HEADS):
        sl = slice(h * DHEAD, (h + 1) * DHEAD)
        qh = q[:, sl]
        kh = k_r[:, sl]
        vh = v_r[:, sl]
        dots = lax.dot_general(qh, kh, (((1,), (1,)), ((), ())),
                               preferred_element_type=jnp.float32) * SCALE + base
        mx = jnp.max(dots, axis=-1, keepdims=True)
        e = jnp.exp(dots - mx)
        s = jnp.sum(e, axis=-1, keepdims=True)
        outs.append(jnp.dot(e, vh, preferred_element_type=jnp.float32) / s)
    o = jnp.concatenate(outs, axis=1)
    o = jnp.maximum(jnp.dot(o, w1_r[...],
                            preferred_element_type=jnp.float32) + b1_r[...], 0.0)
    return jnp.maximum(jnp.dot(o, w2_r[...],
                               preferred_element_type=jnp.float32) + b2_r[...], 0.0)


def _attn_mid_body(g_r, dm_r, cnt_r, k_r, v_r, wp_r, bp_r, wq_r, bq_r,
                   w1_r, b1_r, w2_r, b2_r, sc_r, comm_r,
                   gn_r, praw_r):
    i = pl.program_id(0)
    gn = _attn_core(g_r[...], dm_r[...], cnt_r, k_r, v_r, wp_r, bp_r,
                    wq_r, bq_r, w1_r, b1_r, w2_r, b2_r,
                    sc_r[0, 0], sc_r[0, 1])
    gn_r[...] = gn
    comm = comm_r[0, 0, :]
    m2 = (comm[:, None] == lax.broadcasted_iota(jnp.int32, (BN, C), 1)
          ).astype(jnp.float32)
    g_aug = jnp.concatenate([gn, jnp.ones((BN, AUG), jnp.float32)], axis=1)
    pb = lax.dot_general(m2, g_aug, (((0,), (0,)), ((), ())),
                         preferred_element_type=jnp.float32)

    @pl.when(i == 0)
    def _():
        praw_r[...] = pb

    @pl.when(i > 0)
    def _():
        praw_r[...] += pb


def _attn_mid(g, dmat, cnt_row, k, v, cp, fp, comm2d):
    sc = jnp.stack([cp['w_dis'], cp['b_dis']]).reshape(1, 2)
    return pl.pallas_call(
        _attn_mid_body,
        grid=(GRID_N,),
        in_specs=[
            pl.BlockSpec((BN, HID), lambda i: (i, 0)),
            pl.BlockSpec((BN, C), lambda i: (i, 0)),
            _const2((1, C)),
            _const2((C, HID)), _const2((C, HID)),
            _const2((HID, HID)), _const2((1, HID)),
            _const2((HID, HID)), _const2((1, HID)),
            _const2((HID, HID)), _const2((1, HID)),
            _const2((HID, HID)), _const2((1, HID)),
            _const2((1, 2)),
            pl.BlockSpec((1, 1, BN), lambda i: (i, 0, 0)),
        ],
        out_specs=[
            pl.BlockSpec((BN, HID), lambda i: (i, 0)),
            pl.BlockSpec((C, HID + AUG), lambda i: (0, 0)),
        ],
        out_shape=[
            jax.ShapeDtypeStruct((N, HID), jnp.float32),
            jax.ShapeDtypeStruct((C, HID + AUG), jnp.float32),
        ],
    )(g, dmat, cnt_row, k, v,
      cp['Wp'], cp['bp'].reshape(1, HID), cp['Wq'], cp['bq'].reshape(1, HID),
      fp['W1'], fp['b1'].reshape(1, HID), fp['W2'], fp['b2'].reshape(1, HID),
      sc, comm2d)


def _attn_last_body(g_r, dm_r, cnt_r, k_r, v_r, wp_r, bp_r, wq_r, bq_r,
                    w1_r, b1_r, w2_r, b2_r, sc_r, wo_r, bo_r,
                    out_r):
    gn = _attn_core(g_r[...], dm_r[...], cnt_r, k_r, v_r, wp_r, bp_r,
                    wq_r, bq_r, w1_r, b1_r, w2_r, b2_r,
                    sc_r[0, 0], sc_r[0, 1])
    out_r[...] = (jnp.dot(gn, wo_r[...], preferred_element_type=jnp.float32)
                  + bo_r[...])


def _attn_last(g, dmat, cnt_row, k, v, cp, fp, op):
    sc = jnp.stack([cp['w_dis'], cp['b_dis']]).reshape(1, 2)
    return pl.pallas_call(
        _attn_last_body,
        grid=(GRID_N,),
        in_specs=[
            pl.BlockSpec((BN, HID), lambda i: (i, 0)),
            pl.BlockSpec((BN, C), lambda i: (i, 0)),
            _const2((1, C)),
            _const2((C, HID)), _const2((C, HID)),
            _const2((HID, HID)), _const2((1, HID)),
            _const2((HID, HID)), _const2((1, HID)),
            _const2((HID, HID)), _const2((1, HID)),
            _const2((HID, HID)), _const2((1, HID)),
            _const2((1, 2)),
            _const2((HID, OUT)), _const2((1, OUT)),
        ],
        out_specs=pl.BlockSpec((BN, OUT), lambda i: (i, 0)),
        out_shape=jax.ShapeDtypeStruct((N, OUT), jnp.float32),
    )(g, dmat, cnt_row, k, v,
      cp['Wp'], cp['bp'].reshape(1, HID), cp['Wq'], cp['bq'].reshape(1, HID),
      fp['W1'], fp['b1'].reshape(1, HID), fp['W2'], fp['b2'].reshape(1, HID),
      sc, op['W'], op['b'].reshape(1, OUT))


def _sage1_body(ap_r, x_r, wl_r, bl_r, wr_r, h1a_r, h1b_r, di_r):
    a = ap_r[0] + ap_r[1]
    sums = a[:, :D_IN]
    deg = a[:, D_IN:D_IN + 1]
    deginv = 1.0 / jnp.maximum(deg, 1.0)
    mean = sums * deginv
    h = jnp.dot(mean, wl_r[...], preferred_element_type=jnp.float32) + bl_r[...]
    h = h + jnp.dot(x_r[...], wr_r[...], preferred_element_type=jnp.float32)
    h = jnp.maximum(h, 0.0)
    h1a_r[...] = h[:, :D_IN]
    h1b_r[...] = h[:, D_IN:]
    di_r[...] = jnp.broadcast_to(deginv, (BN, D_IN))


def _sage1(agg1p, x, gp):
    return pl.pallas_call(
        _sage1_body,
        grid=(GRID_N,),
        in_specs=[
            pl.BlockSpec((2, BN, D_IN + AUG), lambda i: (0, i, 0)),
            pl.BlockSpec((BN, D_IN), lambda i: (i, 0)),
            _const2((D_IN, HID)), _const2((1, HID)),
            _const2((D_IN, HID)),
        ],
        out_specs=[
            pl.BlockSpec((BN, D_IN), lambda i: (i, 0)),
            pl.BlockSpec((BN, D_IN), lambda i: (i, 0)),
            pl.BlockSpec((BN, D_IN), lambda i: (i, 0)),
        ],
        out_shape=[
            jax.ShapeDtypeStruct((N, D_IN), jnp.float32),
            jax.ShapeDtypeStruct((N, D_IN), jnp.float32),
            jax.ShapeDtypeStruct((N, D_IN), jnp.float32),
        ],
    )(agg1p, x, gp['Wl'], gp['bl'].reshape(1, HID), gp['Wr'])


def _sage2_body(ag_r, h1a_r, h1b_r, di_r, wl_r, bl_r, wr_r, wl3_r,
                h2_r, z_r):
    di = di_r[...]
    mean = jnp.concatenate([ag_r[0] * di, ag_r[1] * di], axis=1)
    h1 = jnp.concatenate([h1a_r[...], h1b_r[...]], axis=1)
    h = jnp.dot(mean, wl_r[...], preferred_element_type=jnp.float32) + bl_r[...]
    h = h + jnp.dot(h1, wr_r[...], preferred_element_type=jnp.float32)
    h = jnp.maximum(h, 0.0)
    h2_r[...] = h
    z_r[...] = jnp.dot(h, wl3_r[...], preferred_element_type=jnp.float32)


def _sage2(agg2, h1a, h1b, deginv, gp, gp3):
    return pl.pallas_call(
        _sage2_body,
        grid=(GRID_N,),
        in_specs=[
            pl.BlockSpec((2, BN, D_IN), lambda i: (0, i, 0)),
            pl.BlockSpec((BN, D_IN), lambda i: (i, 0)),
            pl.BlockSpec((BN, D_IN), lambda i: (i, 0)),
            pl.BlockSpec((BN, D_IN), lambda i: (i, 0)),
            _const2((HID, HID)), _const2((1, HID)),
            _const2((HID, HID)),
            _const2((HID, OUT)),
        ],
        out_specs=[
            pl.BlockSpec((BN, HID), lambda i: (i, 0)),
            pl.BlockSpec((BN, OUT), lambda i: (i, 0)),
        ],
        out_shape=[
            jax.ShapeDtypeStruct((N, HID), jnp.float32),
            jax.ShapeDtypeStruct((N, OUT), jnp.float32),
        ],
    )(agg2, h1a, h1b, deginv, gp['Wl'], gp['bl'].reshape(1, HID), gp['Wr'],
      gp3['Wl'])


def _sage3_body(ap_r, h2_r, di_r, xg_r, bl_r, wr_r, out_r):
    agg = (ap_r[0] + ap_r[1]) * di_r[...]
    out_r[...] = (agg + bl_r[...] + xg_r[...]
                  + jnp.dot(h2_r[...], wr_r[...],
                            preferred_element_type=jnp.float32))


def _sage3(agg3p, h2, deginv, xglobal, gp3):
    return pl.pallas_call(
        _sage3_body,
        grid=(GRID_N,),
        in_specs=[
            pl.BlockSpec((2, BN, OUT), lambda i: (0, i, 0)),
            pl.BlockSpec((BN, HID), lambda i: (i, 0)),
            pl.BlockSpec((BN, OUT), lambda i: (i, 0)),
            pl.BlockSpec((BN, OUT), lambda i: (i, 0)),
            _const2((1, OUT)),
            _const2((HID, OUT)),
        ],
        out_specs=pl.BlockSpec((BN, OUT), lambda i: (i, 0)),
        out_shape=jax.ShapeDtypeStruct((N, OUT), jnp.float32),
    )(agg3p, h2, deginv, xglobal, gp3['bl'].reshape(1, OUT), gp3['Wr'])


NTILES = 16
ZR = 624
ZR_LAST = N - (NTILES - 1) * ZR


def _make_edge_agg(ncols, edge_split, kch, grp):
    mesh = plsc.VectorSubcoreMesh(core_axis_name="c", subcore_axis_name="s")
    nch = (E // 32 if edge_split else E // NTILES) // kch
    ngrp = nch // grp
    gpairs = grp // 2
    tail = grp % 2

    @functools.partial(
        pl.kernel,
        out_type=jax.ShapeDtypeStruct((2, N, ncols), jnp.float32),
        mesh=mesh,
        scratch_types=[
            pltpu.VMEM_SHARED((N, ncols), jnp.float32),
            pltpu.VMEM((grp, kch), jnp.int32),
            pltpu.VMEM((grp, kch), jnp.int32),
            pltpu.VMEM((kch, ncols), jnp.float32),
            pltpu.VMEM((kch, ncols), jnp.float32),
            pltpu.SemaphoreType.DMA,
            pltpu.SemaphoreType.DMA,
            pltpu.SemaphoreType.DMA,
            pltpu.SemaphoreType.DMA,
        ],
        compiler_params=pltpu.CompilerParams(use_tc_tiling_on_sc=False),
    )
    def k(tab_a, tab_b, src2_h, dst2_h, zeros_h, out_h,
          acc, sidxs, didxs, rows0, rows1, g0, g1, s0, s1):
        c = lax.axis_index("c")
        s = lax.axis_index("s")

        @pl.when(s < NTILES - 1)
        def _():
            pltpu.sync_copy(zeros_h.at[pl.ds(0, ZR)],
                            acc.at[pl.ds(s * ZR, ZR)])

        @pl.when(s == NTILES - 1)
        def _():
            pltpu.sync_copy(zeros_h, acc.at[pl.ds((NTILES - 1) * ZR, ZR_LAST)])

        plsc.subcore_barrier()

        def run(tab, cb):
            def group(gi, carry):
                gb = cb + gi * grp
                pltpu.sync_copy(src2_h.at[pl.ds(gb, grp)], sidxs)
                pltpu.sync_copy(dst2_h.at[pl.ds(gb, grp)], didxs)
                pltpu.async_copy(tab.at[sidxs.at[0]], rows0, g0)
                pltpu.async_copy(tab.at[sidxs.at[1]], rows1, g1)

                def pair(j, carry2):
                    i0 = 2 * j
                    pltpu.make_async_copy(tab.at[sidxs.at[i0]], rows0,
                                          g0).wait()
                    pltpu.async_copy(rows0, acc.at[didxs.at[i0]], s0,
                                     add=True)
                    pltpu.make_async_copy(tab.at[sidxs.at[i0 + 1]], rows1,
                                          g1).wait()
                    pltpu.async_copy(rows1, acc.at[didxs.at[i0 + 1]], s1,
                                     add=True)

                    @pl.when(j < gpairs - 1)
                    def _():
                        pltpu.make_async_copy(rows0, acc.at[didxs.at[i0]],
                                              s0).wait()
                        pltpu.async_copy(tab.at[sidxs.at[i0 + 2]], rows0, g0)
                        pltpu.make_async_copy(rows1,
                                              acc.at[didxs.at[i0 + 1]],
                                              s1).wait()
                        pltpu.async_copy(tab.at[sidxs.at[i0 + 3]], rows1, g1)

                    return carry2

                lax.fori_loop(0, gpairs, pair, 0)
                pltpu.make_async_copy(rows0, acc.at[didxs.at[2 * gpairs - 2]],
                                      s0).wait()
                pltpu.make_async_copy(rows1, acc.at[didxs.at[2 * gpairs - 1]],
                                      s1).wait()
                if tail:
                    pltpu.async_copy(tab.at[sidxs.at[grp - 1]], rows0,
                                     g0).wait()
                    pltpu.async_copy(rows0, acc.at[didxs.at[grp - 1]], s0,
                                     add=True).wait()
                return carry

            lax.fori_loop(0, ngrp, group, 0)

        if edge_split:
            run(tab_a, (c * NTILES + s) * nch)
        else:
            @pl.when(c == 0)
            def _():
                run(tab_a, s * nch)

            @pl.when(c == 1)
            def _():
                run(tab_b, s * nch)

        plsc.subcore_barrier()

        @pl.when(s < NTILES - 1)
        def _():
            pltpu.sync_copy(acc.at[pl.ds(s * ZR, ZR)],
                            out_h.at[c, pl.ds(s * ZR, ZR)])

        @pl.when(s == NTILES - 1)
        def _():
            pltpu.sync_copy(acc.at[pl.ds((NTILES - 1) * ZR, ZR_LAST)],
                            out_h.at[c, pl.ds((NTILES - 1) * ZR, ZR_LAST)])

    return k


_edge_agg_aug = _make_edge_agg(D_IN + AUG, True, 40, 50)
_edge_agg_feat = _make_edge_agg(D_IN, False, 80, 50)
_edge_agg_128 = _make_edge_agg(D_IN, True, 80, 25)


def _agg_edge_split_aug(table, src, dst):
    zeros = jnp.zeros((ZR_LAST, D_IN + AUG), jnp.float32)
    return _edge_agg_aug(table, table,
                         src.reshape(-1, 40), dst.reshape(-1, 40), zeros)


def _agg_edge_split_128(table, src, dst):
    zeros = jnp.zeros((ZR_LAST, D_IN), jnp.float32)
    return _edge_agg_128(table, table,
                         src.reshape(-1, 80), dst.reshape(-1, 80), zeros)


def _agg_feat_split(ta, tb, src, dst):
    zeros = jnp.zeros((ZR_LAST, D_IN), jnp.float32)
    return _edge_agg_feat(ta, tb,
                          src.reshape(-1, 80), dst.reshape(-1, 80), zeros)


def kernel(x, edge_index, distance_matrix, nodes_to_community, params):
    src = edge_index[0]
    dst = edge_index[1]
    comm2d = nodes_to_community.reshape(GRID_N, 1, BN)

    g, praw, cnt_row = _fcin(x, comm2d, params['fc_in'])
    k0, v0 = _cprep(praw, params['convs'][0])
    g, praw1 = _attn_mid(g, distance_matrix, cnt_row, k0, v0,
                         params['convs'][0], params['ffs'][0], comm2d)
    k1, v1 = _cprep(praw1, params['convs'][1])
    xglobal = _attn_last(g, distance_matrix, cnt_row, k1, v1,
                         params['convs'][1], params['ffs'][1],
                         params['fc_out'])

    x_aug = jnp.concatenate([x, jnp.ones((N, AUG), jnp.float32)], axis=1)
    agg1p = _agg_edge_split_aug(x_aug, src, dst)
    h1a, h1b, deginv = _sage1(agg1p, x, params['gnn'][0])
    h1a, h1b, g = lax.optimization_barrier((h1a, h1b, g))
    agg2 = _agg_feat_split(h1a, h1b, src, dst)
    h2, z = _sage2(agg2, h1a, h1b, deginv, params['gnn'][1], params['gnn'][2])
    z, xglobal = lax.optimization_barrier((z, xglobal))
    agg3p = _agg_edge_split_128(z, src, dst)
    return _sage3(agg3p, h2, deginv, xglobal, params['gnn'][2])


# speedup vs baseline: 8.0741x; 1.0307x over previous
import functools

import jax
import jax.numpy as jnp
from jax import lax
from jax.experimental import pallas as pl
from jax.experimental.pallas import tpu as pltpu
from jax.experimental.pallas import tpu_sc as plsc

N = 10000
E = 320000
D_IN = 128
HID = 256
OUT = 128
C = 512
HEADS = 4
DHEAD = HID // HEADS
SCALE = 1.0 / (DHEAD ** 0.5)
BN = 1000
GRID_N = N // BN
AUG = 16
DAUG = 8


def _const2(shape):
    return pl.BlockSpec(shape, lambda i: (0, 0))


def _fcin_body(x_r, w1_r, b1_r, w2_r, b2_r, comm_r,
               g_r, praw_r, cnt_r):
    i = pl.program_id(0)
    g = jnp.maximum(jnp.dot(x_r[...], w1_r[...],
                            preferred_element_type=jnp.float32) + b1_r[...], 0.0)
    g = jnp.dot(g, w2_r[...], preferred_element_type=jnp.float32) + b2_r[...]
    g_r[...] = g
    comm = comm_r[0, 0, :]
    m2 = (comm[:, None] == lax.broadcasted_iota(jnp.int32, (BN, C), 1)
          ).astype(jnp.float32)
    g_aug = jnp.concatenate([g, jnp.ones((BN, AUG), jnp.float32)], axis=1)
    pb = lax.dot_general(m2, g_aug, (((0,), (0,)), ((), ())),
                         preferred_element_type=jnp.float32)
    cb = jnp.sum(m2, axis=0, keepdims=True)

    @pl.when(i == 0)
    def _():
        praw_r[...] = pb
        cnt_r[...] = cb

    @pl.when(i > 0)
    def _():
        praw_r[...] += pb
        cnt_r[...] += cb


def _fcin(x, comm2d, fp):
    return pl.pallas_call(
        _fcin_body,
        grid=(GRID_N,),
        in_specs=[
            pl.BlockSpec((BN, D_IN), lambda i: (i, 0)),
            _const2((D_IN, HID)), _const2((1, HID)),
            _const2((HID, HID)), _const2((1, HID)),
            pl.BlockSpec((1, 1, BN), lambda i: (i, 0, 0)),
        ],
        out_specs=[
            pl.BlockSpec((BN, HID), lambda i: (i, 0)),
            pl.BlockSpec((C, HID + AUG), lambda i: (0, 0)),
            pl.BlockSpec((1, C), lambda i: (0, 0)),
        ],
        out_shape=[
            jax.ShapeDtypeStruct((N, HID), jnp.float32),
            jax.ShapeDtypeStruct((C, HID + AUG), jnp.float32),
            jax.ShapeDtypeStruct((1, C), jnp.float32),
        ],
    )(x, fp['W1'], fp['b1'].reshape(1, HID), fp['W2'], fp['b2'].reshape(1, HID),
      comm2d)


def _cprep_body(pa_r, wk_r, bk_r, wv_r, bv_r, k_r, v_r):
    praw = pa_r[:, :HID]
    cnt = pa_r[:, HID:HID + 1]
    inv = 1.0 / jnp.maximum(cnt, 1.0)
    cavg = praw * inv
    k_r[...] = jnp.dot(cavg, wk_r[...],
                       preferred_element_type=jnp.float32) + bk_r[...]
    v_r[...] = jnp.dot(cavg, wv_r[...],
                       preferred_element_type=jnp.float32) + bv_r[...]


def _cprep(praw_aug, cp):
    return pl.pallas_call(
        _cprep_body,
        grid=(1,),
        in_specs=[
            _const2((C, HID + AUG)),
            _const2((HID, HID)), _const2((1, HID)),
            _const2((HID, HID)), _const2((1, HID)),
        ],
        out_specs=[
            pl.BlockSpec((C, HID), lambda i: (0, 0)),
            pl.BlockSpec((C, HID), lambda i: (0, 0)),
        ],
        out_shape=[
            jax.ShapeDtypeStruct((C, HID), jnp.float32),
            jax.ShapeDtypeStruct((C, HID), jnp.float32),
        ],
    )(praw_aug, cp['Wk'], cp['bk'].reshape(1, HID),
      cp['Wv'], cp['bv'].reshape(1, HID))


def _attn_core(g, dmat, cnt_r, k_r, v_r, wp_r, bp_r, wq_r, bq_r,
               w1_r, b1_r, w2_r, b2_r, wdis, bdis):
    q = jnp.dot(g, wp_r[...], preferred_element_type=jnp.float32) + bp_r[...]
    q = jnp.dot(q, wq_r[...], preferred_element_type=jnp.float32) + bq_r[...]
    base = dmat * wdis + bdis + jnp.log(cnt_r[...])
    outs = []
    for h in range(HEADS):
        sl = slice(h * DHEAD, (h + 1) * DHEAD)
        qh = q[:, sl]
        kh = k_r[:, sl]
        vh = v_r[:, sl]
        dots = lax.dot_general(qh, kh, (((1,), (1,)), ((), ())),
                               preferred_element_type=jnp.float32) * SCALE + base
        mx = jnp.max(dots, axis=-1, keepdims=True)
        e = jnp.exp(dots - mx)
        s = jnp.sum(e, axis=-1, keepdims=True)
        outs.append(jnp.dot(e, vh, preferred_element_type=jnp.float32) / s)
    o = jnp.concatenate(outs, axis=1)
    o = jnp.maximum(jnp.dot(o, w1_r[...],
                            preferred_element_type=jnp.float32) + b1_r[...], 0.0)
    return jnp.maximum(jnp.dot(o, w2_r[...],
                               preferred_element_type=jnp.float32) + b2_r[...], 0.0)


def _attn_mid_body(g_r, dm_r, cnt_r, k_r, v_r, wp_r, bp_r, wq_r, bq_r,
                   w1_r, b1_r, w2_r, b2_r, sc_r, comm_r,
                   gn_r, praw_r):
    i = pl.program_id(0)
    gn = _attn_core(g_r[...], dm_r[...], cnt_r, k_r, v_r, wp_r, bp_r,
                    wq_r, bq_r, w1_r, b1_r, w2_r, b2_r,
                    sc_r[0, 0], sc_r[0, 1])
    gn_r[...] = gn
    comm = comm_r[0, 0, :]
    m2 = (comm[:, None] == lax.broadcasted_iota(jnp.int32, (BN, C), 1)
          ).astype(jnp.float32)
    g_aug = jnp.concatenate([gn, jnp.ones((BN, AUG), jnp.float32)], axis=1)
    pb = lax.dot_general(m2, g_aug, (((0,), (0,)), ((), ())),
                         preferred_element_type=jnp.float32)

    @pl.when(i == 0)
    def _():
        praw_r[...] = pb

    @pl.when(i > 0)
    def _():
        praw_r[...] += pb


def _attn_mid(g, dmat, cnt_row, k, v, cp, fp, comm2d):
    sc = jnp.stack([cp['w_dis'], cp['b_dis']]).reshape(1, 2)
    return pl.pallas_call(
        _attn_mid_body,
        grid=(GRID_N,),
        in_specs=[
            pl.BlockSpec((BN, HID), lambda i: (i, 0)),
            pl.BlockSpec((BN, C), lambda i: (i, 0)),
            _const2((1, C)),
            _const2((C, HID)), _const2((C, HID)),
            _const2((HID, HID)), _const2((1, HID)),
            _const2((HID, HID)), _const2((1, HID)),
            _const2((HID, HID)), _const2((1, HID)),
            _const2((HID, HID)), _const2((1, HID)),
            _const2((1, 2)),
            pl.BlockSpec((1, 1, BN), lambda i: (i, 0, 0)),
        ],
        out_specs=[
            pl.BlockSpec((BN, HID), lambda i: (i, 0)),
            pl.BlockSpec((C, HID + AUG), lambda i: (0, 0)),
        ],
        out_shape=[
            jax.ShapeDtypeStruct((N, HID), jnp.float32),
            jax.ShapeDtypeStruct((C, HID + AUG), jnp.float32),
        ],
    )(g, dmat, cnt_row, k, v,
      cp['Wp'], cp['bp'].reshape(1, HID), cp['Wq'], cp['bq'].reshape(1, HID),
      fp['W1'], fp['b1'].reshape(1, HID), fp['W2'], fp['b2'].reshape(1, HID),
      sc, comm2d)


def _attn_last_body(g_r, dm_r, cnt_r, k_r, v_r, wp_r, bp_r, wq_r, bq_r,
                    w1_r, b1_r, w2_r, b2_r, sc_r, wo_r, bo_r,
                    out_r):
    gn = _attn_core(g_r[...], dm_r[...], cnt_r, k_r, v_r, wp_r, bp_r,
                    wq_r, bq_r, w1_r, b1_r, w2_r, b2_r,
                    sc_r[0, 0], sc_r[0, 1])
    out_r[...] = (jnp.dot(gn, wo_r[...], preferred_element_type=jnp.float32)
                  + bo_r[...])


def _attn_last(g, dmat, cnt_row, k, v, cp, fp, op):
    sc = jnp.stack([cp['w_dis'], cp['b_dis']]).reshape(1, 2)
    return pl.pallas_call(
        _attn_last_body,
        grid=(GRID_N,),
        in_specs=[
            pl.BlockSpec((BN, HID), lambda i: (i, 0)),
            pl.BlockSpec((BN, C), lambda i: (i, 0)),
            _const2((1, C)),
            _const2((C, HID)), _const2((C, HID)),
            _const2((HID, HID)), _const2((1, HID)),
            _const2((HID, HID)), _const2((1, HID)),
            _const2((HID, HID)), _const2((1, HID)),
            _const2((HID, HID)), _const2((1, HID)),
            _const2((1, 2)),
            _const2((HID, OUT)), _const2((1, OUT)),
        ],
        out_specs=pl.BlockSpec((BN, OUT), lambda i: (i, 0)),
        out_shape=jax.ShapeDtypeStruct((N, OUT), jnp.float32),
    )(g, dmat, cnt_row, k, v,
      cp['Wp'], cp['bp'].reshape(1, HID), cp['Wq'], cp['bq'].reshape(1, HID),
      fp['W1'], fp['b1'].reshape(1, HID), fp['W2'], fp['b2'].reshape(1, HID),
      sc, op['W'], op['b'].reshape(1, OUT))


def _sage1_body(ap_r, x_r, wl_r, bl_r, wr_r, h1a_r, h1b_r, di_r):
    a = ap_r[0] + ap_r[1]
    sums = a[:, :D_IN]
    deg = a[:, D_IN:D_IN + 1]
    deginv = 1.0 / jnp.maximum(deg, 1.0)
    mean = sums * deginv
    h = jnp.dot(mean, wl_r[...], preferred_element_type=jnp.float32) + bl_r[...]
    h = h + jnp.dot(x_r[...], wr_r[...], preferred_element_type=jnp.float32)
    h = jnp.maximum(h, 0.0)
    h1a_r[...] = h[:, :D_IN]
    h1b_r[...] = h[:, D_IN:]
    di_r[...] = jnp.broadcast_to(deginv, (BN, D_IN))


def _sage1(agg1p, x, gp):
    return pl.pallas_call(
        _sage1_body,
        grid=(GRID_N,),
        in_specs=[
            pl.BlockSpec((2, BN, D_IN + DAUG), lambda i: (0, i, 0)),
            pl.BlockSpec((BN, D_IN), lambda i: (i, 0)),
            _const2((D_IN, HID)), _const2((1, HID)),
            _const2((D_IN, HID)),
        ],
        out_specs=[
            pl.BlockSpec((BN, D_IN), lambda i: (i, 0)),
            pl.BlockSpec((BN, D_IN), lambda i: (i, 0)),
            pl.BlockSpec((BN, D_IN), lambda i: (i, 0)),
        ],
        out_shape=[
            jax.ShapeDtypeStruct((N, D_IN), jnp.float32),
            jax.ShapeDtypeStruct((N, D_IN), jnp.float32),
            jax.ShapeDtypeStruct((N, D_IN), jnp.float32),
        ],
    )(agg1p, x, gp['Wl'], gp['bl'].reshape(1, HID), gp['Wr'])


def _sage2_body(ag_r, h1a_r, h1b_r, di_r, wl_r, bl_r, wr_r, wl3_r,
                h2_r, z_r):
    di = di_r[...]
    mean = jnp.concatenate([ag_r[0] * di, ag_r[1] * di], axis=1)
    h1 = jnp.concatenate([h1a_r[...], h1b_r[...]], axis=1)
    h = jnp.dot(mean, wl_r[...], preferred_element_type=jnp.float32) + bl_r[...]
    h = h + jnp.dot(h1, wr_r[...], preferred_element_type=jnp.float32)
    h = jnp.maximum(h, 0.0)
    h2_r[...] = h
    z_r[...] = jnp.dot(h, wl3_r[...], preferred_element_type=jnp.float32)


def _sage2(agg2, h1a, h1b, deginv, gp, gp3):
    return pl.pallas_call(
        _sage2_body,
        grid=(GRID_N,),
        in_specs=[
            pl.BlockSpec((2, BN, D_IN), lambda i: (0, i, 0)),
            pl.BlockSpec((BN, D_IN), lambda i: (i, 0)),
            pl.BlockSpec((BN, D_IN), lambda i: (i, 0)),
            pl.BlockSpec((BN, D_IN), lambda i: (i, 0)),
            _const2((HID, HID)), _const2((1, HID)),
            _const2((HID, HID)),
            _const2((HID, OUT)),
        ],
        out_specs=[
            pl.BlockSpec((BN, HID), lambda i: (i, 0)),
            pl.BlockSpec((BN, OUT), lambda i: (i, 0)),
        ],
        out_shape=[
            jax.ShapeDtypeStruct((N, HID), jnp.float32),
            jax.ShapeDtypeStruct((N, OUT), jnp.float32),
        ],
    )(agg2, h1a, h1b, deginv, gp['Wl'], gp['bl'].reshape(1, HID), gp['Wr'],
      gp3['Wl'])


def _sage3_body(ap_r, h2_r, di_r, xg_r, bl_r, wr_r, out_r):
    agg = (ap_r[0] + ap_r[1]) * di_r[...]
    out_r[...] = (agg + bl_r[...] + xg_r[...]
                  + jnp.dot(h2_r[...], wr_r[...],
                            preferred_element_type=jnp.float32))


def _sage3(agg3p, h2, deginv, xglobal, gp3):
    return pl.pallas_call(
        _sage3_body,
        grid=(GRID_N,),
        in_specs=[
            pl.BlockSpec((2, BN, OUT), lambda i: (0, i, 0)),
            pl.BlockSpec((BN, HID), lambda i: (i, 0)),
            pl.BlockSpec((BN, OUT), lambda i: (i, 0)),
            pl.BlockSpec((BN, OUT), lambda i: (i, 0)),
            _const2((1, OUT)),
            _const2((HID, OUT)),
        ],
        out_specs=pl.BlockSpec((BN, OUT), lambda i: (i, 0)),
        out_shape=jax.ShapeDtypeStruct((N, OUT), jnp.float32),
    )(agg3p, h2, deginv, xglobal, gp3['bl'].reshape(1, OUT), gp3['Wr'])


NTILES = 16
ZR = 624
ZR_LAST = N - (NTILES - 1) * ZR


def _make_edge_agg(ncols, edge_split, kch, grp):
    mesh = plsc.VectorSubcoreMesh(core_axis_name="c", subcore_axis_name="s")
    nch = (E // 32 if edge_split else E // NTILES) // kch
    ngrp = nch // grp
    gpairs = grp // 2
    tail = grp % 2

    @functools.partial(
        pl.kernel,
        out_type=jax.ShapeDtypeStruct((2, N, ncols), jnp.float32),
        mesh=mesh,
        scratch_types=[
            pltpu.VMEM_SHARED((N, ncols), jnp.float32),
            pltpu.VMEM((grp, kch), jnp.int32),
            pltpu.VMEM((grp, kch), jnp.int32),
            pltpu.VMEM((kch, ncols), jnp.float32),
            pltpu.VMEM((kch, ncols), jnp.float32),
            pltpu.SemaphoreType.DMA,
            pltpu.SemaphoreType.DMA,
            pltpu.SemaphoreType.DMA,
            pltpu.SemaphoreType.DMA,
        ],
        compiler_params=pltpu.CompilerParams(use_tc_tiling_on_sc=False),
    )
    def k(tab_a, tab_b, src2_h, dst2_h, zeros_h, out_h,
          acc, sidxs, didxs, rows0, rows1, g0, g1, s0, s1):
        c = lax.axis_index("c")
        s = lax.axis_index("s")

        @pl.when(s < NTILES - 1)
        def _():
            pltpu.sync_copy(zeros_h.at[pl.ds(0, ZR)],
                            acc.at[pl.ds(s * ZR, ZR)])

        @pl.when(s == NTILES - 1)
        def _():
            pltpu.sync_copy(zeros_h, acc.at[pl.ds((NTILES - 1) * ZR, ZR_LAST)])

        plsc.subcore_barrier()

        def run(tab, cb):
            def group(gi, carry):
                gb = cb + gi * grp
                pltpu.sync_copy(src2_h.at[pl.ds(gb, grp)], sidxs)
                pltpu.sync_copy(dst2_h.at[pl.ds(gb, grp)], didxs)
                pltpu.async_copy(tab.at[sidxs.at[0]], rows0, g0)
                pltpu.async_copy(tab.at[sidxs.at[1]], rows1, g1)

                def pair(j, carry2):
                    i0 = 2 * j
                    pltpu.make_async_copy(tab.at[sidxs.at[i0]], rows0,
                                          g0).wait()
                    pltpu.async_copy(rows0, acc.at[didxs.at[i0]], s0,
                                     add=True)
                    pltpu.make_async_copy(tab.at[sidxs.at[i0 + 1]], rows1,
                                          g1).wait()
                    pltpu.async_copy(rows1, acc.at[didxs.at[i0 + 1]], s1,
                                     add=True)

                    @pl.when(j < gpairs - 1)
                    def _():
                        pltpu.make_async_copy(rows0, acc.at[didxs.at[i0]],
                                              s0).wait()
                        pltpu.async_copy(tab.at[sidxs.at[i0 + 2]], rows0, g0)
                        pltpu.make_async_copy(rows1,
                                              acc.at[didxs.at[i0 + 1]],
                                              s1).wait()
                        pltpu.async_copy(tab.at[sidxs.at[i0 + 3]], rows1, g1)

                    return carry2

                lax.fori_loop(0, gpairs, pair, 0)
                pltpu.make_async_copy(rows0, acc.at[didxs.at[2 * gpairs - 2]],
                                      s0).wait()
                pltpu.make_async_copy(rows1, acc.at[didxs.at[2 * gpairs - 1]],
                                      s1).wait()
                if tail:
                    pltpu.async_copy(tab.at[sidxs.at[grp - 1]], rows0,
                                     g0).wait()
                    pltpu.async_copy(rows0, acc.at[didxs.at[grp - 1]], s0,
                                     add=True).wait()
                return carry

            lax.fori_loop(0, ngrp, group, 0)

        if edge_split:
            run(tab_a, (c * NTILES + s) * nch)
        else:
            @pl.when(c == 0)
            def _():
                run(tab_a, s * nch)

            @pl.when(c == 1)
            def _():
                run(tab_b, s * nch)

        plsc.subcore_barrier()

        @pl.when(s < NTILES - 1)
        def _():
            pltpu.sync_copy(acc.at[pl.ds(s * ZR, ZR)],
                            out_h.at[c, pl.ds(s * ZR, ZR)])

        @pl.when(s == NTILES - 1)
        def _():
            pltpu.sync_copy(acc.at[pl.ds((NTILES - 1) * ZR, ZR_LAST)],
                            out_h.at[c, pl.ds((NTILES - 1) * ZR, ZR_LAST)])

    return k


_edge_agg_aug = _make_edge_agg(D_IN + DAUG, True, 80, 25)
_edge_agg_feat = _make_edge_agg(D_IN, False, 80, 50)
_edge_agg_128 = _make_edge_agg(D_IN, True, 80, 25)


def _agg_edge_split_aug(table, src, dst):
    zeros = jnp.zeros((ZR_LAST, D_IN + DAUG), jnp.float32)
    return _edge_agg_aug(table, table,
                         src.reshape(-1, 80), dst.reshape(-1, 80), zeros)


def _agg_edge_split_128(table, src, dst):
    zeros = jnp.zeros((ZR_LAST, D_IN), jnp.float32)
    return _edge_agg_128(table, table,
                         src.reshape(-1, 80), dst.reshape(-1, 80), zeros)


def _agg_feat_split(ta, tb, src, dst):
    zeros = jnp.zeros((ZR_LAST, D_IN), jnp.float32)
    return _edge_agg_feat(ta, tb,
                          src.reshape(-1, 80), dst.reshape(-1, 80), zeros)


def kernel(x, edge_index, distance_matrix, nodes_to_community, params):
    src = edge_index[0]
    dst = edge_index[1]
    comm2d = nodes_to_community.reshape(GRID_N, 1, BN)

    g, praw, cnt_row = _fcin(x, comm2d, params['fc_in'])
    k0, v0 = _cprep(praw, params['convs'][0])
    g, praw1 = _attn_mid(g, distance_matrix, cnt_row, k0, v0,
                         params['convs'][0], params['ffs'][0], comm2d)
    k1, v1 = _cprep(praw1, params['convs'][1])
    xglobal = _attn_last(g, distance_matrix, cnt_row, k1, v1,
                         params['convs'][1], params['ffs'][1],
                         params['fc_out'])

    x_aug = jnp.concatenate([x, jnp.ones((N, DAUG), jnp.float32)], axis=1)
    agg1p = _agg_edge_split_aug(x_aug, src, dst)
    h1a, h1b, deginv = _sage1(agg1p, x, params['gnn'][0])
    h1a, h1b, g = lax.optimization_barrier((h1a, h1b, g))
    agg2 = _agg_feat_split(h1a, h1b, src, dst)
    h2, z = _sage2(agg2, h1a, h1b, deginv, params['gnn'][1], params['gnn'][2])
    z, xglobal = lax.optimization_barrier((z, xglobal))
    agg3p = _agg_edge_split_128(z, src, dst)
    return _sage3(agg3p, h2, deginv, xglobal, params['gnn'][2])


# speedup vs baseline: 8.5028x; 1.0531x over previous
import functools

import jax
import jax.numpy as jnp
from jax import lax
from jax.experimental import pallas as pl
from jax.experimental.pallas import tpu as pltpu
from jax.experimental.pallas import tpu_sc as plsc

N = 10000
E = 320000
D_IN = 128
HID = 256
OUT = 128
C = 512
HEADS = 4
DHEAD = HID // HEADS
SCALE = 1.0 / (DHEAD ** 0.5)
BN = 1000
GRID_N = N // BN
AUG = 16
DAUG = 8


def _const2(shape):
    return pl.BlockSpec(shape, lambda i: (0, 0))


def _fcin_body(x_r, w1_r, b1_r, w2_r, b2_r, comm_r,
               g_r, praw_r, cnt_r):
    i = pl.program_id(0)
    g = jnp.maximum(jnp.dot(x_r[...], w1_r[...],
                            preferred_element_type=jnp.float32) + b1_r[...], 0.0)
    g = jnp.dot(g, w2_r[...], preferred_element_type=jnp.float32) + b2_r[...]
    g_r[...] = g
    comm = comm_r[0, 0, :]
    m2 = (comm[:, None] == lax.broadcasted_iota(jnp.int32, (BN, C), 1)
          ).astype(jnp.float32)
    g_aug = jnp.concatenate([g, jnp.ones((BN, AUG), jnp.float32)], axis=1)
    pb = lax.dot_general(m2, g_aug, (((0,), (0,)), ((), ())),
                         preferred_element_type=jnp.float32)
    cb = jnp.sum(m2, axis=0, keepdims=True)

    @pl.when(i == 0)
    def _():
        praw_r[...] = pb
        cnt_r[...] = cb

    @pl.when(i > 0)
    def _():
        praw_r[...] += pb
        cnt_r[...] += cb


def _fcin(x, comm2d, fp):
    return pl.pallas_call(
        _fcin_body,
        grid=(GRID_N,),
        in_specs=[
            pl.BlockSpec((BN, D_IN), lambda i: (i, 0)),
            _const2((D_IN, HID)), _const2((1, HID)),
            _const2((HID, HID)), _const2((1, HID)),
            pl.BlockSpec((1, 1, BN), lambda i: (i, 0, 0)),
        ],
        out_specs=[
            pl.BlockSpec((BN, HID), lambda i: (i, 0)),
            pl.BlockSpec((C, HID + AUG), lambda i: (0, 0)),
            pl.BlockSpec((1, C), lambda i: (0, 0)),
        ],
        out_shape=[
            jax.ShapeDtypeStruct((N, HID), jnp.float32),
            jax.ShapeDtypeStruct((C, HID + AUG), jnp.float32),
            jax.ShapeDtypeStruct((1, C), jnp.float32),
        ],
    )(x, fp['W1'], fp['b1'].reshape(1, HID), fp['W2'], fp['b2'].reshape(1, HID),
      comm2d)


def _cprep_body(pa_r, wk_r, bk_r, wv_r, bv_r, k_r, v_r):
    praw = pa_r[:, :HID]
    cnt = pa_r[:, HID:HID + 1]
    inv = 1.0 / jnp.maximum(cnt, 1.0)
    cavg = praw * inv
    k_r[...] = jnp.dot(cavg, wk_r[...],
                       preferred_element_type=jnp.float32) + bk_r[...]
    v_r[...] = jnp.dot(cavg, wv_r[...],
                       preferred_element_type=jnp.float32) + bv_r[...]


def _cprep(praw_aug, cp):
    return pl.pallas_call(
        _cprep_body,
        grid=(1,),
        in_specs=[
            _const2((C, HID + AUG)),
            _const2((HID, HID)), _const2((1, HID)),
            _const2((HID, HID)), _const2((1, HID)),
        ],
        out_specs=[
            pl.BlockSpec((C, HID), lambda i: (0, 0)),
            pl.BlockSpec((C, HID), lambda i: (0, 0)),
        ],
        out_shape=[
            jax.ShapeDtypeStruct((C, HID), jnp.float32),
            jax.ShapeDtypeStruct((C, HID), jnp.float32),
        ],
    )(praw_aug, cp['Wk'], cp['bk'].reshape(1, HID),
      cp['Wv'], cp['bv'].reshape(1, HID))


def _attn_core(g, dmat, cnt_r, k_r, v_r, wp_r, bp_r, wq_r, bq_r,
               w1_r, b1_r, w2_r, b2_r, wdis, bdis):
    q = jnp.dot(g, wp_r[...], preferred_element_type=jnp.float32) + bp_r[...]
    q = jnp.dot(q, wq_r[...], preferred_element_type=jnp.float32) + bq_r[...]
    base = dmat * wdis + bdis + jnp.log(cnt_r[...])
    outs = []
    for h in range(HEADS):
        sl = slice(h * DHEAD, (h + 1) * DHEAD)
        qh = q[:, sl]
        kh = k_r[:, sl]
        vh = v_r[:, sl]
        dots = lax.dot_general(qh, kh, (((1,), (1,)), ((), ())),
                               preferred_element_type=jnp.float32) * SCALE + base
        mx = jnp.max(dots, axis=-1, keepdims=True)
        e = jnp.exp(dots - mx)
        s = jnp.sum(e, axis=-1, keepdims=True)
        outs.append(jnp.dot(e, vh, preferred_element_type=jnp.float32) / s)
    o = jnp.concatenate(outs, axis=1)
    o = jnp.maximum(jnp.dot(o, w1_r[...],
                            preferred_element_type=jnp.float32) + b1_r[...], 0.0)
    return jnp.maximum(jnp.dot(o, w2_r[...],
                               preferred_element_type=jnp.float32) + b2_r[...], 0.0)


def _attn_mid_body(g_r, dm_r, cnt_r, k_r, v_r, wp_r, bp_r, wq_r, bq_r,
                   w1_r, b1_r, w2_r, b2_r, sc_r, comm_r,
                   gn_r, praw_r):
    i = pl.program_id(0)
    gn = _attn_core(g_r[...], dm_r[...], cnt_r, k_r, v_r, wp_r, bp_r,
                    wq_r, bq_r, w1_r, b1_r, w2_r, b2_r,
                    sc_r[0, 0], sc_r[0, 1])
    gn_r[...] = gn
    comm = comm_r[0, 0, :]
    m2 = (comm[:, None] == lax.broadcasted_iota(jnp.int32, (BN, C), 1)
          ).astype(jnp.float32)
    g_aug = jnp.concatenate([gn, jnp.ones((BN, AUG), jnp.float32)], axis=1)
    pb = lax.dot_general(m2, g_aug, (((0,), (0,)), ((), ())),
                         preferred_element_type=jnp.float32)

    @pl.when(i == 0)
    def _():
        praw_r[...] = pb

    @pl.when(i > 0)
    def _():
        praw_r[...] += pb


def _attn_mid(g, dmat, cnt_row, k, v, cp, fp, comm2d):
    sc = jnp.stack([cp['w_dis'], cp['b_dis']]).reshape(1, 2)
    return pl.pallas_call(
        _attn_mid_body,
        grid=(GRID_N,),
        in_specs=[
            pl.BlockSpec((BN, HID), lambda i: (i, 0)),
            pl.BlockSpec((BN, C), lambda i: (i, 0)),
            _const2((1, C)),
            _const2((C, HID)), _const2((C, HID)),
            _const2((HID, HID)), _const2((1, HID)),
            _const2((HID, HID)), _const2((1, HID)),
            _const2((HID, HID)), _const2((1, HID)),
            _const2((HID, HID)), _const2((1, HID)),
            _const2((1, 2)),
            pl.BlockSpec((1, 1, BN), lambda i: (i, 0, 0)),
        ],
        out_specs=[
            pl.BlockSpec((BN, HID), lambda i: (i, 0)),
            pl.BlockSpec((C, HID + AUG), lambda i: (0, 0)),
        ],
        out_shape=[
            jax.ShapeDtypeStruct((N, HID), jnp.float32),
            jax.ShapeDtypeStruct((C, HID + AUG), jnp.float32),
        ],
    )(g, dmat, cnt_row, k, v,
      cp['Wp'], cp['bp'].reshape(1, HID), cp['Wq'], cp['bq'].reshape(1, HID),
      fp['W1'], fp['b1'].reshape(1, HID), fp['W2'], fp['b2'].reshape(1, HID),
      sc, comm2d)


def _attn_last_body(g_r, dm_r, cnt_r, k_r, v_r, wp_r, bp_r, wq_r, bq_r,
                    w1_r, b1_r, w2_r, b2_r, sc_r, wo_r, bo_r,
                    out_r):
    gn = _attn_core(g_r[...], dm_r[...], cnt_r, k_r, v_r, wp_r, bp_r,
                    wq_r, bq_r, w1_r, b1_r, w2_r, b2_r,
                    sc_r[0, 0], sc_r[0, 1])
    out_r[...] = (jnp.dot(gn, wo_r[...], preferred_element_type=jnp.float32)
                  + bo_r[...])


def _attn_last(g, dmat, cnt_row, k, v, cp, fp, op):
    sc = jnp.stack([cp['w_dis'], cp['b_dis']]).reshape(1, 2)
    return pl.pallas_call(
        _attn_last_body,
        grid=(GRID_N,),
        in_specs=[
            pl.BlockSpec((BN, HID), lambda i: (i, 0)),
            pl.BlockSpec((BN, C), lambda i: (i, 0)),
            _const2((1, C)),
            _const2((C, HID)), _const2((C, HID)),
            _const2((HID, HID)), _const2((1, HID)),
            _const2((HID, HID)), _const2((1, HID)),
            _const2((HID, HID)), _const2((1, HID)),
            _const2((HID, HID)), _const2((1, HID)),
            _const2((1, 2)),
            _const2((HID, OUT)), _const2((1, OUT)),
        ],
        out_specs=pl.BlockSpec((BN, OUT), lambda i: (i, 0)),
        out_shape=jax.ShapeDtypeStruct((N, OUT), jnp.float32),
    )(g, dmat, cnt_row, k, v,
      cp['Wp'], cp['bp'].reshape(1, HID), cp['Wq'], cp['bq'].reshape(1, HID),
      fp['W1'], fp['b1'].reshape(1, HID), fp['W2'], fp['b2'].reshape(1, HID),
      sc, op['W'], op['b'].reshape(1, OUT))


def _sage1_body(ap_r, x_r, wl_r, bl_r, wr_r, h1a_r, h1b_r, di_r):
    a = ap_r[0] + ap_r[1]
    sums = a[:, :D_IN]
    deg = a[:, D_IN:D_IN + 1]
    deginv = 1.0 / jnp.maximum(deg, 1.0)
    mean = sums * deginv
    h = jnp.dot(mean, wl_r[...], preferred_element_type=jnp.float32) + bl_r[...]
    h = h + jnp.dot(x_r[...], wr_r[...], preferred_element_type=jnp.float32)
    h = jnp.maximum(h, 0.0)
    h1a_r[...] = h[:, :D_IN].astype(jnp.bfloat16)
    h1b_r[...] = h[:, D_IN:].astype(jnp.bfloat16)
    di_r[...] = jnp.broadcast_to(deginv, (BN, D_IN))


def _sage1(agg1p, x, gp):
    return pl.pallas_call(
        _sage1_body,
        grid=(GRID_N,),
        in_specs=[
            pl.BlockSpec((2, BN, D_IN + DAUG), lambda i: (0, i, 0)),
            pl.BlockSpec((BN, D_IN), lambda i: (i, 0)),
            _const2((D_IN, HID)), _const2((1, HID)),
            _const2((D_IN, HID)),
        ],
        out_specs=[
            pl.BlockSpec((BN, D_IN), lambda i: (i, 0)),
            pl.BlockSpec((BN, D_IN), lambda i: (i, 0)),
            pl.BlockSpec((BN, D_IN), lambda i: (i, 0)),
        ],
        out_shape=[
            jax.ShapeDtypeStruct((N, D_IN), jnp.bfloat16),
            jax.ShapeDtypeStruct((N, D_IN), jnp.bfloat16),
            jax.ShapeDtypeStruct((N, D_IN), jnp.float32),
        ],
    )(agg1p, x, gp['Wl'], gp['bl'].reshape(1, HID), gp['Wr'])


def _sage2_body(ag_r, h1a_r, h1b_r, di_r, wl_r, bl_r, wr_r, wl3_r,
                h2_r, z_r):
    di = di_r[...]
    suma = (ag_r[0, 0].astype(jnp.float32) + ag_r[0, 1].astype(jnp.float32))
    sumb = (ag_r[1, 0].astype(jnp.float32) + ag_r[1, 1].astype(jnp.float32))
    mean = jnp.concatenate([suma * di, sumb * di], axis=1)
    h1 = jnp.concatenate([h1a_r[...], h1b_r[...]],
                         axis=1).astype(jnp.float32)
    h = jnp.dot(mean, wl_r[...], preferred_element_type=jnp.float32) + bl_r[...]
    h = h + jnp.dot(h1, wr_r[...], preferred_element_type=jnp.float32)
    h = jnp.maximum(h, 0.0)
    h2_r[...] = h
    z_r[...] = jnp.dot(h, wl3_r[...], preferred_element_type=jnp.float32)


def _sage2(agg2, h1a, h1b, deginv, gp, gp3):
    return pl.pallas_call(
        _sage2_body,
        grid=(GRID_N,),
        in_specs=[
            pl.BlockSpec((2, 2, BN, D_IN), lambda i: (0, 0, i, 0)),
            pl.BlockSpec((BN, D_IN), lambda i: (i, 0)),
            pl.BlockSpec((BN, D_IN), lambda i: (i, 0)),
            pl.BlockSpec((BN, D_IN), lambda i: (i, 0)),
            _const2((HID, HID)), _const2((1, HID)),
            _const2((HID, HID)),
            _const2((HID, OUT)),
        ],
        out_specs=[
            pl.BlockSpec((BN, HID), lambda i: (i, 0)),
            pl.BlockSpec((BN, OUT), lambda i: (i, 0)),
        ],
        out_shape=[
            jax.ShapeDtypeStruct((N, HID), jnp.float32),
            jax.ShapeDtypeStruct((N, OUT), jnp.float32),
        ],
    )(agg2, h1a, h1b, deginv, gp['Wl'], gp['bl'].reshape(1, HID), gp['Wr'],
      gp3['Wl'])


def _sage3_body(ap_r, h2_r, di_r, xg_r, bl_r, wr_r, out_r):
    agg = (ap_r[0] + ap_r[1]) * di_r[...]
    out_r[...] = (agg + bl_r[...] + xg_r[...]
                  + jnp.dot(h2_r[...], wr_r[...],
                            preferred_element_type=jnp.float32))


def _sage3(agg3p, h2, deginv, xglobal, gp3):
    return pl.pallas_call(
        _sage3_body,
        grid=(GRID_N,),
        in_specs=[
            pl.BlockSpec((2, BN, OUT), lambda i: (0, i, 0)),
            pl.BlockSpec((BN, HID), lambda i: (i, 0)),
            pl.BlockSpec((BN, OUT), lambda i: (i, 0)),
            pl.BlockSpec((BN, OUT), lambda i: (i, 0)),
            _const2((1, OUT)),
            _const2((HID, OUT)),
        ],
        out_specs=pl.BlockSpec((BN, OUT), lambda i: (i, 0)),
        out_shape=jax.ShapeDtypeStruct((N, OUT), jnp.float32),
    )(agg3p, h2, deginv, xglobal, gp3['bl'].reshape(1, OUT), gp3['Wr'])


NTILES = 16
ZR = 624
ZR_LAST = N - (NTILES - 1) * ZR


def _make_edge_agg(ncols, edge_split, kch, grp):
    mesh = plsc.VectorSubcoreMesh(core_axis_name="c", subcore_axis_name="s")
    nch = (E // 32 if edge_split else E // NTILES) // kch
    ngrp = nch // grp
    gpairs = grp // 2
    tail = grp % 2

    @functools.partial(
        pl.kernel,
        out_type=jax.ShapeDtypeStruct((2, N, ncols), jnp.float32),
        mesh=mesh,
        scratch_types=[
            pltpu.VMEM_SHARED((N, ncols), jnp.float32),
            pltpu.VMEM((grp, kch), jnp.int32),
            pltpu.VMEM((grp, kch), jnp.int32),
            pltpu.VMEM((kch, ncols), jnp.float32),
            pltpu.VMEM((kch, ncols), jnp.float32),
            pltpu.SemaphoreType.DMA,
            pltpu.SemaphoreType.DMA,
            pltpu.SemaphoreType.DMA,
            pltpu.SemaphoreType.DMA,
        ],
        compiler_params=pltpu.CompilerParams(use_tc_tiling_on_sc=False),
    )
    def k(tab_a, tab_b, src2_h, dst2_h, zeros_h, out_h,
          acc, sidxs, didxs, rows0, rows1, g0, g1, s0, s1):
        c = lax.axis_index("c")
        s = lax.axis_index("s")

        @pl.when(s < NTILES - 1)
        def _():
            pltpu.sync_copy(zeros_h.at[pl.ds(0, ZR)],
                            acc.at[pl.ds(s * ZR, ZR)])

        @pl.when(s == NTILES - 1)
        def _():
            pltpu.sync_copy(zeros_h, acc.at[pl.ds((NTILES - 1) * ZR, ZR_LAST)])

        plsc.subcore_barrier()

        def run(tab, cb):
            def group(gi, carry):
                gb = cb + gi * grp
                pltpu.sync_copy(src2_h.at[pl.ds(gb, grp)], sidxs)
                pltpu.sync_copy(dst2_h.at[pl.ds(gb, grp)], didxs)
                pltpu.async_copy(tab.at[sidxs.at[0]], rows0, g0)
                pltpu.async_copy(tab.at[sidxs.at[1]], rows1, g1)

                def pair(j, carry2):
                    i0 = 2 * j
                    pltpu.make_async_copy(tab.at[sidxs.at[i0]], rows0,
                                          g0).wait()
                    pltpu.async_copy(rows0, acc.at[didxs.at[i0]], s0,
                                     add=True)
                    pltpu.make_async_copy(tab.at[sidxs.at[i0 + 1]], rows1,
                                          g1).wait()
                    pltpu.async_copy(rows1, acc.at[didxs.at[i0 + 1]], s1,
                                     add=True)

                    @pl.when(j < gpairs - 1)
                    def _():
                        pltpu.make_async_copy(rows0, acc.at[didxs.at[i0]],
                                              s0).wait()
                        pltpu.async_copy(tab.at[sidxs.at[i0 + 2]], rows0, g0)
                        pltpu.make_async_copy(rows1,
                                              acc.at[didxs.at[i0 + 1]],
                                              s1).wait()
                        pltpu.async_copy(tab.at[sidxs.at[i0 + 3]], rows1, g1)

                    return carry2

                lax.fori_loop(0, gpairs, pair, 0)
                pltpu.make_async_copy(rows0, acc.at[didxs.at[2 * gpairs - 2]],
                                      s0).wait()
                pltpu.make_async_copy(rows1, acc.at[didxs.at[2 * gpairs - 1]],
                                      s1).wait()
                if tail:
                    pltpu.async_copy(tab.at[sidxs.at[grp - 1]], rows0,
                                     g0).wait()
                    pltpu.async_copy(rows0, acc.at[didxs.at[grp - 1]], s0,
                                     add=True).wait()
                return carry

            lax.fori_loop(0, ngrp, group, 0)

        if edge_split:
            run(tab_a, (c * NTILES + s) * nch)
        else:
            @pl.when(c == 0)
            def _():
                run(tab_a, s * nch)

            @pl.when(c == 1)
            def _():
                run(tab_b, s * nch)

        plsc.subcore_barrier()

        @pl.when(s < NTILES - 1)
        def _():
            pltpu.sync_copy(acc.at[pl.ds(s * ZR, ZR)],
                            out_h.at[c, pl.ds(s * ZR, ZR)])

        @pl.when(s == NTILES - 1)
        def _():
            pltpu.sync_copy(acc.at[pl.ds((NTILES - 1) * ZR, ZR_LAST)],
                            out_h.at[c, pl.ds((NTILES - 1) * ZR, ZR_LAST)])

    return k


def _make_edge_agg_bf16():
    mesh = plsc.VectorSubcoreMesh(core_axis_name="c", subcore_axis_name="s")
    kch = 80
    nch = (E // NTILES) // kch
    grp = 50
    ngrp = nch // grp
    gpairs = grp // 2

    @functools.partial(
        pl.kernel,
        out_type=jax.ShapeDtypeStruct((2, 2, N, D_IN), jnp.bfloat16),
        mesh=mesh,
        scratch_types=[
            pltpu.VMEM_SHARED((N, D_IN), jnp.bfloat16),
            pltpu.VMEM_SHARED((N, D_IN), jnp.bfloat16),
            pltpu.VMEM((grp, kch), jnp.int32),
            pltpu.VMEM((grp, kch), jnp.int32),
            pltpu.VMEM((kch, D_IN), jnp.bfloat16),
            pltpu.VMEM((kch, D_IN), jnp.bfloat16),
            pltpu.SemaphoreType.DMA,
            pltpu.SemaphoreType.DMA,
            pltpu.SemaphoreType.DMA,
            pltpu.SemaphoreType.DMA,
        ],
        compiler_params=pltpu.CompilerParams(use_tc_tiling_on_sc=False),
    )
    def k(tab_a, tab_b, src2_h, dst2_h, zeros_h, out_h,
          acc0, acc1, sidxs, didxs, rows0, rows1, g0, g1, s0, s1):
        c = lax.axis_index("c")
        s = lax.axis_index("s")

        @pl.when(s < NTILES - 1)
        def _():
            pltpu.sync_copy(zeros_h.at[pl.ds(0, ZR)],
                            acc0.at[pl.ds(s * ZR, ZR)])
            pltpu.sync_copy(zeros_h.at[pl.ds(0, ZR)],
                            acc1.at[pl.ds(s * ZR, ZR)])

        @pl.when(s == NTILES - 1)
        def _():
            pltpu.sync_copy(zeros_h, acc0.at[pl.ds((NTILES - 1) * ZR,
                                                   ZR_LAST)])
            pltpu.sync_copy(zeros_h, acc1.at[pl.ds((NTILES - 1) * ZR,
                                                   ZR_LAST)])

        plsc.subcore_barrier()

        def run(tab, acc):
            cb = s * nch

            def group(gi, carry):
                gb = cb + gi * grp
                pltpu.sync_copy(src2_h.at[pl.ds(gb, grp)], sidxs)
                pltpu.sync_copy(dst2_h.at[pl.ds(gb, grp)], didxs)
                pltpu.async_copy(tab.at[sidxs.at[0]], rows0, g0)
                pltpu.async_copy(tab.at[sidxs.at[1]], rows1, g1)

                def pair(j, carry2):
                    i0 = 2 * j
                    pltpu.make_async_copy(tab.at[sidxs.at[i0]], rows0,
                                          g0).wait()
                    pltpu.async_copy(rows0, acc.at[didxs.at[i0]], s0,
                                     add=True)
                    pltpu.make_async_copy(tab.at[sidxs.at[i0 + 1]], rows1,
                                          g1).wait()
                    pltpu.async_copy(rows1, acc.at[didxs.at[i0 + 1]], s1,
                                     add=True)

                    @pl.when(j < gpairs - 1)
                    def _():
                        pltpu.make_async_copy(rows0, acc.at[didxs.at[i0]],
                                              s0).wait()
                        pltpu.async_copy(tab.at[sidxs.at[i0 + 2]], rows0, g0)
                        pltpu.make_async_copy(rows1,
                                              acc.at[didxs.at[i0 + 1]],
                                              s1).wait()
                        pltpu.async_copy(tab.at[sidxs.at[i0 + 3]], rows1, g1)

                    return carry2

                lax.fori_loop(0, gpairs, pair, 0)
                pltpu.make_async_copy(rows0, acc.at[didxs.at[grp - 2]],
                                      s0).wait()
                pltpu.make_async_copy(rows1, acc.at[didxs.at[grp - 1]],
                                      s1).wait()
                return carry

            lax.fori_loop(0, ngrp, group, 0)

        half = NTILES // 2
        for ci, tab in ((0, tab_a), (1, tab_b)):
            @pl.when(jnp.logical_and(c == ci, s < half))
            def _(tab=tab):
                run(tab, acc0)

            @pl.when(jnp.logical_and(c == ci, s >= half))
            def _(tab=tab):
                run(tab, acc1)

        plsc.subcore_barrier()

        @pl.when(s < NTILES - 1)
        def _():
            pltpu.sync_copy(acc0.at[pl.ds(s * ZR, ZR)],
                            out_h.at[c, 0, pl.ds(s * ZR, ZR)])
            pltpu.sync_copy(acc1.at[pl.ds(s * ZR, ZR)],
                            out_h.at[c, 1, pl.ds(s * ZR, ZR)])

        @pl.when(s == NTILES - 1)
        def _():
            pltpu.sync_copy(acc0.at[pl.ds((NTILES - 1) * ZR, ZR_LAST)],
                            out_h.at[c, 0, pl.ds((NTILES - 1) * ZR, ZR_LAST)])
            pltpu.sync_copy(acc1.at[pl.ds((NTILES - 1) * ZR, ZR_LAST)],
                            out_h.at[c, 1, pl.ds((NTILES - 1) * ZR, ZR_LAST)])

    return k


_edge_agg_aug = _make_edge_agg(D_IN + DAUG, True, 80, 25)
_edge_agg_feat = _make_edge_agg_bf16()
_edge_agg_128 = _make_edge_agg(D_IN, True, 80, 25)


def _agg_edge_split_aug(table, src, dst):
    zeros = jnp.zeros((ZR_LAST, D_IN + DAUG), jnp.float32)
    return _edge_agg_aug(table, table,
                         src.reshape(-1, 80), dst.reshape(-1, 80), zeros)


def _agg_edge_split_128(table, src, dst):
    zeros = jnp.zeros((ZR_LAST, D_IN), jnp.float32)
    return _edge_agg_128(table, table,
                         src.reshape(-1, 80), dst.reshape(-1, 80), zeros)


def _agg_feat_split(ta, tb, src, dst):
    zeros = jnp.zeros((ZR_LAST, D_IN), jnp.bfloat16)
    return _edge_agg_feat(ta, tb,
                          src.reshape(-1, 80), dst.reshape(-1, 80), zeros)


def kernel(x, edge_index, distance_matrix, nodes_to_community, params):
    src = edge_index[0]
    dst = edge_index[1]
    comm2d = nodes_to_community.reshape(GRID_N, 1, BN)

    g, praw, cnt_row = _fcin(x, comm2d, params['fc_in'])
    k0, v0 = _cprep(praw, params['convs'][0])
    g, praw1 = _attn_mid(g, distance_matrix, cnt_row, k0, v0,
                         params['convs'][0], params['ffs'][0], comm2d)
    k1, v1 = _cprep(praw1, params['convs'][1])
    xglobal = _attn_last(g, distance_matrix, cnt_row, k1, v1,
                         params['convs'][1], params['ffs'][1],
                         params['fc_out'])

    x_aug = jnp.concatenate([x, jnp.ones((N, DAUG), jnp.float32)], axis=1)
    agg1p = _agg_edge_split_aug(x_aug, src, dst)
    h1a, h1b, deginv = _sage1(agg1p, x, params['gnn'][0])
    h1a, h1b, g = lax.optimization_barrier((h1a, h1b, g))
    agg2 = _agg_feat_split(h1a, h1b, src, dst)
    h2, z = _sage2(agg2, h1a, h1b, deginv, params['gnn'][1], params['gnn'][2])
    z, xglobal = lax.optimization_barrier((z, xglobal))
    agg3p = _agg_edge_split_128(z, src, dst)
    return _sage3(agg3p, h2, deginv, xglobal, params['gnn'][2])


# speedup vs baseline: 10.2253x; 1.2026x over previous
import functools

import jax
import jax.numpy as jnp
from jax import lax
from jax.experimental import pallas as pl
from jax.experimental.pallas import tpu as pltpu
from jax.experimental.pallas import tpu_sc as plsc

N = 10000
E = 320000
D_IN = 128
HID = 256
OUT = 128
C = 512
HEADS = 4
DHEAD = HID // HEADS
SCALE = 1.0 / (DHEAD ** 0.5)
BN = 2000
GRID_N = N // BN
AUG = 16
DAUG = 32


def _const2(shape):
    return pl.BlockSpec(shape, lambda i: (0, 0))


def _fcin_body(x_r, w1_r, b1_r, w2_r, b2_r, comm_r,
               g_r, praw_r, cnt_r):
    i = pl.program_id(0)
    g = jnp.maximum(jnp.dot(x_r[...], w1_r[...],
                            preferred_element_type=jnp.float32) + b1_r[...], 0.0)
    g = jnp.dot(g, w2_r[...], preferred_element_type=jnp.float32) + b2_r[...]
    g_r[...] = g
    comm = comm_r[0, 0, :]
    m2 = (comm[:, None] == lax.broadcasted_iota(jnp.int32, (BN, C), 1)
          ).astype(jnp.float32)
    g_aug = jnp.concatenate([g, jnp.ones((BN, AUG), jnp.float32)], axis=1)
    pb = lax.dot_general(m2, g_aug, (((0,), (0,)), ((), ())),
                         preferred_element_type=jnp.float32)
    cb = jnp.sum(m2, axis=0, keepdims=True)

    @pl.when(i == 0)
    def _():
        praw_r[...] = pb
        cnt_r[...] = cb

    @pl.when(i > 0)
    def _():
        praw_r[...] += pb
        cnt_r[...] += cb


def _fcin(x, comm2d, fp):
    return pl.pallas_call(
        _fcin_body,
        grid=(GRID_N,),
        in_specs=[
            pl.BlockSpec((BN, D_IN), lambda i: (i, 0)),
            _const2((D_IN, HID)), _const2((1, HID)),
            _const2((HID, HID)), _const2((1, HID)),
            pl.BlockSpec((1, 1, BN), lambda i: (i, 0, 0)),
        ],
        out_specs=[
            pl.BlockSpec((BN, HID), lambda i: (i, 0)),
            pl.BlockSpec((C, HID + AUG), lambda i: (0, 0)),
            pl.BlockSpec((1, C), lambda i: (0, 0)),
        ],
        out_shape=[
            jax.ShapeDtypeStruct((N, HID), jnp.float32),
            jax.ShapeDtypeStruct((C, HID + AUG), jnp.float32),
            jax.ShapeDtypeStruct((1, C), jnp.float32),
        ],
    )(x, fp['W1'], fp['b1'].reshape(1, HID), fp['W2'], fp['b2'].reshape(1, HID),
      comm2d)


def _cprep_body(pa_r, wk_r, bk_r, wv_r, bv_r, k_r, v_r):
    praw = pa_r[:, :HID]
    cnt = pa_r[:, HID:HID + 1]
    inv = 1.0 / jnp.maximum(cnt, 1.0)
    cavg = praw * inv
    k_r[...] = jnp.dot(cavg, wk_r[...],
                       preferred_element_type=jnp.float32) + bk_r[...]
    v_r[...] = jnp.dot(cavg, wv_r[...],
                       preferred_element_type=jnp.float32) + bv_r[...]


def _cprep(praw_aug, cp):
    return pl.pallas_call(
        _cprep_body,
        grid=(1,),
        in_specs=[
            _const2((C, HID + AUG)),
            _const2((HID, HID)), _const2((1, HID)),
            _const2((HID, HID)), _const2((1, HID)),
        ],
        out_specs=[
            pl.BlockSpec((C, HID), lambda i: (0, 0)),
            pl.BlockSpec((C, HID), lambda i: (0, 0)),
        ],
        out_shape=[
            jax.ShapeDtypeStruct((C, HID), jnp.float32),
            jax.ShapeDtypeStruct((C, HID), jnp.float32),
        ],
    )(praw_aug, cp['Wk'], cp['bk'].reshape(1, HID),
      cp['Wv'], cp['bv'].reshape(1, HID))


def _attn_core(g, dmat, cnt_r, k_r, v_r, wp_r, bp_r, wq_r, bq_r,
               w1_r, b1_r, w2_r, b2_r, wdis, bdis):
    q = jnp.dot(g, wp_r[...], preferred_element_type=jnp.float32) + bp_r[...]
    q = jnp.dot(q, wq_r[...], preferred_element_type=jnp.float32) + bq_r[...]
    base = dmat * wdis + bdis + jnp.log(cnt_r[...])
    outs = []
    for h in range(HEADS):
        sl = slice(h * DHEAD, (h + 1) * DHEAD)
        qh = q[:, sl]
        kh = k_r[:, sl]
        vh = v_r[:, sl]
        dots = lax.dot_general(qh, kh, (((1,), (1,)), ((), ())),
                               preferred_element_type=jnp.float32) * SCALE + base
        mx = jnp.max(dots, axis=-1, keepdims=True)
        e = jnp.exp(dots - mx)
        s = jnp.sum(e, axis=-1, keepdims=True)
        outs.append(jnp.dot(e, vh, preferred_element_type=jnp.float32) / s)
    o = jnp.concatenate(outs, axis=1)
    o = jnp.maximum(jnp.dot(o, w1_r[...],
                            preferred_element_type=jnp.float32) + b1_r[...], 0.0)
    return jnp.maximum(jnp.dot(o, w2_r[...],
                               preferred_element_type=jnp.float32) + b2_r[...], 0.0)


def _attn_mid_body(g_r, dm_r, cnt_r, k_r, v_r, wp_r, bp_r, wq_r, bq_r,
                   w1_r, b1_r, w2_r, b2_r, sc_r, comm_r,
                   gn_r, praw_r):
    i = pl.program_id(0)
    gn = _attn_core(g_r[...], dm_r[...], cnt_r, k_r, v_r, wp_r, bp_r,
                    wq_r, bq_r, w1_r, b1_r, w2_r, b2_r,
                    sc_r[0, 0], sc_r[0, 1])
    gn_r[...] = gn
    comm = comm_r[0, 0, :]
    m2 = (comm[:, None] == lax.broadcasted_iota(jnp.int32, (BN, C), 1)
          ).astype(jnp.float32)
    g_aug = jnp.concatenate([gn, jnp.ones((BN, AUG), jnp.float32)], axis=1)
    pb = lax.dot_general(m2, g_aug, (((0,), (0,)), ((), ())),
                         preferred_element_type=jnp.float32)

    @pl.when(i == 0)
    def _():
        praw_r[...] = pb

    @pl.when(i > 0)
    def _():
        praw_r[...] += pb


def _attn_mid(g, dmat, cnt_row, k, v, cp, fp, comm2d):
    sc = jnp.stack([cp['w_dis'], cp['b_dis']]).reshape(1, 2)
    return pl.pallas_call(
        _attn_mid_body,
        grid=(GRID_N,),
        in_specs=[
            pl.BlockSpec((BN, HID), lambda i: (i, 0)),
            pl.BlockSpec((BN, C), lambda i: (i, 0)),
            _const2((1, C)),
            _const2((C, HID)), _const2((C, HID)),
            _const2((HID, HID)), _const2((1, HID)),
            _const2((HID, HID)), _const2((1, HID)),
            _const2((HID, HID)), _const2((1, HID)),
            _const2((HID, HID)), _const2((1, HID)),
            _const2((1, 2)),
            pl.BlockSpec((1, 1, BN), lambda i: (i, 0, 0)),
        ],
        out_specs=[
            pl.BlockSpec((BN, HID), lambda i: (i, 0)),
            pl.BlockSpec((C, HID + AUG), lambda i: (0, 0)),
        ],
        out_shape=[
            jax.ShapeDtypeStruct((N, HID), jnp.float32),
            jax.ShapeDtypeStruct((C, HID + AUG), jnp.float32),
        ],
    )(g, dmat, cnt_row, k, v,
      cp['Wp'], cp['bp'].reshape(1, HID), cp['Wq'], cp['bq'].reshape(1, HID),
      fp['W1'], fp['b1'].reshape(1, HID), fp['W2'], fp['b2'].reshape(1, HID),
      sc, comm2d)


def _attn_last_body(g_r, dm_r, cnt_r, k_r, v_r, wp_r, bp_r, wq_r, bq_r,
                    w1_r, b1_r, w2_r, b2_r, sc_r, wo_r, bo_r,
                    out_r):
    gn = _attn_core(g_r[...], dm_r[...], cnt_r, k_r, v_r, wp_r, bp_r,
                    wq_r, bq_r, w1_r, b1_r, w2_r, b2_r,
                    sc_r[0, 0], sc_r[0, 1])
    out_r[...] = (jnp.dot(gn, wo_r[...], preferred_element_type=jnp.float32)
                  + bo_r[...])


def _attn_last(g, dmat, cnt_row, k, v, cp, fp, op):
    sc = jnp.stack([cp['w_dis'], cp['b_dis']]).reshape(1, 2)
    return pl.pallas_call(
        _attn_last_body,
        grid=(GRID_N,),
        in_specs=[
            pl.BlockSpec((BN, HID), lambda i: (i, 0)),
            pl.BlockSpec((BN, C), lambda i: (i, 0)),
            _const2((1, C)),
            _const2((C, HID)), _const2((C, HID)),
            _const2((HID, HID)), _const2((1, HID)),
            _const2((HID, HID)), _const2((1, HID)),
            _const2((HID, HID)), _const2((1, HID)),
            _const2((HID, HID)), _const2((1, HID)),
            _const2((1, 2)),
            _const2((HID, OUT)), _const2((1, OUT)),
        ],
        out_specs=pl.BlockSpec((BN, OUT), lambda i: (i, 0)),
        out_shape=jax.ShapeDtypeStruct((N, OUT), jnp.float32),
    )(g, dmat, cnt_row, k, v,
      cp['Wp'], cp['bp'].reshape(1, HID), cp['Wq'], cp['bq'].reshape(1, HID),
      fp['W1'], fp['b1'].reshape(1, HID), fp['W2'], fp['b2'].reshape(1, HID),
      sc, op['W'], op['b'].reshape(1, OUT))


def _sage1_body(ap_r, x_r, wl_r, bl_r, wr_r, h1_r, di_r):
    a = ap_r[0].astype(jnp.float32) + ap_r[1].astype(jnp.float32)
    sums = a[:, :D_IN]
    deg = a[:, D_IN:D_IN + 1]
    deginv = 1.0 / jnp.maximum(deg, 1.0)
    mean = sums * deginv
    h = jnp.dot(mean, wl_r[...], preferred_element_type=jnp.float32) + bl_r[...]
    h = h + jnp.dot(x_r[...], wr_r[...], preferred_element_type=jnp.float32)
    h = jnp.maximum(h, 0.0)
    h1_r[...] = h.astype(jnp.bfloat16)
    di_r[...] = jnp.broadcast_to(deginv, (BN, D_IN))


def _sage1(agg1p, x, gp):
    return pl.pallas_call(
        _sage1_body,
        grid=(GRID_N,),
        in_specs=[
            pl.BlockSpec((2, BN, D_IN + DAUG), lambda i: (0, i, 0)),
            pl.BlockSpec((BN, D_IN), lambda i: (i, 0)),
            _const2((D_IN, HID)), _const2((1, HID)),
            _const2((D_IN, HID)),
        ],
        out_specs=[
            pl.BlockSpec((BN, HID), lambda i: (i, 0)),
            pl.BlockSpec((BN, D_IN), lambda i: (i, 0)),
        ],
        out_shape=[
            jax.ShapeDtypeStruct((N, HID), jnp.bfloat16),
            jax.ShapeDtypeStruct((N, D_IN), jnp.float32),
        ],
    )(agg1p, x, gp['Wl'], gp['bl'].reshape(1, HID), gp['Wr'])


def _sage2_body(ag_r, h1_r, di_r, wl_r, bl_r, wr_r, wl3_r,
                h2_r, z_r):
    di = jnp.concatenate([di_r[...], di_r[...]], axis=1)
    sums = ag_r[0].astype(jnp.float32) + ag_r[1].astype(jnp.float32)
    mean = sums * di
    h1 = h1_r[...].astype(jnp.float32)
    h = jnp.dot(mean, wl_r[...], preferred_element_type=jnp.float32) + bl_r[...]
    h = h + jnp.dot(h1, wr_r[...], preferred_element_type=jnp.float32)
    h = jnp.maximum(h, 0.0)
    h2_r[...] = h
    z_r[...] = jnp.dot(h, wl3_r[...],
                       preferred_element_type=jnp.float32).astype(jnp.bfloat16)


def _sage2(agg2, h1, deginv, gp, gp3):
    return pl.pallas_call(
        _sage2_body,
        grid=(GRID_N,),
        in_specs=[
            pl.BlockSpec((2, BN, HID), lambda i: (0, i, 0)),
            pl.BlockSpec((BN, HID), lambda i: (i, 0)),
            pl.BlockSpec((BN, D_IN), lambda i: (i, 0)),
            _const2((HID, HID)), _const2((1, HID)),
            _const2((HID, HID)),
            _const2((HID, OUT)),
        ],
        out_specs=[
            pl.BlockSpec((BN, HID), lambda i: (i, 0)),
            pl.BlockSpec((BN, OUT), lambda i: (i, 0)),
        ],
        out_shape=[
            jax.ShapeDtypeStruct((N, HID), jnp.float32),
            jax.ShapeDtypeStruct((N, OUT), jnp.bfloat16),
        ],
    )(agg2, h1, deginv, gp['Wl'], gp['bl'].reshape(1, HID), gp['Wr'],
      gp3['Wl'])


def _sage3_body(ap_r, h2_r, di_r, xg_r, bl_r, wr_r, out_r):
    agg = (ap_r[0].astype(jnp.float32)
           + ap_r[1].astype(jnp.float32)) * di_r[...]
    out_r[...] = (agg + bl_r[...] + xg_r[...]
                  + jnp.dot(h2_r[...], wr_r[...],
                            preferred_element_type=jnp.float32))


def _sage3(agg3p, h2, deginv, xglobal, gp3):
    return pl.pallas_call(
        _sage3_body,
        grid=(GRID_N,),
        in_specs=[
            pl.BlockSpec((2, BN, OUT), lambda i: (0, i, 0)),
            pl.BlockSpec((BN, HID), lambda i: (i, 0)),
            pl.BlockSpec((BN, OUT), lambda i: (i, 0)),
            pl.BlockSpec((BN, OUT), lambda i: (i, 0)),
            _const2((1, OUT)),
            _const2((HID, OUT)),
        ],
        out_specs=pl.BlockSpec((BN, OUT), lambda i: (i, 0)),
        out_shape=jax.ShapeDtypeStruct((N, OUT), jnp.float32),
    )(agg3p, h2, deginv, xglobal, gp3['bl'].reshape(1, OUT), gp3['Wr'])


NTILES = 16
ZR = 624
ZR_LAST = N - (NTILES - 1) * ZR


def _make_edge_agg(ncols, edge_split, kch, grp, dt=jnp.float32):
    mesh = plsc.VectorSubcoreMesh(core_axis_name="c", subcore_axis_name="s")
    nch = (E // 32 if edge_split else E // NTILES) // kch
    ngrp = nch // grp
    gpairs = grp // 2
    tail = grp % 2

    @functools.partial(
        pl.kernel,
        out_type=jax.ShapeDtypeStruct((2, N, ncols), dt),
        mesh=mesh,
        scratch_types=[
            pltpu.VMEM_SHARED((N, ncols), dt),
            pltpu.VMEM((grp, kch), jnp.int32),
            pltpu.VMEM((grp, kch), jnp.int32),
            pltpu.VMEM((kch, ncols), dt),
            pltpu.VMEM((kch, ncols), dt),
            pltpu.SemaphoreType.DMA,
            pltpu.SemaphoreType.DMA,
            pltpu.SemaphoreType.DMA,
            pltpu.SemaphoreType.DMA,
        ],
        compiler_params=pltpu.CompilerParams(use_tc_tiling_on_sc=False),
    )
    def k(tab_a, tab_b, src2_h, dst2_h, zeros_h, out_h,
          acc, sidxs, didxs, rows0, rows1, g0, g1, s0, s1):
        c = lax.axis_index("c")
        s = lax.axis_index("s")

        @pl.when(s < NTILES - 1)
        def _():
            pltpu.sync_copy(zeros_h.at[pl.ds(0, ZR)],
                            acc.at[pl.ds(s * ZR, ZR)])

        @pl.when(s == NTILES - 1)
        def _():
            pltpu.sync_copy(zeros_h, acc.at[pl.ds((NTILES - 1) * ZR, ZR_LAST)])

        plsc.subcore_barrier()

        def run(tab, cb):
            def group(gi, carry):
                gb = cb + gi * grp
                pltpu.sync_copy(src2_h.at[pl.ds(gb, grp)], sidxs)
                pltpu.sync_copy(dst2_h.at[pl.ds(gb, grp)], didxs)
                pltpu.async_copy(tab.at[sidxs.at[0]], rows0, g0)
                pltpu.async_copy(tab.at[sidxs.at[1]], rows1, g1)

                def pair(j, carry2):
                    i0 = 2 * j
                    pltpu.make_async_copy(tab.at[sidxs.at[i0]], rows0,
                                          g0).wait()
                    pltpu.async_copy(rows0, acc.at[didxs.at[i0]], s0,
                                     add=True)
                    pltpu.make_async_copy(tab.at[sidxs.at[i0 + 1]], rows1,
                                          g1).wait()
                    pltpu.async_copy(rows1, acc.at[didxs.at[i0 + 1]], s1,
                                     add=True)

                    @pl.when(j < gpairs - 1)
                    def _():
                        pltpu.make_async_copy(rows0, acc.at[didxs.at[i0]],
                                              s0).wait()
                        pltpu.async_copy(tab.at[sidxs.at[i0 + 2]], rows0, g0)
                        pltpu.make_async_copy(rows1,
                                              acc.at[didxs.at[i0 + 1]],
                                              s1).wait()
                        pltpu.async_copy(tab.at[sidxs.at[i0 + 3]], rows1, g1)

                    return carry2

                lax.fori_loop(0, gpairs, pair, 0)
                pltpu.make_async_copy(rows0, acc.at[didxs.at[2 * gpairs - 2]],
                                      s0).wait()
                pltpu.make_async_copy(rows1, acc.at[didxs.at[2 * gpairs - 1]],
                                      s1).wait()
                if tail:
                    pltpu.async_copy(tab.at[sidxs.at[grp - 1]], rows0,
                                     g0).wait()
                    pltpu.async_copy(rows0, acc.at[didxs.at[grp - 1]], s0,
                                     add=True).wait()
                return carry

            lax.fori_loop(0, ngrp, group, 0)

        if edge_split:
            run(tab_a, (c * NTILES + s) * nch)
        else:
            @pl.when(c == 0)
            def _():
                run(tab_a, s * nch)

            @pl.when(c == 1)
            def _():
                run(tab_b, s * nch)

        plsc.subcore_barrier()

        @pl.when(s < NTILES - 1)
        def _():
            pltpu.sync_copy(acc.at[pl.ds(s * ZR, ZR)],
                            out_h.at[c, pl.ds(s * ZR, ZR)])

        @pl.when(s == NTILES - 1)
        def _():
            pltpu.sync_copy(acc.at[pl.ds((NTILES - 1) * ZR, ZR_LAST)],
                            out_h.at[c, pl.ds((NTILES - 1) * ZR, ZR_LAST)])

    return k


_edge_agg_aug = _make_edge_agg(D_IN + DAUG, True, 80, 25, jnp.bfloat16)
_edge_agg_feat = _make_edge_agg(HID, True, 80, 25, jnp.bfloat16)
_edge_agg_128 = _make_edge_agg(D_IN, True, 80, 25, jnp.bfloat16)


def _agg_edge_split_aug(table, src, dst):
    zeros = jnp.zeros((ZR_LAST, D_IN + DAUG), jnp.bfloat16)
    return _edge_agg_aug(table, table,
                         src.reshape(-1, 80), dst.reshape(-1, 80), zeros)


def _agg_edge_split_128(table, src, dst):
    zeros = jnp.zeros((ZR_LAST, D_IN), jnp.bfloat16)
    return _edge_agg_128(table, table,
                         src.reshape(-1, 80), dst.reshape(-1, 80), zeros)


def _agg_h1_split(h1, src, dst):
    zeros = jnp.zeros((ZR_LAST, HID), jnp.bfloat16)
    return _edge_agg_feat(h1, h1,
                          src.reshape(-1, 80), dst.reshape(-1, 80), zeros)


def kernel(x, edge_index, distance_matrix, nodes_to_community, params):
    src = edge_index[0]
    dst = edge_index[1]
    comm2d = nodes_to_community.reshape(GRID_N, 1, BN)

    g, praw, cnt_row = _fcin(x, comm2d, params['fc_in'])
    k0, v0 = _cprep(praw, params['convs'][0])
    g, praw1 = _attn_mid(g, distance_matrix, cnt_row, k0, v0,
                         params['convs'][0], params['ffs'][0], comm2d)
    k1, v1 = _cprep(praw1, params['convs'][1])
    xglobal = _attn_last(g, distance_matrix, cnt_row, k1, v1,
                         params['convs'][1], params['ffs'][1],
                         params['fc_out'])

    x_aug = jnp.concatenate([x, jnp.ones((N, DAUG), jnp.float32)],
                            axis=1).astype(jnp.bfloat16)
    agg1p = _agg_edge_split_aug(x_aug, src, dst)
    h1, deginv = _sage1(agg1p, x, params['gnn'][0])
    h1, g = lax.optimization_barrier((h1, g))
    agg2 = _agg_h1_split(h1, src, dst)
    h2, z = _sage2(agg2, h1, deginv, params['gnn'][1], params['gnn'][2])
    z, xglobal = lax.optimization_barrier((z, xglobal))
    agg3p = _agg_edge_split_128(z, src, dst)
    return _sage3(agg3p, h2, deginv, xglobal, params['gnn'][2])


# speedup vs baseline: 10.6520x; 1.0417x over previous
import functools

import jax
import jax.numpy as jnp
from jax import lax
from jax.experimental import pallas as pl
from jax.experimental.pallas import tpu as pltpu
from jax.experimental.pallas import tpu_sc as plsc

N = 10000
E = 320000
D_IN = 128
HID = 256
OUT = 128
C = 512
HEADS = 4
DHEAD = HID // HEADS
SCALE = 1.0 / (DHEAD ** 0.5)
BN = 2000
GRID_N = N // BN
AUG = 16
DAUG = 32


def _const2(shape):
    return pl.BlockSpec(shape, lambda i: (0, 0))


def _fcin_body(x_r, w1_r, b1_r, w2_r, b2_r, comm_r,
               g_r, praw_r, cnt_r):
    i = pl.program_id(0)
    g = jnp.maximum(jnp.dot(x_r[...], w1_r[...],
                            preferred_element_type=jnp.float32) + b1_r[...], 0.0)
    g = jnp.dot(g, w2_r[...], preferred_element_type=jnp.float32) + b2_r[...]
    g_r[...] = g
    comm = comm_r[0, 0, :]
    m2 = (comm[:, None] == lax.broadcasted_iota(jnp.int32, (BN, C), 1)
          ).astype(jnp.float32)
    g_aug = jnp.concatenate([g, jnp.ones((BN, AUG), jnp.float32)], axis=1)
    pb = lax.dot_general(m2, g_aug, (((0,), (0,)), ((), ())),
                         preferred_element_type=jnp.float32)
    cb = jnp.sum(m2, axis=0, keepdims=True)

    @pl.when(i == 0)
    def _():
        praw_r[...] = pb
        cnt_r[...] = cb

    @pl.when(i > 0)
    def _():
        praw_r[...] += pb
        cnt_r[...] += cb


def _fcin(x, comm2d, fp):
    return pl.pallas_call(
        _fcin_body,
        grid=(GRID_N,),
        in_specs=[
            pl.BlockSpec((BN, D_IN), lambda i: (i, 0)),
            _const2((D_IN, HID)), _const2((1, HID)),
            _const2((HID, HID)), _const2((1, HID)),
            pl.BlockSpec((1, 1, BN), lambda i: (i, 0, 0)),
        ],
        out_specs=[
            pl.BlockSpec((BN, HID), lambda i: (i, 0)),
            pl.BlockSpec((C, HID + AUG), lambda i: (0, 0)),
            pl.BlockSpec((1, C), lambda i: (0, 0)),
        ],
        out_shape=[
            jax.ShapeDtypeStruct((N, HID), jnp.float32),
            jax.ShapeDtypeStruct((C, HID + AUG), jnp.float32),
            jax.ShapeDtypeStruct((1, C), jnp.float32),
        ],
    )(x, fp['W1'], fp['b1'].reshape(1, HID), fp['W2'], fp['b2'].reshape(1, HID),
      comm2d)


def _cprep_body(pa_r, wk_r, bk_r, wv_r, bv_r, k_r, v_r):
    praw = pa_r[:, :HID]
    cnt = pa_r[:, HID:HID + 1]
    inv = 1.0 / jnp.maximum(cnt, 1.0)
    cavg = praw * inv
    k_r[...] = jnp.dot(cavg, wk_r[...],
                       preferred_element_type=jnp.float32) + bk_r[...]
    v_r[...] = jnp.dot(cavg, wv_r[...],
                       preferred_element_type=jnp.float32) + bv_r[...]


def _cprep(praw_aug, cp):
    return pl.pallas_call(
        _cprep_body,
        grid=(1,),
        in_specs=[
            _const2((C, HID + AUG)),
            _const2((HID, HID)), _const2((1, HID)),
            _const2((HID, HID)), _const2((1, HID)),
        ],
        out_specs=[
            pl.BlockSpec((C, HID), lambda i: (0, 0)),
            pl.BlockSpec((C, HID), lambda i: (0, 0)),
        ],
        out_shape=[
            jax.ShapeDtypeStruct((C, HID), jnp.float32),
            jax.ShapeDtypeStruct((C, HID), jnp.float32),
        ],
    )(praw_aug, cp['Wk'], cp['bk'].reshape(1, HID),
      cp['Wv'], cp['bv'].reshape(1, HID))


def _attn_core(g, dmat, cnt_r, k_r, v_r, wp_r, bp_r, wq_r, bq_r,
               w1_r, b1_r, w2_r, b2_r, wdis, bdis):
    q = jnp.dot(g, wp_r[...], preferred_element_type=jnp.float32) + bp_r[...]
    q = jnp.dot(q, wq_r[...], preferred_element_type=jnp.float32) + bq_r[...]
    base = dmat * wdis + bdis + jnp.log(cnt_r[...])
    outs = []
    for h in range(HEADS):
        sl = slice(h * DHEAD, (h + 1) * DHEAD)
        qh = q[:, sl]
        kh = k_r[:, sl]
        vh = v_r[:, sl]
        dots = lax.dot_general(qh, kh, (((1,), (1,)), ((), ())),
                               preferred_element_type=jnp.float32) * SCALE + base
        mx = jnp.max(dots, axis=-1, keepdims=True)
        e = jnp.exp(dots - mx)
        s = jnp.sum(e, axis=-1, keepdims=True)
        outs.append(jnp.dot(e, vh, preferred_element_type=jnp.float32) / s)
    o = jnp.concatenate(outs, axis=1)
    o = jnp.maximum(jnp.dot(o, w1_r[...],
                            preferred_element_type=jnp.float32) + b1_r[...], 0.0)
    return jnp.maximum(jnp.dot(o, w2_r[...],
                               preferred_element_type=jnp.float32) + b2_r[...], 0.0)


def _attn_mid_body(g_r, dm_r, cnt_r, k_r, v_r, wp_r, bp_r, wq_r, bq_r,
                   w1_r, b1_r, w2_r, b2_r, sc_r, comm_r,
                   gn_r, praw_r):
    i = pl.program_id(0)
    gn = _attn_core(g_r[...], dm_r[...], cnt_r, k_r, v_r, wp_r, bp_r,
                    wq_r, bq_r, w1_r, b1_r, w2_r, b2_r,
                    sc_r[0, 0], sc_r[0, 1])
    gn_r[...] = gn
    comm = comm_r[0, 0, :]
    m2 = (comm[:, None] == lax.broadcasted_iota(jnp.int32, (BN, C), 1)
          ).astype(jnp.float32)
    g_aug = jnp.concatenate([gn, jnp.ones((BN, AUG), jnp.float32)], axis=1)
    pb = lax.dot_general(m2, g_aug, (((0,), (0,)), ((), ())),
                         preferred_element_type=jnp.float32)

    @pl.when(i == 0)
    def _():
        praw_r[...] = pb

    @pl.when(i > 0)
    def _():
        praw_r[...] += pb


def _attn_mid(g, dmat, cnt_row, k, v, cp, fp, comm2d):
    sc = jnp.stack([cp['w_dis'], cp['b_dis']]).reshape(1, 2)
    return pl.pallas_call(
        _attn_mid_body,
        grid=(GRID_N,),
        in_specs=[
            pl.BlockSpec((BN, HID), lambda i: (i, 0)),
            pl.BlockSpec((BN, C), lambda i: (i, 0)),
            _const2((1, C)),
            _const2((C, HID)), _const2((C, HID)),
            _const2((HID, HID)), _const2((1, HID)),
            _const2((HID, HID)), _const2((1, HID)),
            _const2((HID, HID)), _const2((1, HID)),
            _const2((HID, HID)), _const2((1, HID)),
            _const2((1, 2)),
            pl.BlockSpec((1, 1, BN), lambda i: (i, 0, 0)),
        ],
        out_specs=[
            pl.BlockSpec((BN, HID), lambda i: (i, 0)),
            pl.BlockSpec((C, HID + AUG), lambda i: (0, 0)),
        ],
        out_shape=[
            jax.ShapeDtypeStruct((N, HID), jnp.float32),
            jax.ShapeDtypeStruct((C, HID + AUG), jnp.float32),
        ],
    )(g, dmat, cnt_row, k, v,
      cp['Wp'], cp['bp'].reshape(1, HID), cp['Wq'], cp['bq'].reshape(1, HID),
      fp['W1'], fp['b1'].reshape(1, HID), fp['W2'], fp['b2'].reshape(1, HID),
      sc, comm2d)


def _attn_last_body(g_r, dm_r, cnt_r, k_r, v_r, wp_r, bp_r, wq_r, bq_r,
                    w1_r, b1_r, w2_r, b2_r, sc_r, wo_r, bo_r,
                    out_r):
    gn = _attn_core(g_r[...], dm_r[...], cnt_r, k_r, v_r, wp_r, bp_r,
                    wq_r, bq_r, w1_r, b1_r, w2_r, b2_r,
                    sc_r[0, 0], sc_r[0, 1])
    out_r[...] = (jnp.dot(gn, wo_r[...], preferred_element_type=jnp.float32)
                  + bo_r[...])


def _attn_last(g, dmat, cnt_row, k, v, cp, fp, op):
    sc = jnp.stack([cp['w_dis'], cp['b_dis']]).reshape(1, 2)
    return pl.pallas_call(
        _attn_last_body,
        grid=(GRID_N,),
        in_specs=[
            pl.BlockSpec((BN, HID), lambda i: (i, 0)),
            pl.BlockSpec((BN, C), lambda i: (i, 0)),
            _const2((1, C)),
            _const2((C, HID)), _const2((C, HID)),
            _const2((HID, HID)), _const2((1, HID)),
            _const2((HID, HID)), _const2((1, HID)),
            _const2((HID, HID)), _const2((1, HID)),
            _const2((HID, HID)), _const2((1, HID)),
            _const2((1, 2)),
            _const2((HID, OUT)), _const2((1, OUT)),
        ],
        out_specs=pl.BlockSpec((BN, OUT), lambda i: (i, 0)),
        out_shape=jax.ShapeDtypeStruct((N, OUT), jnp.float32),
    )(g, dmat, cnt_row, k, v,
      cp['Wp'], cp['bp'].reshape(1, HID), cp['Wq'], cp['bq'].reshape(1, HID),
      fp['W1'], fp['b1'].reshape(1, HID), fp['W2'], fp['b2'].reshape(1, HID),
      sc, op['W'], op['b'].reshape(1, OUT))


def _sage1_body(ap_r, x_r, wl_r, bl_r, wr_r, h1_r, di_r):
    a = ap_r[0].astype(jnp.float32) + ap_r[1].astype(jnp.float32)
    sums = a[:, :D_IN]
    deg = a[:, D_IN:D_IN + 1]
    deginv = 1.0 / jnp.maximum(deg, 1.0)
    mean = sums * deginv
    h = jnp.dot(mean, wl_r[...], preferred_element_type=jnp.float32) + bl_r[...]
    h = h + jnp.dot(x_r[...], wr_r[...], preferred_element_type=jnp.float32)
    h = jnp.maximum(h, 0.0)
    h1_r[...] = h.astype(jnp.bfloat16)
    di_r[...] = jnp.broadcast_to(deginv, (BN, D_IN))


def _sage1(agg1p, x, gp):
    return pl.pallas_call(
        _sage1_body,
        grid=(GRID_N,),
        in_specs=[
            pl.BlockSpec((2, BN, D_IN + DAUG), lambda i: (0, i, 0)),
            pl.BlockSpec((BN, D_IN), lambda i: (i, 0)),
            _const2((D_IN, HID)), _const2((1, HID)),
            _const2((D_IN, HID)),
        ],
        out_specs=[
            pl.BlockSpec((BN, HID), lambda i: (i, 0)),
            pl.BlockSpec((BN, D_IN), lambda i: (i, 0)),
        ],
        out_shape=[
            jax.ShapeDtypeStruct((N, HID), jnp.bfloat16),
            jax.ShapeDtypeStruct((N, D_IN), jnp.float32),
        ],
    )(agg1p, x, gp['Wl'], gp['bl'].reshape(1, HID), gp['Wr'])


def _sage2_body(ag_r, h1_r, di_r, wl_r, bl_r, wr_r, wl3_r,
                h2_r, z_r):
    di = jnp.concatenate([di_r[...], di_r[...]], axis=1)
    sums = ag_r[0].astype(jnp.float32) + ag_r[1].astype(jnp.float32)
    mean = sums * di
    h1 = h1_r[...].astype(jnp.float32)
    h = jnp.dot(mean, wl_r[...], preferred_element_type=jnp.float32) + bl_r[...]
    h = h + jnp.dot(h1, wr_r[...], preferred_element_type=jnp.float32)
    h = jnp.maximum(h, 0.0)
    h2_r[...] = h
    z_r[...] = jnp.dot(h, wl3_r[...],
                       preferred_element_type=jnp.float32).astype(jnp.bfloat16)


def _sage2(agg2, h1, deginv, gp, gp3):
    return pl.pallas_call(
        _sage2_body,
        grid=(GRID_N,),
        in_specs=[
            pl.BlockSpec((2, BN, HID), lambda i: (0, i, 0)),
            pl.BlockSpec((BN, HID), lambda i: (i, 0)),
            pl.BlockSpec((BN, D_IN), lambda i: (i, 0)),
            _const2((HID, HID)), _const2((1, HID)),
            _const2((HID, HID)),
            _const2((HID, OUT)),
        ],
        out_specs=[
            pl.BlockSpec((BN, HID), lambda i: (i, 0)),
            pl.BlockSpec((BN, OUT), lambda i: (i, 0)),
        ],
        out_shape=[
            jax.ShapeDtypeStruct((N, HID), jnp.float32),
            jax.ShapeDtypeStruct((N, OUT), jnp.bfloat16),
        ],
    )(agg2, h1, deginv, gp['Wl'], gp['bl'].reshape(1, HID), gp['Wr'],
      gp3['Wl'])


def _sage3_body(ap_r, h2_r, di_r, xg_r, bl_r, wr_r, out_r):
    agg = (ap_r[0].astype(jnp.float32)
           + ap_r[1].astype(jnp.float32)) * di_r[...]
    out_r[...] = (agg + bl_r[...] + xg_r[...]
                  + jnp.dot(h2_r[...], wr_r[...],
                            preferred_element_type=jnp.float32))


def _sage3(agg3p, h2, deginv, xglobal, gp3):
    return pl.pallas_call(
        _sage3_body,
        grid=(GRID_N,),
        in_specs=[
            pl.BlockSpec((2, BN, OUT), lambda i: (0, i, 0)),
            pl.BlockSpec((BN, HID), lambda i: (i, 0)),
            pl.BlockSpec((BN, OUT), lambda i: (i, 0)),
            pl.BlockSpec((BN, OUT), lambda i: (i, 0)),
            _const2((1, OUT)),
            _const2((HID, OUT)),
        ],
        out_specs=pl.BlockSpec((BN, OUT), lambda i: (i, 0)),
        out_shape=jax.ShapeDtypeStruct((N, OUT), jnp.float32),
    )(agg3p, h2, deginv, xglobal, gp3['bl'].reshape(1, OUT), gp3['Wr'])


NTILES = 16
ZR = 624
ZR_LAST = N - (NTILES - 1) * ZR


def _make_edge_agg(ncols, edge_split, kch, grp, dt=jnp.float32):
    mesh = plsc.VectorSubcoreMesh(core_axis_name="c", subcore_axis_name="s")
    nch = (E // 32 if edge_split else E // NTILES) // kch
    ngrp = nch // grp
    gpairs = grp // 2
    tail = grp % 2

    @functools.partial(
        pl.kernel,
        out_type=jax.ShapeDtypeStruct((2, N, ncols), dt),
        mesh=mesh,
        scratch_types=[
            pltpu.VMEM_SHARED((N, ncols), dt),
            pltpu.VMEM((grp, kch), jnp.int32),
            pltpu.VMEM((grp, kch), jnp.int32),
            pltpu.VMEM((kch, ncols), dt),
            pltpu.VMEM((kch, ncols), dt),
            pltpu.SemaphoreType.DMA,
            pltpu.SemaphoreType.DMA,
            pltpu.SemaphoreType.DMA,
            pltpu.SemaphoreType.DMA,
        ],
        compiler_params=pltpu.CompilerParams(use_tc_tiling_on_sc=False),
    )
    def k(tab_a, tab_b, src2_h, dst2_h, zeros_h, out_h,
          acc, sidxs, didxs, rows0, rows1, g0, g1, s0, s1):
        c = lax.axis_index("c")
        s = lax.axis_index("s")

        @pl.when(s < NTILES - 1)
        def _():
            pltpu.sync_copy(zeros_h.at[pl.ds(0, ZR)],
                            acc.at[pl.ds(s * ZR, ZR)])

        @pl.when(s == NTILES - 1)
        def _():
            pltpu.sync_copy(zeros_h, acc.at[pl.ds((NTILES - 1) * ZR, ZR_LAST)])

        plsc.subcore_barrier()

        def run(tab, cb):
            def group(gi, carry):
                gb = cb + gi * grp
                pltpu.sync_copy(src2_h.at[pl.ds(gb, grp)], sidxs)
                pltpu.sync_copy(dst2_h.at[pl.ds(gb, grp)], didxs)
                pltpu.async_copy(tab.at[sidxs.at[0]], rows0, g0)
                pltpu.async_copy(tab.at[sidxs.at[1]], rows1, g1)

                def pair(j, carry2):
                    i0 = 2 * j
                    pltpu.make_async_copy(tab.at[sidxs.at[i0]], rows0,
                                          g0).wait()
                    pltpu.async_copy(rows0, acc.at[didxs.at[i0]], s0,
                                     add=True)
                    pltpu.make_async_copy(tab.at[sidxs.at[i0 + 1]], rows1,
                                          g1).wait()
                    pltpu.async_copy(rows1, acc.at[didxs.at[i0 + 1]], s1,
                                     add=True)

                    @pl.when(j < gpairs - 1)
                    def _():
                        pltpu.make_async_copy(rows0, acc.at[didxs.at[i0]],
                                              s0).wait()
                        pltpu.async_copy(tab.at[sidxs.at[i0 + 2]], rows0, g0)
                        pltpu.make_async_copy(rows1,
                                              acc.at[didxs.at[i0 + 1]],
                                              s1).wait()
                        pltpu.async_copy(tab.at[sidxs.at[i0 + 3]], rows1, g1)

                    return carry2

                lax.fori_loop(0, gpairs, pair, 0)
                pltpu.make_async_copy(rows0, acc.at[didxs.at[2 * gpairs - 2]],
                                      s0).wait()
                pltpu.make_async_copy(rows1, acc.at[didxs.at[2 * gpairs - 1]],
                                      s1).wait()
                if tail:
                    pltpu.async_copy(tab.at[sidxs.at[grp - 1]], rows0,
                                     g0).wait()
                    pltpu.async_copy(rows0, acc.at[didxs.at[grp - 1]], s0,
                                     add=True).wait()
                return carry

            lax.fori_loop(0, ngrp, group, 0)

        if edge_split:
            run(tab_a, (c * NTILES + s) * nch)
        else:
            @pl.when(c == 0)
            def _():
                run(tab_a, s * nch)

            @pl.when(c == 1)
            def _():
                run(tab_b, s * nch)

        plsc.subcore_barrier()

        @pl.when(s < NTILES - 1)
        def _():
            pltpu.sync_copy(acc.at[pl.ds(s * ZR, ZR)],
                            out_h.at[c, pl.ds(s * ZR, ZR)])

        @pl.when(s == NTILES - 1)
        def _():
            pltpu.sync_copy(acc.at[pl.ds((NTILES - 1) * ZR, ZR_LAST)],
                            out_h.at[c, pl.ds((NTILES - 1) * ZR, ZR_LAST)])

    return k


_edge_agg_aug = _make_edge_agg(D_IN + DAUG, True, 100, 50, jnp.bfloat16)
_edge_agg_feat = _make_edge_agg(HID, True, 100, 50, jnp.bfloat16)
_edge_agg_128 = _make_edge_agg(D_IN, True, 100, 50, jnp.bfloat16)


def _agg_edge_split_aug(table, src, dst):
    zeros = jnp.zeros((ZR_LAST, D_IN + DAUG), jnp.bfloat16)
    return _edge_agg_aug(table, table,
                         src.reshape(-1, 100), dst.reshape(-1, 100), zeros)


def _agg_edge_split_128(table, src, dst):
    zeros = jnp.zeros((ZR_LAST, D_IN), jnp.bfloat16)
    return _edge_agg_128(table, table,
                         src.reshape(-1, 100), dst.reshape(-1, 100), zeros)


def _agg_h1_split(h1, src, dst):
    zeros = jnp.zeros((ZR_LAST, HID), jnp.bfloat16)
    return _edge_agg_feat(h1, h1,
                          src.reshape(-1, 100), dst.reshape(-1, 100), zeros)


def kernel(x, edge_index, distance_matrix, nodes_to_community, params):
    src = edge_index[0]
    dst = edge_index[1]
    comm2d = nodes_to_community.reshape(GRID_N, 1, BN)

    g, praw, cnt_row = _fcin(x, comm2d, params['fc_in'])
    k0, v0 = _cprep(praw, params['convs'][0])
    g, praw1 = _attn_mid(g, distance_matrix, cnt_row, k0, v0,
                         params['convs'][0], params['ffs'][0], comm2d)
    k1, v1 = _cprep(praw1, params['convs'][1])
    xglobal = _attn_last(g, distance_matrix, cnt_row, k1, v1,
                         params['convs'][1], params['ffs'][1],
                         params['fc_out'])

    x_aug = jnp.concatenate([x, jnp.ones((N, DAUG), jnp.float32)],
                            axis=1).astype(jnp.bfloat16)
    agg1p = _agg_edge_split_aug(x_aug, src, dst)
    h1, deginv = _sage1(agg1p, x, params['gnn'][0])
    h1, g = lax.optimization_barrier((h1, g))
    agg2 = _agg_h1_split(h1, src, dst)
    h2, z = _sage2(agg2, h1, deginv, params['gnn'][1], params['gnn'][2])
    z, xglobal = lax.optimization_barrier((z, xglobal))
    agg3p = _agg_edge_split_128(z, src, dst)
    return _sage3(agg3p, h2, deginv, xglobal, params['gnn'][2])


# speedup vs baseline: 10.9908x; 1.0318x over previous
import functools

import jax
import jax.numpy as jnp
from jax import lax
from jax.experimental import pallas as pl
from jax.experimental.pallas import tpu as pltpu
from jax.experimental.pallas import tpu_sc as plsc

N = 10000
E = 320000
D_IN = 128
HID = 256
OUT = 128
C = 512
HEADS = 4
DHEAD = HID // HEADS
SCALE = 1.0 / (DHEAD ** 0.5)
BN = 2000
GRID_N = N // BN
AUG = 16
DAUG = 32


def _const2(shape):
    return pl.BlockSpec(shape, lambda i: (0, 0))


def _fcin_body(x_r, w1_r, b1_r, w2_r, b2_r, comm_r,
               g_r, praw_r, cnt_r):
    i = pl.program_id(0)
    g = jnp.maximum(jnp.dot(x_r[...], w1_r[...],
                            preferred_element_type=jnp.float32) + b1_r[...], 0.0)
    g = jnp.dot(g, w2_r[...], preferred_element_type=jnp.float32) + b2_r[...]
    g_r[...] = g
    comm = comm_r[0, 0, :]
    m2 = (comm[:, None] == lax.broadcasted_iota(jnp.int32, (BN, C), 1)
          ).astype(jnp.float32)
    g_aug = jnp.concatenate([g, jnp.ones((BN, AUG), jnp.float32)], axis=1)
    pb = lax.dot_general(m2, g_aug, (((0,), (0,)), ((), ())),
                         preferred_element_type=jnp.float32)
    cb = jnp.sum(m2, axis=0, keepdims=True)

    @pl.when(i == 0)
    def _():
        praw_r[...] = pb
        cnt_r[...] = cb

    @pl.when(i > 0)
    def _():
        praw_r[...] += pb
        cnt_r[...] += cb


def _fcin(x, comm2d, fp):
    return pl.pallas_call(
        _fcin_body,
        grid=(GRID_N,),
        in_specs=[
            pl.BlockSpec((BN, D_IN), lambda i: (i, 0)),
            _const2((D_IN, HID)), _const2((1, HID)),
            _const2((HID, HID)), _const2((1, HID)),
            pl.BlockSpec((1, 1, BN), lambda i: (i, 0, 0)),
        ],
        out_specs=[
            pl.BlockSpec((BN, HID), lambda i: (i, 0)),
            pl.BlockSpec((C, HID + AUG), lambda i: (0, 0)),
            pl.BlockSpec((1, C), lambda i: (0, 0)),
        ],
        out_shape=[
            jax.ShapeDtypeStruct((N, HID), jnp.float32),
            jax.ShapeDtypeStruct((C, HID + AUG), jnp.float32),
            jax.ShapeDtypeStruct((1, C), jnp.float32),
        ],
    )(x, fp['W1'], fp['b1'].reshape(1, HID), fp['W2'], fp['b2'].reshape(1, HID),
      comm2d)


def _cprep_body(pa_r, wk_r, bk_r, wv_r, bv_r, k_r, v_r):
    praw = pa_r[:, :HID]
    cnt = pa_r[:, HID:HID + 1]
    inv = 1.0 / jnp.maximum(cnt, 1.0)
    cavg = praw * inv
    k_r[...] = jnp.dot(cavg, wk_r[...],
                       preferred_element_type=jnp.float32) + bk_r[...]
    v_r[...] = jnp.dot(cavg, wv_r[...],
                       preferred_element_type=jnp.float32) + bv_r[...]


def _cprep(praw_aug, cp):
    return pl.pallas_call(
        _cprep_body,
        grid=(1,),
        in_specs=[
            _const2((C, HID + AUG)),
            _const2((HID, HID)), _const2((1, HID)),
            _const2((HID, HID)), _const2((1, HID)),
        ],
        out_specs=[
            pl.BlockSpec((C, HID), lambda i: (0, 0)),
            pl.BlockSpec((C, HID), lambda i: (0, 0)),
        ],
        out_shape=[
            jax.ShapeDtypeStruct((C, HID), jnp.float32),
            jax.ShapeDtypeStruct((C, HID), jnp.float32),
        ],
    )(praw_aug, cp['Wk'], cp['bk'].reshape(1, HID),
      cp['Wv'], cp['bv'].reshape(1, HID))


def _attn_core(g, dmat, cnt_r, k_r, v_r, wp_r, bp_r, wq_r, bq_r,
               w1_r, b1_r, w2_r, b2_r, wdis, bdis):
    q = jnp.dot(g, wp_r[...], preferred_element_type=jnp.float32) + bp_r[...]
    q = jnp.dot(q, wq_r[...], preferred_element_type=jnp.float32) + bq_r[...]
    base = dmat * wdis + bdis + jnp.log(cnt_r[...])
    outs = []
    for h in range(HEADS):
        sl = slice(h * DHEAD, (h + 1) * DHEAD)
        qh = q[:, sl]
        kh = k_r[:, sl]
        vh = v_r[:, sl]
        dots = lax.dot_general(qh, kh, (((1,), (1,)), ((), ())),
                               preferred_element_type=jnp.float32) * SCALE + base
        mx = jnp.max(dots, axis=-1, keepdims=True)
        e = jnp.exp(dots - mx)
        s = jnp.sum(e, axis=-1, keepdims=True)
        outs.append(jnp.dot(e, vh, preferred_element_type=jnp.float32) / s)
    o = jnp.concatenate(outs, axis=1)
    o = jnp.maximum(jnp.dot(o, w1_r[...],
                            preferred_element_type=jnp.float32) + b1_r[...], 0.0)
    return jnp.maximum(jnp.dot(o, w2_r[...],
                               preferred_element_type=jnp.float32) + b2_r[...], 0.0)


def _attn_mid_body(g_r, dm_r, cnt_r, k_r, v_r, wp_r, bp_r, wq_r, bq_r,
                   w1_r, b1_r, w2_r, b2_r, sc_r, comm_r,
                   gn_r, praw_r):
    i = pl.program_id(0)
    gn = _attn_core(g_r[...], dm_r[...], cnt_r, k_r, v_r, wp_r, bp_r,
                    wq_r, bq_r, w1_r, b1_r, w2_r, b2_r,
                    sc_r[0, 0], sc_r[0, 1])
    gn_r[...] = gn
    comm = comm_r[0, 0, :]
    m2 = (comm[:, None] == lax.broadcasted_iota(jnp.int32, (BN, C), 1)
          ).astype(jnp.float32)
    g_aug = jnp.concatenate([gn, jnp.ones((BN, AUG), jnp.float32)], axis=1)
    pb = lax.dot_general(m2, g_aug, (((0,), (0,)), ((), ())),
                         preferred_element_type=jnp.float32)

    @pl.when(i == 0)
    def _():
        praw_r[...] = pb

    @pl.when(i > 0)
    def _():
        praw_r[...] += pb


def _attn_mid(g, dmat, cnt_row, k, v, cp, fp, comm2d):
    sc = jnp.stack([cp['w_dis'], cp['b_dis']]).reshape(1, 2)
    return pl.pallas_call(
        _attn_mid_body,
        grid=(GRID_N,),
        in_specs=[
            pl.BlockSpec((BN, HID), lambda i: (i, 0)),
            pl.BlockSpec((BN, C), lambda i: (i, 0)),
            _const2((1, C)),
            _const2((C, HID)), _const2((C, HID)),
            _const2((HID, HID)), _const2((1, HID)),
            _const2((HID, HID)), _const2((1, HID)),
            _const2((HID, HID)), _const2((1, HID)),
            _const2((HID, HID)), _const2((1, HID)),
            _const2((1, 2)),
            pl.BlockSpec((1, 1, BN), lambda i: (i, 0, 0)),
        ],
        out_specs=[
            pl.BlockSpec((BN, HID), lambda i: (i, 0)),
            pl.BlockSpec((C, HID + AUG), lambda i: (0, 0)),
        ],
        out_shape=[
            jax.ShapeDtypeStruct((N, HID), jnp.float32),
            jax.ShapeDtypeStruct((C, HID + AUG), jnp.float32),
        ],
    )(g, dmat, cnt_row, k, v,
      cp['Wp'], cp['bp'].reshape(1, HID), cp['Wq'], cp['bq'].reshape(1, HID),
      fp['W1'], fp['b1'].reshape(1, HID), fp['W2'], fp['b2'].reshape(1, HID),
      sc, comm2d)


def _attn_last_body(g_r, dm_r, cnt_r, k_r, v_r, wp_r, bp_r, wq_r, bq_r,
                    w1_r, b1_r, w2_r, b2_r, sc_r, wo_r, bo_r,
                    out_r):
    gn = _attn_core(g_r[...], dm_r[...], cnt_r, k_r, v_r, wp_r, bp_r,
                    wq_r, bq_r, w1_r, b1_r, w2_r, b2_r,
                    sc_r[0, 0], sc_r[0, 1])
    out_r[...] = (jnp.dot(gn, wo_r[...], preferred_element_type=jnp.float32)
                  + bo_r[...])


def _attn_last(g, dmat, cnt_row, k, v, cp, fp, op):
    sc = jnp.stack([cp['w_dis'], cp['b_dis']]).reshape(1, 2)
    return pl.pallas_call(
        _attn_last_body,
        grid=(GRID_N,),
        in_specs=[
            pl.BlockSpec((BN, HID), lambda i: (i, 0)),
            pl.BlockSpec((BN, C), lambda i: (i, 0)),
            _const2((1, C)),
            _const2((C, HID)), _const2((C, HID)),
            _const2((HID, HID)), _const2((1, HID)),
            _const2((HID, HID)), _const2((1, HID)),
            _const2((HID, HID)), _const2((1, HID)),
            _const2((HID, HID)), _const2((1, HID)),
            _const2((1, 2)),
            _const2((HID, OUT)), _const2((1, OUT)),
        ],
        out_specs=pl.BlockSpec((BN, OUT), lambda i: (i, 0)),
        out_shape=jax.ShapeDtypeStruct((N, OUT), jnp.float32),
    )(g, dmat, cnt_row, k, v,
      cp['Wp'], cp['bp'].reshape(1, HID), cp['Wq'], cp['bq'].reshape(1, HID),
      fp['W1'], fp['b1'].reshape(1, HID), fp['W2'], fp['b2'].reshape(1, HID),
      sc, op['W'], op['b'].reshape(1, OUT))


def _sage1_body(ap_r, x_r, wl_r, bl_r, wr_r, h1_r, di_r):
    a = ap_r[0].astype(jnp.float32) + ap_r[1].astype(jnp.float32)
    sums = a[:, :D_IN]
    deg = a[:, D_IN:D_IN + 1]
    deginv = 1.0 / jnp.maximum(deg, 1.0)
    mean = sums * deginv
    h = jnp.dot(mean, wl_r[...], preferred_element_type=jnp.float32) + bl_r[...]
    h = h + jnp.dot(x_r[...], wr_r[...], preferred_element_type=jnp.float32)
    h = jnp.maximum(h, 0.0)
    h1_r[...] = h.astype(jnp.bfloat16)
    di_r[...] = jnp.broadcast_to(deginv, (BN, D_IN))


def _sage1(agg1p, x, gp):
    return pl.pallas_call(
        _sage1_body,
        grid=(GRID_N,),
        in_specs=[
            pl.BlockSpec((2, BN, D_IN + DAUG), lambda i: (0, i, 0)),
            pl.BlockSpec((BN, D_IN), lambda i: (i, 0)),
            _const2((D_IN, HID)), _const2((1, HID)),
            _const2((D_IN, HID)),
        ],
        out_specs=[
            pl.BlockSpec((BN, HID), lambda i: (i, 0)),
            pl.BlockSpec((BN, D_IN), lambda i: (i, 0)),
        ],
        out_shape=[
            jax.ShapeDtypeStruct((N, HID), jnp.bfloat16),
            jax.ShapeDtypeStruct((N, D_IN), jnp.float32),
        ],
    )(agg1p, x, gp['Wl'], gp['bl'].reshape(1, HID), gp['Wr'])


def _sage2_body(ag_r, h1_r, di_r, wl_r, bl_r, wr_r, wl3_r,
                h2_r, z_r):
    di = jnp.concatenate([di_r[...], di_r[...]], axis=1)
    sums = ag_r[0].astype(jnp.float32) + ag_r[1].astype(jnp.float32)
    mean = sums * di
    h1 = h1_r[...].astype(jnp.float32)
    h = jnp.dot(mean, wl_r[...], preferred_element_type=jnp.float32) + bl_r[...]
    h = h + jnp.dot(h1, wr_r[...], preferred_element_type=jnp.float32)
    h = jnp.maximum(h, 0.0)
    h2_r[...] = h
    z_r[...] = jnp.dot(h, wl3_r[...],
                       preferred_element_type=jnp.float32).astype(jnp.bfloat16)


def _sage2(agg2, h1, deginv, gp, gp3):
    return pl.pallas_call(
        _sage2_body,
        grid=(GRID_N,),
        in_specs=[
            pl.BlockSpec((2, BN, HID), lambda i: (0, i, 0)),
            pl.BlockSpec((BN, HID), lambda i: (i, 0)),
            pl.BlockSpec((BN, D_IN), lambda i: (i, 0)),
            _const2((HID, HID)), _const2((1, HID)),
            _const2((HID, HID)),
            _const2((HID, OUT)),
        ],
        out_specs=[
            pl.BlockSpec((BN, HID), lambda i: (i, 0)),
            pl.BlockSpec((BN, OUT), lambda i: (i, 0)),
        ],
        out_shape=[
            jax.ShapeDtypeStruct((N, HID), jnp.float32),
            jax.ShapeDtypeStruct((N, OUT), jnp.bfloat16),
        ],
    )(agg2, h1, deginv, gp['Wl'], gp['bl'].reshape(1, HID), gp['Wr'],
      gp3['Wl'])


def _sage3_body(ap_r, h2_r, di_r, xg_r, bl_r, wr_r, out_r):
    agg = (ap_r[0].astype(jnp.float32)
           + ap_r[1].astype(jnp.float32)) * di_r[...]
    out_r[...] = (agg + bl_r[...] + xg_r[...]
                  + jnp.dot(h2_r[...], wr_r[...],
                            preferred_element_type=jnp.float32))


def _sage3(agg3p, h2, deginv, xglobal, gp3):
    return pl.pallas_call(
        _sage3_body,
        grid=(GRID_N,),
        in_specs=[
            pl.BlockSpec((2, BN, OUT), lambda i: (0, i, 0)),
            pl.BlockSpec((BN, HID), lambda i: (i, 0)),
            pl.BlockSpec((BN, OUT), lambda i: (i, 0)),
            pl.BlockSpec((BN, OUT), lambda i: (i, 0)),
            _const2((1, OUT)),
            _const2((HID, OUT)),
        ],
        out_specs=pl.BlockSpec((BN, OUT), lambda i: (i, 0)),
        out_shape=jax.ShapeDtypeStruct((N, OUT), jnp.float32),
    )(agg3p, h2, deginv, xglobal, gp3['bl'].reshape(1, OUT), gp3['Wr'])


NTILES = 16
ZR = 624
ZR_LAST = N - (NTILES - 1) * ZR


def _make_edge_agg(ncols, edge_split, kch, grp, dt=jnp.float32):
    mesh = plsc.VectorSubcoreMesh(core_axis_name="c", subcore_axis_name="s")
    nch = (E // 32 if edge_split else E // NTILES) // kch
    ngrp = nch // grp
    gpairs = grp // 2
    tail = grp % 2

    @functools.partial(
        pl.kernel,
        out_type=jax.ShapeDtypeStruct((2, N, ncols), dt),
        mesh=mesh,
        scratch_types=[
            pltpu.VMEM_SHARED((N, ncols), dt),
            pltpu.VMEM((grp, kch), jnp.int32),
            pltpu.VMEM((grp, kch), jnp.int32),
            pltpu.VMEM((kch, ncols), dt),
            pltpu.VMEM((kch, ncols), dt),
            pltpu.SemaphoreType.DMA,
            pltpu.SemaphoreType.DMA,
            pltpu.SemaphoreType.DMA,
            pltpu.SemaphoreType.DMA,
        ],
        compiler_params=pltpu.CompilerParams(use_tc_tiling_on_sc=False),
    )
    def k(tab_a, tab_b, src2_h, dst2_h, zeros_h, out_h,
          acc, sidxs, didxs, rows0, rows1, g0, g1, s0, s1):
        c = lax.axis_index("c")
        s = lax.axis_index("s")

        @pl.when(s < NTILES - 1)
        def _():
            pltpu.sync_copy(zeros_h.at[pl.ds(0, ZR)],
                            acc.at[pl.ds(s * ZR, ZR)])

        @pl.when(s == NTILES - 1)
        def _():
            pltpu.sync_copy(zeros_h, acc.at[pl.ds((NTILES - 1) * ZR, ZR_LAST)])

        plsc.subcore_barrier()

        def run(tab, cb):
            def group(gi, carry):
                gb = cb + gi * grp
                pltpu.sync_copy(src2_h.at[pl.ds(gb, grp)], sidxs)
                pltpu.sync_copy(dst2_h.at[pl.ds(gb, grp)], didxs)
                pltpu.async_copy(tab.at[sidxs.at[0]], rows0, g0)
                pltpu.async_copy(tab.at[sidxs.at[1]], rows1, g1)

                def pair(j, carry2):
                    i0 = 2 * j
                    pltpu.make_async_copy(tab.at[sidxs.at[i0]], rows0,
                                          g0).wait()
                    pltpu.async_copy(rows0, acc.at[didxs.at[i0]], s0,
                                     add=True)
                    pltpu.make_async_copy(tab.at[sidxs.at[i0 + 1]], rows1,
                                          g1).wait()
                    pltpu.async_copy(rows1, acc.at[didxs.at[i0 + 1]], s1,
                                     add=True)

                    @pl.when(j < gpairs - 1)
                    def _():
                        pltpu.make_async_copy(rows0, acc.at[didxs.at[i0]],
                                              s0).wait()
                        pltpu.async_copy(tab.at[sidxs.at[i0 + 2]], rows0, g0)
                        pltpu.make_async_copy(rows1,
                                              acc.at[didxs.at[i0 + 1]],
                                              s1).wait()
                        pltpu.async_copy(tab.at[sidxs.at[i0 + 3]], rows1, g1)

                    return carry2

                lax.fori_loop(0, gpairs, pair, 0)
                pltpu.make_async_copy(rows0, acc.at[didxs.at[2 * gpairs - 2]],
                                      s0).wait()
                pltpu.make_async_copy(rows1, acc.at[didxs.at[2 * gpairs - 1]],
                                      s1).wait()
                if tail:
                    pltpu.async_copy(tab.at[sidxs.at[grp - 1]], rows0,
                                     g0).wait()
                    pltpu.async_copy(rows0, acc.at[didxs.at[grp - 1]], s0,
                                     add=True).wait()
                return carry

            lax.fori_loop(0, ngrp, group, 0)

        if edge_split:
            run(tab_a, (c * NTILES + s) * nch)
        else:
            @pl.when(c == 0)
            def _():
                run(tab_a, s * nch)

            @pl.when(c == 1)
            def _():
                run(tab_b, s * nch)

        plsc.subcore_barrier()

        @pl.when(s < NTILES - 1)
        def _():
            pltpu.sync_copy(acc.at[pl.ds(s * ZR, ZR)],
                            out_h.at[c, pl.ds(s * ZR, ZR)])

        @pl.when(s == NTILES - 1)
        def _():
            pltpu.sync_copy(acc.at[pl.ds((NTILES - 1) * ZR, ZR_LAST)],
                            out_h.at[c, pl.ds((NTILES - 1) * ZR, ZR_LAST)])

    return k


_edge_agg_aug = _make_edge_agg(D_IN + DAUG, True, 125, 40, jnp.bfloat16)
_edge_agg_feat = _make_edge_agg(HID, True, 125, 40, jnp.bfloat16)
_edge_agg_128 = _make_edge_agg(D_IN, True, 125, 40, jnp.bfloat16)


def _agg_edge_split_aug(table, src, dst):
    zeros = jnp.zeros((ZR_LAST, D_IN + DAUG), jnp.bfloat16)
    return _edge_agg_aug(table, table,
                         src.reshape(-1, 125), dst.reshape(-1, 125), zeros)


def _agg_edge_split_128(table, src, dst):
    zeros = jnp.zeros((ZR_LAST, D_IN), jnp.bfloat16)
    return _edge_agg_128(table, table,
                         src.reshape(-1, 125), dst.reshape(-1, 125), zeros)


def _agg_h1_split(h1, src, dst):
    zeros = jnp.zeros((ZR_LAST, HID), jnp.bfloat16)
    return _edge_agg_feat(h1, h1,
                          src.reshape(-1, 125), dst.reshape(-1, 125), zeros)


def kernel(x, edge_index, distance_matrix, nodes_to_community, params):
    src = edge_index[0]
    dst = edge_index[1]
    comm2d = nodes_to_community.reshape(GRID_N, 1, BN)

    g, praw, cnt_row = _fcin(x, comm2d, params['fc_in'])
    k0, v0 = _cprep(praw, params['convs'][0])
    g, praw1 = _attn_mid(g, distance_matrix, cnt_row, k0, v0,
                         params['convs'][0], params['ffs'][0], comm2d)
    k1, v1 = _cprep(praw1, params['convs'][1])
    xglobal = _attn_last(g, distance_matrix, cnt_row, k1, v1,
                         params['convs'][1], params['ffs'][1],
                         params['fc_out'])

    x_aug = jnp.concatenate([x, jnp.ones((N, DAUG), jnp.float32)],
                            axis=1).astype(jnp.bfloat16)
    agg1p = _agg_edge_split_aug(x_aug, src, dst)
    h1, deginv = _sage1(agg1p, x, params['gnn'][0])
    h1, g = lax.optimization_barrier((h1, g))
    agg2 = _agg_h1_split(h1, src, dst)
    h2, z = _sage2(agg2, h1, deginv, params['gnn'][1], params['gnn'][2])
    z, xglobal = lax.optimization_barrier((z, xglobal))
    agg3p = _agg_edge_split_128(z, src, dst)
    return _sage3(agg3p, h2, deginv, xglobal, params['gnn'][2])
